# Initial kernel scaffold; baseline (speedup 1.0000x reference)
import jax
import jax.numpy as jnp
from jax.experimental import pallas as pl


def kernel(x, edge_index, edge_attr, batch, params):
    raise NotImplementedError("write your pallas kernel here")


# speedup vs baseline: 1.4719x; 1.4719x over previous
import functools
import math

import jax
import jax.numpy as jnp
from jax import lax
from jax.experimental import pallas as pl
from jax.experimental.pallas import tpu as pltpu

_N = 10000
_E = 160000
_HID = 256
_H = 8
_C = 32
_G = 16

_INTERPRET = False

_EBR = 4000
_NBR = 2000


def _headsum(m):
    r = lax.broadcasted_iota(jnp.int32, (_HID, _H), 0) // _C
    c = lax.broadcasted_iota(jnp.int32, (_HID, _H), 1)
    s = (r == c).astype(jnp.float32)
    return jnp.dot(m, s, preferred_element_type=jnp.float32)


def _headexpand(w):
    r = lax.broadcasted_iota(jnp.int32, (_H, _HID), 0)
    c = lax.broadcasted_iota(jnp.int32, (_H, _HID), 1) // _C
    s = (r == c).astype(jnp.float32)
    return jnp.dot(w, s, preferred_element_type=jnp.float32)


def _mm_body(act, x_ref, w_ref, b_ref, o_ref):
    y = jnp.dot(x_ref[...], w_ref[...], preferred_element_type=jnp.float32)
    y = y + b_ref[...]
    if act == "relu":
        y = jnp.maximum(y, 0.0)
    elif act == "gelu":
        y = jax.nn.gelu(y)
    o_ref[...] = y


def _mm_add_body(act, x_ref, w_ref, b_ref, a_ref, o_ref):
    y = jnp.dot(x_ref[...], w_ref[...], preferred_element_type=jnp.float32)
    y = y + b_ref[...]
    if act == "relu":
        y = jnp.maximum(y, 0.0)
    elif act == "gelu":
        y = jax.nn.gelu(y)
    o_ref[...] = y + a_ref[...]


def _mm(x, w, b, act=None, br=_NBR, add=None):
    m, k = x.shape
    n = w.shape[1]
    grid = (m // br,)
    in_specs = [
        pl.BlockSpec((br, k), lambda i: (i, 0)),
        pl.BlockSpec((k, n), lambda i: (0, 0)),
        pl.BlockSpec((1, n), lambda i: (0, 0)),
    ]
    args = [x, w, b.reshape(1, n)]
    if add is None:
        body = functools.partial(_mm_body, act)
    else:
        body = functools.partial(_mm_add_body, act)
        in_specs.append(pl.BlockSpec((br, n), lambda i: (i, 0)))
        args.append(add)
    return pl.pallas_call(
        body,
        grid=grid,
        in_specs=in_specs,
        out_specs=pl.BlockSpec((br, n), lambda i: (i, 0)),
        out_shape=jax.ShapeDtypeStruct((m, n), jnp.float32),
        interpret=_INTERPRET,
    )(*args)


def _norm_body(x_ref, a_ref, w_ref, b_ref, ms_ref, o_ref):
    x = x_ref[...] + a_ref[...]
    mean = jnp.mean(x, axis=0, keepdims=True)
    out = x - ms_ref[...] * mean
    var = jnp.mean(out * out, axis=0, keepdims=True)
    o_ref[...] = w_ref[...] * out / jnp.sqrt(var + 1e-5) + b_ref[...]


def _graph_norm(x, add, w, b, ms):
    return pl.pallas_call(
        _norm_body,
        out_shape=jax.ShapeDtypeStruct((_N, _HID), jnp.float32),
        interpret=_INTERPRET,
    )(x, add, w.reshape(1, _HID), b.reshape(1, _HID), ms.reshape(1, _HID))


def _tc_logits_body(qd_ref, ks_ref, e_ref, a_ref, m_ref):
    m = qd_ref[...] * (ks_ref[...] + e_ref[...])
    a = _headsum(m) * (1.0 / math.sqrt(float(_C)))
    a_ref[...] = a
    bm = jnp.max(a, axis=0, keepdims=True)

    @pl.when(pl.program_id(0) == 0)
    def _():
        m_ref[...] = bm

    @pl.when(pl.program_id(0) != 0)
    def _():
        m_ref[...] = jnp.maximum(m_ref[...], bm)


def _tconv_logits(qd, ks, e):
    grid = (_E // _EBR,)
    return pl.pallas_call(
        _tc_logits_body,
        grid=grid,
        in_specs=[
            pl.BlockSpec((_EBR, _HID), lambda i: (i, 0)),
            pl.BlockSpec((_EBR, _HID), lambda i: (i, 0)),
            pl.BlockSpec((_EBR, _HID), lambda i: (i, 0)),
        ],
        out_specs=[
            pl.BlockSpec((_EBR, _H), lambda i: (i, 0)),
            pl.BlockSpec((1, _H), lambda i: (0, 0)),
        ],
        out_shape=[
            jax.ShapeDtypeStruct((_E, _H), jnp.float32),
            jax.ShapeDtypeStruct((1, _H), jnp.float32),
        ],
        interpret=_INTERPRET,
    )(qd, ks, e)


def _tc_msg_body(a_ref, g_ref, vs_ref, e_ref, w_ref, msg_ref):
    w = jnp.exp(a_ref[...] - g_ref[...])
    w_ref[...] = w
    msg_ref[...] = (vs_ref[...] + e_ref[...]) * _headexpand(w)


def _tconv_msg(a, gmax, vs, e):
    grid = (_E // _EBR,)
    return pl.pallas_call(
        _tc_msg_body,
        grid=grid,
        in_specs=[
            pl.BlockSpec((_EBR, _H), lambda i: (i, 0)),
            pl.BlockSpec((1, _H), lambda i: (0, 0)),
            pl.BlockSpec((_EBR, _HID), lambda i: (i, 0)),
            pl.BlockSpec((_EBR, _HID), lambda i: (i, 0)),
        ],
        out_specs=[
            pl.BlockSpec((_EBR, _H), lambda i: (i, 0)),
            pl.BlockSpec((_EBR, _HID), lambda i: (i, 0)),
        ],
        out_shape=[
            jax.ShapeDtypeStruct((_E, _H), jnp.float32),
            jax.ShapeDtypeStruct((_E, _HID), jnp.float32),
        ],
        interpret=_INTERPRET,
    )(a, gmax, vs, e)


def _tc_final_body(num_ref, den_ref, xs_ref, wa_ref, wb_ref, o_ref):
    denw = _headexpand(den_ref[...])
    out = num_ref[...] / (denw + 1e-16)
    xs = xs_ref[...]
    logit = jnp.sum(out * wa_ref[...] + xs * wb_ref[...], axis=1, keepdims=True)
    beta = jax.nn.sigmoid(logit)
    o_ref[...] = beta * xs + (1.0 - beta) * out


def _tconv_final(num, den, xs, wa, wb):
    return pl.pallas_call(
        _tc_final_body,
        out_shape=jax.ShapeDtypeStruct((_N, _HID), jnp.float32),
        interpret=_INTERPRET,
    )(num, den, xs, wa.reshape(1, _HID), wb.reshape(1, _HID))


def _gat_node_body(hh_ref, asf_ref, adf_ref, as_ref, ad_ref):
    hh = hh_ref[...]
    as_ref[...] = _headsum(hh * asf_ref[...])
    ad_ref[...] = _headsum(hh * adf_ref[...])


def _gat_node(hh, asf, adf):
    return pl.pallas_call(
        _gat_node_body,
        out_shape=[
            jax.ShapeDtypeStruct((_N, _H), jnp.float32),
            jax.ShapeDtypeStruct((_N, _H), jnp.float32),
        ],
        interpret=_INTERPRET,
    )(hh, asf.reshape(1, _HID), adf.reshape(1, _HID))


def _gat_logits_body(asg_ref, adg_ref, e_ref, aef_ref, a_ref, m_ref):
    ae = _headsum(e_ref[...] * aef_ref[...])
    a = asg_ref[...] + adg_ref[...] + ae
    a = jnp.where(a >= 0.0, a, 0.2 * a)
    a_ref[...] = a
    bm = jnp.max(a, axis=0, keepdims=True)

    @pl.when(pl.program_id(0) == 0)
    def _():
        m_ref[...] = bm

    @pl.when(pl.program_id(0) != 0)
    def _():
        m_ref[...] = jnp.maximum(m_ref[...], bm)


def _gat_logits(asg, adg, e, aef):
    grid = (_E // _EBR,)
    return pl.pallas_call(
        _gat_logits_body,
        grid=grid,
        in_specs=[
            pl.BlockSpec((_EBR, _H), lambda i: (i, 0)),
            pl.BlockSpec((_EBR, _H), lambda i: (i, 0)),
            pl.BlockSpec((_EBR, _HID), lambda i: (i, 0)),
            pl.BlockSpec((1, _HID), lambda i: (0, 0)),
        ],
        out_specs=[
            pl.BlockSpec((_EBR, _H), lambda i: (i, 0)),
            pl.BlockSpec((1, _H), lambda i: (0, 0)),
        ],
        out_shape=[
            jax.ShapeDtypeStruct((_E, _H), jnp.float32),
            jax.ShapeDtypeStruct((1, _H), jnp.float32),
        ],
        interpret=_INTERPRET,
    )(asg, adg, e, aef.reshape(1, _HID))


def _gat_msg_body(a_ref, g_ref, hs_ref, w_ref, msg_ref):
    w = jnp.exp(a_ref[...] - g_ref[...])
    w_ref[...] = w
    msg_ref[...] = hs_ref[...] * _headexpand(w)


def _gat_msg(a, gmax, hs):
    grid = (_E // _EBR,)
    return pl.pallas_call(
        _gat_msg_body,
        grid=grid,
        in_specs=[
            pl.BlockSpec((_EBR, _H), lambda i: (i, 0)),
            pl.BlockSpec((1, _H), lambda i: (0, 0)),
            pl.BlockSpec((_EBR, _HID), lambda i: (i, 0)),
        ],
        out_specs=[
            pl.BlockSpec((_EBR, _H), lambda i: (i, 0)),
            pl.BlockSpec((_EBR, _HID), lambda i: (i, 0)),
        ],
        out_shape=[
            jax.ShapeDtypeStruct((_E, _H), jnp.float32),
            jax.ShapeDtypeStruct((_E, _HID), jnp.float32),
        ],
        interpret=_INTERPRET,
    )(a, gmax, hs)


def _gat_final_body(num_ref, den_ref, b_ref, o_ref):
    denw = _headexpand(den_ref[...])
    o_ref[...] = num_ref[...] / (denw + 1e-16) + b_ref[...]


def _gat_final(num, den, bias):
    return pl.pallas_call(
        _gat_final_body,
        out_shape=jax.ShapeDtypeStruct((_N, _HID), jnp.float32),
        interpret=_INTERPRET,
    )(num, den, bias.reshape(1, _HID))


def _gin1_body(h_ref, agg_ref, eps_ref, w_ref, b_ref, g_ref, bt_ref, o_ref):
    h0 = (1.0 + eps_ref[0, 0]) * h_ref[...] + agg_ref[...]
    h1 = jnp.dot(h0, w_ref[...], preferred_element_type=jnp.float32) + b_ref[...]
    h1 = g_ref[...] * h1 / jnp.sqrt(1.0 + 1e-5) + bt_ref[...]
    o_ref[...] = jnp.maximum(h1, 0.0)


def _gin1(h, agg, eps, w1, b1, gamma, beta, br=_NBR):
    grid = (_N // br,)
    n2 = w1.shape[1]
    return pl.pallas_call(
        _gin1_body,
        grid=grid,
        in_specs=[
            pl.BlockSpec((br, _HID), lambda i: (i, 0)),
            pl.BlockSpec((br, _HID), lambda i: (i, 0)),
            pl.BlockSpec((1, 1), lambda i: (0, 0)),
            pl.BlockSpec((_HID, n2), lambda i: (0, 0)),
            pl.BlockSpec((1, n2), lambda i: (0, 0)),
            pl.BlockSpec((1, n2), lambda i: (0, 0)),
            pl.BlockSpec((1, n2), lambda i: (0, 0)),
        ],
        out_specs=pl.BlockSpec((br, n2), lambda i: (i, 0)),
        out_shape=jax.ShapeDtypeStruct((_N, n2), jnp.float32),
        interpret=_INTERPRET,
    )(h, agg, eps.reshape(1, 1), w1, b1.reshape(1, n2),
      gamma.reshape(1, n2), beta.reshape(1, n2))


_PBR = 1000


def _pool1_body(h_ref, w1_ref, b1_ref, w2_ref, bb2_ref, l_ref, m_ref):
    t = jnp.tanh(jnp.dot(h_ref[...], w1_ref[...],
                         preferred_element_type=jnp.float32) + b1_ref[...])
    logit = jnp.sum(t * w2_ref[...], axis=1, keepdims=True) + bb2_ref[0, 0]
    l_ref[...] = logit
    bm = jnp.max(logit, axis=0, keepdims=True)

    @pl.when(pl.program_id(0) == 0)
    def _():
        m_ref[...] = bm

    @pl.when(pl.program_id(0) != 0)
    def _():
        m_ref[...] = jnp.maximum(m_ref[...], bm)


def _pool2_body(h_ref, l_ref, g_ref, br_ref, bc_ref, s_ref, mx_ref, at_ref,
                cnt_ref, es_ref):
    h = h_ref[...]
    brow = br_ref[0]
    oh_t = (lax.broadcasted_iota(jnp.int32, (_G, _PBR), 0)
            == brow).astype(jnp.float32)
    ex = jnp.exp(l_ref[...] - g_ref[...])
    s_blk = jnp.dot(oh_t, h, preferred_element_type=jnp.float32)
    at_blk = jnp.dot(oh_t, h * ex, preferred_element_type=jnp.float32)
    cnt_blk = jnp.sum(oh_t, axis=1, keepdims=True)
    es_blk = jnp.sum(ex, axis=0, keepdims=True)

    first = pl.program_id(0) == 0

    @pl.when(first)
    def _():
        s_ref[...] = s_blk
        at_ref[...] = at_blk
        cnt_ref[...] = cnt_blk
        es_ref[...] = es_blk

    @pl.when(jnp.logical_not(first))
    def _():
        s_ref[...] = s_ref[...] + s_blk
        at_ref[...] = at_ref[...] + at_blk
        cnt_ref[...] = cnt_ref[...] + cnt_blk
        es_ref[...] = es_ref[...] + es_blk

    bcol = bc_ref[...]
    for g in range(_G):
        mg = jnp.max(jnp.where(bcol == g, h, -jnp.inf), axis=0, keepdims=True)

        @pl.when(first)
        def _():
            mx_ref[pl.ds(g, 1), :] = mg

        @pl.when(jnp.logical_not(first))
        def _():
            mx_ref[pl.ds(g, 1), :] = jnp.maximum(mx_ref[pl.ds(g, 1), :], mg)


def _pool3_body(s_ref, mx_ref, at_ref, cnt_ref, es_ref,
                ow1_ref, ob1_ref, ow2_ref, ob2_ref, o_ref):
    s = s_ref[...]
    cnt = cnt_ref[...]
    mean = s / jnp.maximum(cnt, 1.0)
    attn = at_ref[...] / es_ref[0, 0]
    pooled = jnp.concatenate([mean, mx_ref[...], s, attn], axis=1)
    o1 = jnp.dot(pooled, ow1_ref[...], preferred_element_type=jnp.float32)
    o1 = jnp.maximum(o1 + ob1_ref[...], 0.0)
    o_ref[...] = jnp.dot(o1, ow2_ref[...],
                         preferred_element_type=jnp.float32) + ob2_ref[...]


def _pool_head(h, batch_row, p):
    grid = (_N // _PBR,)
    nh = p["ap_W1"].shape[1]
    logits, gmax = pl.pallas_call(
        _pool1_body,
        grid=grid,
        in_specs=[
            pl.BlockSpec((_PBR, _HID), lambda i: (i, 0)),
            pl.BlockSpec((_HID, nh), lambda i: (0, 0)),
            pl.BlockSpec((1, nh), lambda i: (0, 0)),
            pl.BlockSpec((1, nh), lambda i: (0, 0)),
            pl.BlockSpec((1, 1), lambda i: (0, 0)),
        ],
        out_specs=[
            pl.BlockSpec((_PBR, 1), lambda i: (i, 0)),
            pl.BlockSpec((1, 1), lambda i: (0, 0)),
        ],
        out_shape=[
            jax.ShapeDtypeStruct((_N, 1), jnp.float32),
            jax.ShapeDtypeStruct((1, 1), jnp.float32),
        ],
        interpret=_INTERPRET,
    )(h, p["ap_W1"], p["ap_b1"].reshape(1, nh),
      p["ap_W2"].reshape(1, nh), p["ap_b2"].reshape(1, 1))

    s, mx, at, cnt, es = pl.pallas_call(
        _pool2_body,
        grid=grid,
        in_specs=[
            pl.BlockSpec((_PBR, _HID), lambda i: (i, 0)),
            pl.BlockSpec((_PBR, 1), lambda i: (i, 0)),
            pl.BlockSpec((1, 1), lambda i: (0, 0)),
            pl.BlockSpec((1, 1, _PBR), lambda i: (i, 0, 0)),
            pl.BlockSpec((_PBR, 1), lambda i: (i, 0)),
        ],
        out_specs=[
            pl.BlockSpec((_G, _HID), lambda i: (0, 0)),
            pl.BlockSpec((_G, _HID), lambda i: (0, 0)),
            pl.BlockSpec((_G, _HID), lambda i: (0, 0)),
            pl.BlockSpec((_G, 1), lambda i: (0, 0)),
            pl.BlockSpec((1, 1), lambda i: (0, 0)),
        ],
        out_shape=[
            jax.ShapeDtypeStruct((_G, _HID), jnp.float32),
            jax.ShapeDtypeStruct((_G, _HID), jnp.float32),
            jax.ShapeDtypeStruct((_G, _HID), jnp.float32),
            jax.ShapeDtypeStruct((_G, 1), jnp.float32),
            jax.ShapeDtypeStruct((1, 1), jnp.float32),
        ],
        interpret=_INTERPRET,
    )(h, logits, gmax, batch_row.reshape(_N // _PBR, 1, _PBR),
      batch_row.reshape(_N, 1))

    return pl.pallas_call(
        _pool3_body,
        out_shape=jax.ShapeDtypeStruct((_G, _HID), jnp.float32),
        interpret=_INTERPRET,
    )(s, mx, at, cnt, es,
      p["out_W1"], p["out_b1"].reshape(1, -1),
      p["out_W2"], p["out_b2"].reshape(1, -1))


def _gather_rows(tab, idx):
    return jnp.take(tab, idx, axis=0)


def _scatter_add_rows(rows, idx, n):
    return jax.ops.segment_sum(rows, idx, num_segments=n)


def kernel(x, edge_index, edge_attr, batch, params):
    src = edge_index[0]
    dst = edge_index[1]
    batch_row = batch.reshape(1, _N)

    h = _mm(x, params["node_W"], params["node_b"])
    ea = _mm(edge_attr, params["edge_W"], params["edge_b"], br=_EBR)

    for i in range(6):
        p = params["layer%d" % i]
        if i % 3 == 0:
            wq = jnp.concatenate([p["Wq"], p["Wk"], p["Wv"], p["Wskip"]], axis=1)
            bq = jnp.concatenate([p["bq"], p["bk"], p["bv"], p["bskip"]])
            qkvs = _mm(h, wq, bq)
            q = qkvs[:, :_HID]
            kk = qkvs[:, _HID:2 * _HID]
            vv = qkvs[:, 2 * _HID:3 * _HID]
            xs = qkvs[:, 3 * _HID:]
            e = _mm(ea, p["We"], p["be"], br=_EBR)
            qd = _gather_rows(q, dst)
            ks = _gather_rows(kk, src)
            a, bmax = _tconv_logits(qd, ks, e)
            vs = _gather_rows(vv, src)
            w, msg = _tconv_msg(a, bmax, vs, e)
            num = _scatter_add_rows(msg, dst, _N)
            den = _scatter_add_rows(w, dst, _N)
            wb = p["Wbeta"].reshape(3, _HID)
            wa_out = wb[0] + wb[2]
            wa_xs = wb[1] - wb[2]
            xa = _tconv_final(num, den, xs, wa_out, wa_xs)
            h2 = _graph_norm(h, xa, p["n_w"], p["n_b"], p["n_ms"])
            f1 = _mm(h2, p["f_W1"], p["f_b1"], act="gelu")
            x_new = _mm(f1, p["f_W2"], p["f_b2"], add=h2)
        elif i % 3 == 1:
            hh = _mm(h, p["W"], jnp.zeros((_HID,), jnp.float32))
            asrc, adst = _gat_node(hh, p["att_src"].reshape(_HID),
                                   p["att_dst"].reshape(_HID))
            e = _mm(ea, p["We"], jnp.zeros((_HID,), jnp.float32), br=_EBR)
            asg = _gather_rows(asrc, src)
            adg = _gather_rows(adst, dst)
            a, bmax = _gat_logits(asg, adg, e, p["att_edge"].reshape(_HID))
            hs = _gather_rows(hh, src)
            w, msg = _gat_msg(a, bmax, hs)
            num = _scatter_add_rows(msg, dst, _N)
            den = _scatter_add_rows(w, dst, _N)
            x_new = _gat_final(num, den, p["bias"])
        else:
            agg = _scatter_add_rows(_gather_rows(h, src), dst, _N)
            t = _gin1(h, agg, p["eps"], p["W1"], p["b1"],
                      p["gamma"], p["beta"])
            x_new = _mm(t, p["W2"], p["b2"])
        q = params["norm%d" % i]
        h = _graph_norm(h, x_new, q["w"], q["b"], q["ms"])

    return _pool_head(h, batch_row, params)


# speedup vs baseline: 1.6804x; 1.1417x over previous
import functools
import math

import jax
import jax.numpy as jnp
from jax import lax
from jax.experimental import pallas as pl
from jax.experimental.pallas import tpu as pltpu

_N = 10000
_E = 160000
_HID = 256
_H = 8
_C = 32
_G = 16

_INTERPRET = False

_EBR = 4000
_NBR = 2000


def _headsum(m):
    r = lax.broadcasted_iota(jnp.int32, (_HID, _H), 0) // _C
    c = lax.broadcasted_iota(jnp.int32, (_HID, _H), 1)
    s = (r == c).astype(jnp.float32)
    return jnp.dot(m, s, preferred_element_type=jnp.float32)


def _headexpand(w):
    r = lax.broadcasted_iota(jnp.int32, (_H, _HID), 0)
    c = lax.broadcasted_iota(jnp.int32, (_H, _HID), 1) // _C
    s = (r == c).astype(jnp.float32)
    return jnp.dot(w, s, preferred_element_type=jnp.float32)


def _mm_body(act, x_ref, w_ref, b_ref, o_ref):
    y = jnp.dot(x_ref[...], w_ref[...], preferred_element_type=jnp.float32)
    y = y + b_ref[...]
    if act == "relu":
        y = jnp.maximum(y, 0.0)
    elif act == "gelu":
        y = jax.nn.gelu(y)
    o_ref[...] = y


def _mm_add_body(act, x_ref, w_ref, b_ref, a_ref, o_ref):
    y = jnp.dot(x_ref[...], w_ref[...], preferred_element_type=jnp.float32)
    y = y + b_ref[...]
    if act == "relu":
        y = jnp.maximum(y, 0.0)
    elif act == "gelu":
        y = jax.nn.gelu(y)
    o_ref[...] = y + a_ref[...]


def _mm(x, w, b, act=None, br=_NBR, add=None):
    m, k = x.shape
    n = w.shape[1]
    grid = (m // br,)
    in_specs = [
        pl.BlockSpec((br, k), lambda i: (i, 0)),
        pl.BlockSpec((k, n), lambda i: (0, 0)),
        pl.BlockSpec((1, n), lambda i: (0, 0)),
    ]
    args = [x, w, b.reshape(1, n)]
    if add is None:
        body = functools.partial(_mm_body, act)
    else:
        body = functools.partial(_mm_add_body, act)
        in_specs.append(pl.BlockSpec((br, n), lambda i: (i, 0)))
        args.append(add)
    return pl.pallas_call(
        body,
        grid=grid,
        in_specs=in_specs,
        out_specs=pl.BlockSpec((br, n), lambda i: (i, 0)),
        out_shape=jax.ShapeDtypeStruct((m, n), jnp.float32),
        interpret=_INTERPRET,
    )(*args)


def _norm_body(x_ref, a_ref, w_ref, b_ref, ms_ref, o_ref):
    x = x_ref[...] + a_ref[...]
    mean = jnp.mean(x, axis=0, keepdims=True)
    out = x - ms_ref[...] * mean
    var = jnp.mean(out * out, axis=0, keepdims=True)
    o_ref[...] = w_ref[...] * out / jnp.sqrt(var + 1e-5) + b_ref[...]


def _graph_norm(x, add, w, b, ms):
    return pl.pallas_call(
        _norm_body,
        out_shape=jax.ShapeDtypeStruct((_N, _HID), jnp.float32),
        interpret=_INTERPRET,
    )(x, add, w.reshape(1, _HID), b.reshape(1, _HID), ms.reshape(1, _HID))


def _tc_logits_body(qd_ref, ks_ref, e_ref, a_ref, m_ref):
    m = qd_ref[...] * (ks_ref[...] + e_ref[...])
    a = _headsum(m) * (1.0 / math.sqrt(float(_C)))
    a_ref[...] = a
    bm = jnp.max(a, axis=0, keepdims=True)

    @pl.when(pl.program_id(0) == 0)
    def _():
        m_ref[...] = bm

    @pl.when(pl.program_id(0) != 0)
    def _():
        m_ref[...] = jnp.maximum(m_ref[...], bm)


def _tconv_logits(qd, kvg, e):
    grid = (_E // _EBR,)
    return pl.pallas_call(
        _tc_logits_body,
        grid=grid,
        in_specs=[
            pl.BlockSpec((_EBR, _HID), lambda i: (i, 0)),
            pl.BlockSpec((_EBR, _HID), lambda i: (i, 0)),
            pl.BlockSpec((_EBR, _HID), lambda i: (i, 0)),
        ],
        out_specs=[
            pl.BlockSpec((_EBR, _H), lambda i: (i, 0)),
            pl.BlockSpec((1, _H), lambda i: (0, 0)),
        ],
        out_shape=[
            jax.ShapeDtypeStruct((_E, _H), jnp.float32),
            jax.ShapeDtypeStruct((1, _H), jnp.float32),
        ],
        interpret=_INTERPRET,
    )(qd, kvg, e)


def _tc_msg_body(a_ref, g_ref, vs_ref, e_ref, o_ref):
    w = jnp.exp(a_ref[...] - g_ref[...])
    o_ref[:, :_HID] = (vs_ref[...] + e_ref[...]) * _headexpand(w)
    o_ref[:, _HID:] = w


def _tconv_msg(a, gmax, kvg, e):
    grid = (_E // _EBR,)
    return pl.pallas_call(
        _tc_msg_body,
        grid=grid,
        in_specs=[
            pl.BlockSpec((_EBR, _H), lambda i: (i, 0)),
            pl.BlockSpec((1, _H), lambda i: (0, 0)),
            pl.BlockSpec((_EBR, _HID), lambda i: (i, 1)),
            pl.BlockSpec((_EBR, _HID), lambda i: (i, 0)),
        ],
        out_specs=pl.BlockSpec((_EBR, _HID + _H), lambda i: (i, 0)),
        out_shape=jax.ShapeDtypeStruct((_E, _HID + _H), jnp.float32),
        interpret=_INTERPRET,
    )(a, gmax, kvg, e)


def _tc_final_body(seg_ref, xs_ref, wa_ref, wb_ref, o_ref):
    seg = seg_ref[...]
    denw = _headexpand(seg[:, _HID:])
    out = seg[:, :_HID] / (denw + 1e-16)
    xs = xs_ref[...]
    logit = jnp.sum(out * wa_ref[...] + xs * wb_ref[...], axis=1, keepdims=True)
    beta = jax.nn.sigmoid(logit)
    o_ref[...] = beta * xs + (1.0 - beta) * out


def _tconv_final(seg, xs, wa, wb):
    return pl.pallas_call(
        _tc_final_body,
        out_shape=jax.ShapeDtypeStruct((_N, _HID), jnp.float32),
        interpret=_INTERPRET,
    )(seg, xs, wa.reshape(1, _HID), wb.reshape(1, _HID))


def _gat_node_body(hh_ref, asf_ref, adf_ref, as_ref, ad_ref):
    hh = hh_ref[...]
    as_ref[...] = _headsum(hh * asf_ref[...])
    ad_ref[...] = _headsum(hh * adf_ref[...])


def _gat_node(hh, asf, adf):
    return pl.pallas_call(
        _gat_node_body,
        out_shape=[
            jax.ShapeDtypeStruct((_N, _H), jnp.float32),
            jax.ShapeDtypeStruct((_N, _H), jnp.float32),
        ],
        interpret=_INTERPRET,
    )(hh, asf.reshape(1, _HID), adf.reshape(1, _HID))


def _gat_logits_body(asg_ref, adg_ref, e_ref, aef_ref, a_ref, m_ref):
    ae = _headsum(e_ref[...] * aef_ref[...])
    a = asg_ref[...] + adg_ref[...] + ae
    a = jnp.where(a >= 0.0, a, 0.2 * a)
    a_ref[...] = a
    bm = jnp.max(a, axis=0, keepdims=True)

    @pl.when(pl.program_id(0) == 0)
    def _():
        m_ref[...] = bm

    @pl.when(pl.program_id(0) != 0)
    def _():
        m_ref[...] = jnp.maximum(m_ref[...], bm)


def _gat_logits(asg, adg, e, aef):
    grid = (_E // _EBR,)
    return pl.pallas_call(
        _gat_logits_body,
        grid=grid,
        in_specs=[
            pl.BlockSpec((_EBR, _H), lambda i: (i, 0)),
            pl.BlockSpec((_EBR, _H), lambda i: (i, 0)),
            pl.BlockSpec((_EBR, _HID), lambda i: (i, 0)),
            pl.BlockSpec((1, _HID), lambda i: (0, 0)),
        ],
        out_specs=[
            pl.BlockSpec((_EBR, _H), lambda i: (i, 0)),
            pl.BlockSpec((1, _H), lambda i: (0, 0)),
        ],
        out_shape=[
            jax.ShapeDtypeStruct((_E, _H), jnp.float32),
            jax.ShapeDtypeStruct((1, _H), jnp.float32),
        ],
        interpret=_INTERPRET,
    )(asg, adg, e, aef.reshape(1, _HID))


def _gat_msg_body(a_ref, g_ref, hs_ref, o_ref):
    w = jnp.exp(a_ref[...] - g_ref[...])
    o_ref[:, :_HID] = hs_ref[...] * _headexpand(w)
    o_ref[:, _HID:] = w


def _gat_msg(a, gmax, hs):
    grid = (_E // _EBR,)
    return pl.pallas_call(
        _gat_msg_body,
        grid=grid,
        in_specs=[
            pl.BlockSpec((_EBR, _H), lambda i: (i, 0)),
            pl.BlockSpec((1, _H), lambda i: (0, 0)),
            pl.BlockSpec((_EBR, _HID), lambda i: (i, 0)),
        ],
        out_specs=pl.BlockSpec((_EBR, _HID + _H), lambda i: (i, 0)),
        out_shape=jax.ShapeDtypeStruct((_E, _HID + _H), jnp.float32),
        interpret=_INTERPRET,
    )(a, gmax, hs)


def _gat_final_body(seg_ref, b_ref, o_ref):
    seg = seg_ref[...]
    denw = _headexpand(seg[:, _HID:])
    o_ref[...] = seg[:, :_HID] / (denw + 1e-16) + b_ref[...]


def _gat_final(seg, bias):
    return pl.pallas_call(
        _gat_final_body,
        out_shape=jax.ShapeDtypeStruct((_N, _HID), jnp.float32),
        interpret=_INTERPRET,
    )(seg, bias.reshape(1, _HID))


def _gin1_body(h_ref, agg_ref, eps_ref, w_ref, b_ref, g_ref, bt_ref, o_ref):
    h0 = (1.0 + eps_ref[0, 0]) * h_ref[...] + agg_ref[...]
    h1 = jnp.dot(h0, w_ref[...], preferred_element_type=jnp.float32) + b_ref[...]
    h1 = g_ref[...] * h1 / jnp.sqrt(1.0 + 1e-5) + bt_ref[...]
    o_ref[...] = jnp.maximum(h1, 0.0)


def _gin1(h, agg, eps, w1, b1, gamma, beta, br=_NBR):
    grid = (_N // br,)
    n2 = w1.shape[1]
    return pl.pallas_call(
        _gin1_body,
        grid=grid,
        in_specs=[
            pl.BlockSpec((br, _HID), lambda i: (i, 0)),
            pl.BlockSpec((br, _HID), lambda i: (i, 0)),
            pl.BlockSpec((1, 1), lambda i: (0, 0)),
            pl.BlockSpec((_HID, n2), lambda i: (0, 0)),
            pl.BlockSpec((1, n2), lambda i: (0, 0)),
            pl.BlockSpec((1, n2), lambda i: (0, 0)),
            pl.BlockSpec((1, n2), lambda i: (0, 0)),
        ],
        out_specs=pl.BlockSpec((br, n2), lambda i: (i, 0)),
        out_shape=jax.ShapeDtypeStruct((_N, n2), jnp.float32),
        interpret=_INTERPRET,
    )(h, agg, eps.reshape(1, 1), w1, b1.reshape(1, n2),
      gamma.reshape(1, n2), beta.reshape(1, n2))


_PBR = 1000


def _pool1_body(h_ref, w1_ref, b1_ref, w2_ref, bb2_ref, l_ref, m_ref):
    t = jnp.tanh(jnp.dot(h_ref[...], w1_ref[...],
                         preferred_element_type=jnp.float32) + b1_ref[...])
    logit = jnp.sum(t * w2_ref[...], axis=1, keepdims=True) + bb2_ref[0, 0]
    l_ref[...] = logit
    bm = jnp.max(logit, axis=0, keepdims=True)

    @pl.when(pl.program_id(0) == 0)
    def _():
        m_ref[...] = bm

    @pl.when(pl.program_id(0) != 0)
    def _():
        m_ref[...] = jnp.maximum(m_ref[...], bm)


def _pool2_body(h_ref, l_ref, g_ref, br_ref, bc_ref, s_ref, mx_ref, at_ref,
                cnt_ref, es_ref):
    h = h_ref[...]
    brow = br_ref[0]
    oh_t = (lax.broadcasted_iota(jnp.int32, (_G, _PBR), 0)
            == brow).astype(jnp.float32)
    ex = jnp.exp(l_ref[...] - g_ref[...])
    s_blk = jnp.dot(oh_t, h, preferred_element_type=jnp.float32)
    at_blk = jnp.dot(oh_t, h * ex, preferred_element_type=jnp.float32)
    cnt_blk = jnp.sum(oh_t, axis=1, keepdims=True)
    es_blk = jnp.sum(ex, axis=0, keepdims=True)

    first = pl.program_id(0) == 0

    @pl.when(first)
    def _():
        s_ref[...] = s_blk
        at_ref[...] = at_blk
        cnt_ref[...] = cnt_blk
        es_ref[...] = es_blk

    @pl.when(jnp.logical_not(first))
    def _():
        s_ref[...] = s_ref[...] + s_blk
        at_ref[...] = at_ref[...] + at_blk
        cnt_ref[...] = cnt_ref[...] + cnt_blk
        es_ref[...] = es_ref[...] + es_blk

    bcol = bc_ref[...]
    for g in range(_G):
        mg = jnp.max(jnp.where(bcol == g, h, -jnp.inf), axis=0, keepdims=True)

        @pl.when(first)
        def _():
            mx_ref[pl.ds(g, 1), :] = mg

        @pl.when(jnp.logical_not(first))
        def _():
            mx_ref[pl.ds(g, 1), :] = jnp.maximum(mx_ref[pl.ds(g, 1), :], mg)


def _pool3_body(s_ref, mx_ref, at_ref, cnt_ref, es_ref,
                ow1_ref, ob1_ref, ow2_ref, ob2_ref, o_ref):
    s = s_ref[...]
    cnt = cnt_ref[...]
    mean = s / jnp.maximum(cnt, 1.0)
    attn = at_ref[...] / es_ref[0, 0]
    pooled = jnp.concatenate([mean, mx_ref[...], s, attn], axis=1)
    o1 = jnp.dot(pooled, ow1_ref[...], preferred_element_type=jnp.float32)
    o1 = jnp.maximum(o1 + ob1_ref[...], 0.0)
    o_ref[...] = jnp.dot(o1, ow2_ref[...],
                         preferred_element_type=jnp.float32) + ob2_ref[...]


def _pool_head(h, batch_row, p):
    grid = (_N // _PBR,)
    nh = p["ap_W1"].shape[1]
    logits, gmax = pl.pallas_call(
        _pool1_body,
        grid=grid,
        in_specs=[
            pl.BlockSpec((_PBR, _HID), lambda i: (i, 0)),
            pl.BlockSpec((_HID, nh), lambda i: (0, 0)),
            pl.BlockSpec((1, nh), lambda i: (0, 0)),
            pl.BlockSpec((1, nh), lambda i: (0, 0)),
            pl.BlockSpec((1, 1), lambda i: (0, 0)),
        ],
        out_specs=[
            pl.BlockSpec((_PBR, 1), lambda i: (i, 0)),
            pl.BlockSpec((1, 1), lambda i: (0, 0)),
        ],
        out_shape=[
            jax.ShapeDtypeStruct((_N, 1), jnp.float32),
            jax.ShapeDtypeStruct((1, 1), jnp.float32),
        ],
        interpret=_INTERPRET,
    )(h, p["ap_W1"], p["ap_b1"].reshape(1, nh),
      p["ap_W2"].reshape(1, nh), p["ap_b2"].reshape(1, 1))

    s, mx, at, cnt, es = pl.pallas_call(
        _pool2_body,
        grid=grid,
        in_specs=[
            pl.BlockSpec((_PBR, _HID), lambda i: (i, 0)),
            pl.BlockSpec((_PBR, 1), lambda i: (i, 0)),
            pl.BlockSpec((1, 1), lambda i: (0, 0)),
            pl.BlockSpec((1, 1, _PBR), lambda i: (i, 0, 0)),
            pl.BlockSpec((_PBR, 1), lambda i: (i, 0)),
        ],
        out_specs=[
            pl.BlockSpec((_G, _HID), lambda i: (0, 0)),
            pl.BlockSpec((_G, _HID), lambda i: (0, 0)),
            pl.BlockSpec((_G, _HID), lambda i: (0, 0)),
            pl.BlockSpec((_G, 1), lambda i: (0, 0)),
            pl.BlockSpec((1, 1), lambda i: (0, 0)),
        ],
        out_shape=[
            jax.ShapeDtypeStruct((_G, _HID), jnp.float32),
            jax.ShapeDtypeStruct((_G, _HID), jnp.float32),
            jax.ShapeDtypeStruct((_G, _HID), jnp.float32),
            jax.ShapeDtypeStruct((_G, 1), jnp.float32),
            jax.ShapeDtypeStruct((1, 1), jnp.float32),
        ],
        interpret=_INTERPRET,
    )(h, logits, gmax, batch_row.reshape(_N // _PBR, 1, _PBR),
      batch_row.reshape(_N, 1))

    return pl.pallas_call(
        _pool3_body,
        out_shape=jax.ShapeDtypeStruct((_G, _HID), jnp.float32),
        interpret=_INTERPRET,
    )(s, mx, at, cnt, es,
      p["out_W1"], p["out_b1"].reshape(1, -1),
      p["out_W2"], p["out_b2"].reshape(1, -1))


def _gather_rows(tab, idx):
    return jnp.take(tab, idx, axis=0)


def _scatter_add_rows(rows, idx, n):
    return jax.ops.segment_sum(rows, idx, num_segments=n)


def kernel(x, edge_index, edge_attr, batch, params):
    src = edge_index[0]
    dst = edge_index[1]
    batch_row = batch.reshape(1, _N)

    h = _mm(x, params["node_W"], params["node_b"])
    ea = _mm(edge_attr, params["edge_W"], params["edge_b"], br=_EBR)

    for i in range(6):
        p = params["layer%d" % i]
        if i % 3 == 0:
            wq = jnp.concatenate([p["Wq"], p["Wk"], p["Wv"], p["Wskip"]], axis=1)
            bq = jnp.concatenate([p["bq"], p["bk"], p["bv"], p["bskip"]])
            qkvs = _mm(h, wq, bq)
            q = qkvs[:, :_HID]
            kv = qkvs[:, _HID:3 * _HID]
            xs = qkvs[:, 3 * _HID:]
            e = _mm(ea, p["We"], p["be"], br=_EBR)
            qd = _gather_rows(q, dst)
            kvg = _gather_rows(kv, src)
            a, bmax = _tconv_logits(qd, kvg, e)
            wmsg = _tconv_msg(a, bmax, kvg, e)
            seg = _scatter_add_rows(wmsg, dst, _N)
            wb = p["Wbeta"].reshape(3, _HID)
            wa_out = wb[0] + wb[2]
            wa_xs = wb[1] - wb[2]
            xa = _tconv_final(seg, xs, wa_out, wa_xs)
            h2 = _graph_norm(h, xa, p["n_w"], p["n_b"], p["n_ms"])
            f1 = _mm(h2, p["f_W1"], p["f_b1"], act="gelu")
            x_new = _mm(f1, p["f_W2"], p["f_b2"], add=h2)
        elif i % 3 == 1:
            hh = _mm(h, p["W"], jnp.zeros((_HID,), jnp.float32))
            asrc, adst = _gat_node(hh, p["att_src"].reshape(_HID),
                                   p["att_dst"].reshape(_HID))
            e = _mm(ea, p["We"], jnp.zeros((_HID,), jnp.float32), br=_EBR)
            asg = _gather_rows(asrc, src)
            adg = _gather_rows(adst, dst)
            a, bmax = _gat_logits(asg, adg, e, p["att_edge"].reshape(_HID))
            hs = _gather_rows(hh, src)
            wmsg = _gat_msg(a, bmax, hs)
            seg = _scatter_add_rows(wmsg, dst, _N)
            x_new = _gat_final(seg, p["bias"])
        else:
            agg = _scatter_add_rows(_gather_rows(h, src), dst, _N)
            t = _gin1(h, agg, p["eps"], p["W1"], p["b1"],
                      p["gamma"], p["beta"])
            x_new = _mm(t, p["W2"], p["b2"])
        q = params["norm%d" % i]
        h = _graph_norm(h, x_new, q["w"], q["b"], q["ms"])

    return _pool_head(h, batch_row, params)


# speedup vs baseline: 2.4689x; 1.4692x over previous
import functools
import math

import jax
import jax.numpy as jnp
from jax import lax
from jax.experimental import pallas as pl
from jax.experimental.pallas import tpu as pltpu
from jax.experimental.pallas import tpu_sc as plsc

_N = 10000
_E = 160000
_HID = 256
_H = 8
_C = 32
_G = 16

_INTERPRET = False

_EBR = 4000
_NBR = 2000


def _headsum(m):
    r = lax.broadcasted_iota(jnp.int32, (_HID, _H), 0) // _C
    c = lax.broadcasted_iota(jnp.int32, (_HID, _H), 1)
    s = (r == c).astype(jnp.float32)
    return jnp.dot(m, s, preferred_element_type=jnp.float32)


def _headexpand(w):
    r = lax.broadcasted_iota(jnp.int32, (_H, _HID), 0)
    c = lax.broadcasted_iota(jnp.int32, (_H, _HID), 1) // _C
    s = (r == c).astype(jnp.float32)
    return jnp.dot(w, s, preferred_element_type=jnp.float32)


def _mm_body(act, x_ref, w_ref, b_ref, o_ref):
    y = jnp.dot(x_ref[...], w_ref[...], preferred_element_type=jnp.float32)
    y = y + b_ref[...]
    if act == "relu":
        y = jnp.maximum(y, 0.0)
    elif act == "gelu":
        y = jax.nn.gelu(y)
    o_ref[...] = y


def _mm_add_body(act, x_ref, w_ref, b_ref, a_ref, o_ref):
    y = jnp.dot(x_ref[...], w_ref[...], preferred_element_type=jnp.float32)
    y = y + b_ref[...]
    if act == "relu":
        y = jnp.maximum(y, 0.0)
    elif act == "gelu":
        y = jax.nn.gelu(y)
    o_ref[...] = y + a_ref[...]


def _mm(x, w, b, act=None, br=_NBR, add=None):
    m, k = x.shape
    n = w.shape[1]
    grid = (m // br,)
    in_specs = [
        pl.BlockSpec((br, k), lambda i: (i, 0)),
        pl.BlockSpec((k, n), lambda i: (0, 0)),
        pl.BlockSpec((1, n), lambda i: (0, 0)),
    ]
    args = [x, w, b.reshape(1, n)]
    if add is None:
        body = functools.partial(_mm_body, act)
    else:
        body = functools.partial(_mm_add_body, act)
        in_specs.append(pl.BlockSpec((br, n), lambda i: (i, 0)))
        args.append(add)
    return pl.pallas_call(
        body,
        grid=grid,
        in_specs=in_specs,
        out_specs=pl.BlockSpec((br, n), lambda i: (i, 0)),
        out_shape=jax.ShapeDtypeStruct((m, n), jnp.float32),
        interpret=_INTERPRET,
    )(*args)


def _mm_qkvs_body(x_ref, w_ref, b_ref, q_ref, kv_ref, xs_ref):
    y = jnp.dot(x_ref[...], w_ref[...], preferred_element_type=jnp.float32)
    y = y + b_ref[...]
    q_ref[...] = y[:, :_HID]
    kv_ref[...] = y[:, _HID:3 * _HID]
    xs_ref[...] = y[:, 3 * _HID:]


def _mm_qkvs(x, w, b, br=_NBR):
    grid = (_N // br,)
    return pl.pallas_call(
        _mm_qkvs_body,
        grid=grid,
        in_specs=[
            pl.BlockSpec((br, _HID), lambda i: (i, 0)),
            pl.BlockSpec((_HID, 4 * _HID), lambda i: (0, 0)),
            pl.BlockSpec((1, 4 * _HID), lambda i: (0, 0)),
        ],
        out_specs=[
            pl.BlockSpec((br, _HID), lambda i: (i, 0)),
            pl.BlockSpec((br, 2 * _HID), lambda i: (i, 0)),
            pl.BlockSpec((br, _HID), lambda i: (i, 0)),
        ],
        out_shape=[
            jax.ShapeDtypeStruct((_N, _HID), jnp.float32),
            jax.ShapeDtypeStruct((_N, 2 * _HID), jnp.float32),
            jax.ShapeDtypeStruct((_N, _HID), jnp.float32),
        ],
        interpret=_INTERPRET,
    )(x, w, b.reshape(1, 4 * _HID))


def _norm_body(x_ref, a_ref, w_ref, b_ref, ms_ref, o_ref):
    x = x_ref[...] + a_ref[...]
    mean = jnp.mean(x, axis=0, keepdims=True)
    out = x - ms_ref[...] * mean
    var = jnp.mean(out * out, axis=0, keepdims=True)
    o_ref[...] = w_ref[...] * out / jnp.sqrt(var + 1e-5) + b_ref[...]


def _graph_norm(x, add, w, b, ms):
    return pl.pallas_call(
        _norm_body,
        out_shape=jax.ShapeDtypeStruct((_N, _HID), jnp.float32),
        interpret=_INTERPRET,
    )(x, add, w.reshape(1, _HID), b.reshape(1, _HID), ms.reshape(1, _HID))


def _tc_logits_body(qd_ref, ks_ref, e_ref, a_ref, m_ref):
    m = qd_ref[...] * (ks_ref[...] + e_ref[...])
    a = _headsum(m) * (1.0 / math.sqrt(float(_C)))
    a_ref[...] = a
    bm = jnp.max(a, axis=0, keepdims=True)

    @pl.when(pl.program_id(0) == 0)
    def _():
        m_ref[...] = bm

    @pl.when(pl.program_id(0) != 0)
    def _():
        m_ref[...] = jnp.maximum(m_ref[...], bm)


def _tconv_logits(qd, kvg, e):
    grid = (_E // _EBR,)
    return pl.pallas_call(
        _tc_logits_body,
        grid=grid,
        in_specs=[
            pl.BlockSpec((_EBR, _HID), lambda i: (i, 0)),
            pl.BlockSpec((_EBR, _HID), lambda i: (i, 0)),
            pl.BlockSpec((_EBR, _HID), lambda i: (i, 0)),
        ],
        out_specs=[
            pl.BlockSpec((_EBR, _H), lambda i: (i, 0)),
            pl.BlockSpec((1, _H), lambda i: (0, 0)),
        ],
        out_shape=[
            jax.ShapeDtypeStruct((_E, _H), jnp.float32),
            jax.ShapeDtypeStruct((1, _H), jnp.float32),
        ],
        interpret=_INTERPRET,
    )(qd, kvg, e)


def _tc_msg_body(a_ref, g_ref, vs_ref, e_ref, o_ref):
    w = jnp.exp(a_ref[...] - g_ref[...])
    o_ref[:, :_HID] = (vs_ref[...] + e_ref[...]) * _headexpand(w)
    o_ref[:, _HID:] = w


def _tconv_msg(a, gmax, kvg, e):
    grid = (_E // _EBR,)
    return pl.pallas_call(
        _tc_msg_body,
        grid=grid,
        in_specs=[
            pl.BlockSpec((_EBR, _H), lambda i: (i, 0)),
            pl.BlockSpec((1, _H), lambda i: (0, 0)),
            pl.BlockSpec((_EBR, _HID), lambda i: (i, 1)),
            pl.BlockSpec((_EBR, _HID), lambda i: (i, 0)),
        ],
        out_specs=pl.BlockSpec((_EBR, _HID + _H), lambda i: (i, 0)),
        out_shape=jax.ShapeDtypeStruct((_E, _HID + _H), jnp.float32),
        interpret=_INTERPRET,
    )(a, gmax, kvg, e)


def _tc_final_body(seg_ref, xs_ref, wa_ref, wb_ref, o_ref):
    seg = seg_ref[...]
    denw = _headexpand(seg[:, _HID:])
    out = seg[:, :_HID] / (denw + 1e-16)
    xs = xs_ref[...]
    logit = jnp.sum(out * wa_ref[...] + xs * wb_ref[...], axis=1, keepdims=True)
    beta = jax.nn.sigmoid(logit)
    o_ref[...] = beta * xs + (1.0 - beta) * out


def _tconv_final(seg, xs, wa, wb):
    return pl.pallas_call(
        _tc_final_body,
        out_shape=jax.ShapeDtypeStruct((_N, _HID), jnp.float32),
        interpret=_INTERPRET,
    )(seg, xs, wa.reshape(1, _HID), wb.reshape(1, _HID))


def _gat_node_body(hh_ref, adf_ref, ad_ref):
    ad_ref[...] = _headsum(hh_ref[...] * adf_ref[...])


def _gat_node(hh, adf):
    return pl.pallas_call(
        _gat_node_body,
        out_shape=jax.ShapeDtypeStruct((_N, _H), jnp.float32),
        interpret=_INTERPRET,
    )(hh, adf.reshape(1, _HID))


def _gat_logits_body(hs_ref, adg_ref, e_ref, asf_ref, aef_ref, a_ref, m_ref):
    asg = _headsum(hs_ref[...] * asf_ref[...])
    ae = _headsum(e_ref[...] * aef_ref[...])
    a = asg + adg_ref[...] + ae
    a = jnp.where(a >= 0.0, a, 0.2 * a)
    a_ref[...] = a
    bm = jnp.max(a, axis=0, keepdims=True)

    @pl.when(pl.program_id(0) == 0)
    def _():
        m_ref[...] = bm

    @pl.when(pl.program_id(0) != 0)
    def _():
        m_ref[...] = jnp.maximum(m_ref[...], bm)


def _gat_logits(hs, adg, e, asf, aef):
    grid = (_E // _EBR,)
    return pl.pallas_call(
        _gat_logits_body,
        grid=grid,
        in_specs=[
            pl.BlockSpec((_EBR, _HID), lambda i: (i, 0)),
            pl.BlockSpec((_EBR, _H), lambda i: (i, 0)),
            pl.BlockSpec((_EBR, _HID), lambda i: (i, 0)),
            pl.BlockSpec((1, _HID), lambda i: (0, 0)),
            pl.BlockSpec((1, _HID), lambda i: (0, 0)),
        ],
        out_specs=[
            pl.BlockSpec((_EBR, _H), lambda i: (i, 0)),
            pl.BlockSpec((1, _H), lambda i: (0, 0)),
        ],
        out_shape=[
            jax.ShapeDtypeStruct((_E, _H), jnp.float32),
            jax.ShapeDtypeStruct((1, _H), jnp.float32),
        ],
        interpret=_INTERPRET,
    )(hs, adg, e, asf.reshape(1, _HID), aef.reshape(1, _HID))


def _gat_msg_body(a_ref, g_ref, hs_ref, o_ref):
    w = jnp.exp(a_ref[...] - g_ref[...])
    o_ref[:, :_HID] = hs_ref[...] * _headexpand(w)
    o_ref[:, _HID:] = w


def _gat_msg(a, gmax, hs):
    grid = (_E // _EBR,)
    return pl.pallas_call(
        _gat_msg_body,
        grid=grid,
        in_specs=[
            pl.BlockSpec((_EBR, _H), lambda i: (i, 0)),
            pl.BlockSpec((1, _H), lambda i: (0, 0)),
            pl.BlockSpec((_EBR, _HID), lambda i: (i, 0)),
        ],
        out_specs=pl.BlockSpec((_EBR, _HID + _H), lambda i: (i, 0)),
        out_shape=jax.ShapeDtypeStruct((_E, _HID + _H), jnp.float32),
        interpret=_INTERPRET,
    )(a, gmax, hs)


def _gat_final_body(seg_ref, b_ref, o_ref):
    seg = seg_ref[...]
    denw = _headexpand(seg[:, _HID:])
    o_ref[...] = seg[:, :_HID] / (denw + 1e-16) + b_ref[...]


def _gat_final(seg, bias):
    return pl.pallas_call(
        _gat_final_body,
        out_shape=jax.ShapeDtypeStruct((_N, _HID), jnp.float32),
        interpret=_INTERPRET,
    )(seg, bias.reshape(1, _HID))


def _gin1_body(h_ref, agg_ref, eps_ref, w_ref, b_ref, g_ref, bt_ref, o_ref):
    h0 = (1.0 + eps_ref[0, 0]) * h_ref[...] + agg_ref[...]
    h1 = jnp.dot(h0, w_ref[...], preferred_element_type=jnp.float32) + b_ref[...]
    h1 = g_ref[...] * h1 / jnp.sqrt(1.0 + 1e-5) + bt_ref[...]
    o_ref[...] = jnp.maximum(h1, 0.0)


def _gin1(h, agg, eps, w1, b1, gamma, beta, br=_NBR):
    grid = (_N // br,)
    n2 = w1.shape[1]
    return pl.pallas_call(
        _gin1_body,
        grid=grid,
        in_specs=[
            pl.BlockSpec((br, _HID), lambda i: (i, 0)),
            pl.BlockSpec((br, _HID), lambda i: (i, 0)),
            pl.BlockSpec((1, 1), lambda i: (0, 0)),
            pl.BlockSpec((_HID, n2), lambda i: (0, 0)),
            pl.BlockSpec((1, n2), lambda i: (0, 0)),
            pl.BlockSpec((1, n2), lambda i: (0, 0)),
            pl.BlockSpec((1, n2), lambda i: (0, 0)),
        ],
        out_specs=pl.BlockSpec((br, n2), lambda i: (i, 0)),
        out_shape=jax.ShapeDtypeStruct((_N, n2), jnp.float32),
        interpret=_INTERPRET,
    )(h, agg, eps.reshape(1, 1), w1, b1.reshape(1, n2),
      gamma.reshape(1, n2), beta.reshape(1, n2))


_PBR = 1000


def _pool1_body(h_ref, w1_ref, b1_ref, w2_ref, bb2_ref, l_ref, m_ref):
    t = jnp.tanh(jnp.dot(h_ref[...], w1_ref[...],
                         preferred_element_type=jnp.float32) + b1_ref[...])
    logit = jnp.sum(t * w2_ref[...], axis=1, keepdims=True) + bb2_ref[0, 0]
    l_ref[...] = logit
    bm = jnp.max(logit, axis=0, keepdims=True)

    @pl.when(pl.program_id(0) == 0)
    def _():
        m_ref[...] = bm

    @pl.when(pl.program_id(0) != 0)
    def _():
        m_ref[...] = jnp.maximum(m_ref[...], bm)


def _pool2_body(h_ref, l_ref, g_ref, br_ref, bc_ref, s_ref, mx_ref, at_ref,
                cnt_ref, es_ref):
    h = h_ref[...]
    brow = br_ref[0]
    oh_t = (lax.broadcasted_iota(jnp.int32, (_G, _PBR), 0)
            == brow).astype(jnp.float32)
    ex = jnp.exp(l_ref[...] - g_ref[...])
    s_blk = jnp.dot(oh_t, h, preferred_element_type=jnp.float32)
    at_blk = jnp.dot(oh_t, h * ex, preferred_element_type=jnp.float32)
    cnt_blk = jnp.sum(oh_t, axis=1, keepdims=True)
    es_blk = jnp.sum(ex, axis=0, keepdims=True)

    first = pl.program_id(0) == 0

    @pl.when(first)
    def _():
        s_ref[...] = s_blk
        at_ref[...] = at_blk
        cnt_ref[...] = cnt_blk
        es_ref[...] = es_blk

    @pl.when(jnp.logical_not(first))
    def _():
        s_ref[...] = s_ref[...] + s_blk
        at_ref[...] = at_ref[...] + at_blk
        cnt_ref[...] = cnt_ref[...] + cnt_blk
        es_ref[...] = es_ref[...] + es_blk

    bcol = bc_ref[...]
    for g in range(_G):
        mg = jnp.max(jnp.where(bcol == g, h, -jnp.inf), axis=0, keepdims=True)

        @pl.when(first)
        def _():
            mx_ref[pl.ds(g, 1), :] = mg

        @pl.when(jnp.logical_not(first))
        def _():
            mx_ref[pl.ds(g, 1), :] = jnp.maximum(mx_ref[pl.ds(g, 1), :], mg)


def _pool3_body(s_ref, mx_ref, at_ref, cnt_ref, es_ref,
                ow1_ref, ob1_ref, ow2_ref, ob2_ref, o_ref):
    s = s_ref[...]
    cnt = cnt_ref[...]
    mean = s / jnp.maximum(cnt, 1.0)
    attn = at_ref[...] / es_ref[0, 0]
    pooled = jnp.concatenate([mean, mx_ref[...], s, attn], axis=1)
    o1 = jnp.dot(pooled, ow1_ref[...], preferred_element_type=jnp.float32)
    o1 = jnp.maximum(o1 + ob1_ref[...], 0.0)
    o_ref[...] = jnp.dot(o1, ow2_ref[...],
                         preferred_element_type=jnp.float32) + ob2_ref[...]


def _pool_head(h, batch_row, p):
    grid = (_N // _PBR,)
    nh = p["ap_W1"].shape[1]
    logits, gmax = pl.pallas_call(
        _pool1_body,
        grid=grid,
        in_specs=[
            pl.BlockSpec((_PBR, _HID), lambda i: (i, 0)),
            pl.BlockSpec((_HID, nh), lambda i: (0, 0)),
            pl.BlockSpec((1, nh), lambda i: (0, 0)),
            pl.BlockSpec((1, nh), lambda i: (0, 0)),
            pl.BlockSpec((1, 1), lambda i: (0, 0)),
        ],
        out_specs=[
            pl.BlockSpec((_PBR, 1), lambda i: (i, 0)),
            pl.BlockSpec((1, 1), lambda i: (0, 0)),
        ],
        out_shape=[
            jax.ShapeDtypeStruct((_N, 1), jnp.float32),
            jax.ShapeDtypeStruct((1, 1), jnp.float32),
        ],
        interpret=_INTERPRET,
    )(h, p["ap_W1"], p["ap_b1"].reshape(1, nh),
      p["ap_W2"].reshape(1, nh), p["ap_b2"].reshape(1, 1))

    s, mx, at, cnt, es = pl.pallas_call(
        _pool2_body,
        grid=grid,
        in_specs=[
            pl.BlockSpec((_PBR, _HID), lambda i: (i, 0)),
            pl.BlockSpec((_PBR, 1), lambda i: (i, 0)),
            pl.BlockSpec((1, 1), lambda i: (0, 0)),
            pl.BlockSpec((1, 1, _PBR), lambda i: (i, 0, 0)),
            pl.BlockSpec((_PBR, 1), lambda i: (i, 0)),
        ],
        out_specs=[
            pl.BlockSpec((_G, _HID), lambda i: (0, 0)),
            pl.BlockSpec((_G, _HID), lambda i: (0, 0)),
            pl.BlockSpec((_G, _HID), lambda i: (0, 0)),
            pl.BlockSpec((_G, 1), lambda i: (0, 0)),
            pl.BlockSpec((1, 1), lambda i: (0, 0)),
        ],
        out_shape=[
            jax.ShapeDtypeStruct((_G, _HID), jnp.float32),
            jax.ShapeDtypeStruct((_G, _HID), jnp.float32),
            jax.ShapeDtypeStruct((_G, _HID), jnp.float32),
            jax.ShapeDtypeStruct((_G, 1), jnp.float32),
            jax.ShapeDtypeStruct((1, 1), jnp.float32),
        ],
        interpret=_INTERPRET,
    )(h, logits, gmax, batch_row.reshape(_N // _PBR, 1, _PBR),
      batch_row.reshape(_N, 1))

    return pl.pallas_call(
        _pool3_body,
        out_shape=jax.ShapeDtypeStruct((_G, _HID), jnp.float32),
        interpret=_INTERPRET,
    )(s, mx, at, cnt, es,
      p["out_W1"], p["out_b1"].reshape(1, -1),
      p["out_W2"], p["out_b2"].reshape(1, -1))


_NW = 32
_BPW = _E // _NW


def _sc_gather_fn(d, ch):
    n_full, rem = divmod(_BPW, ch)
    sizes = [ch] * n_full + ([rem] if rem else [])
    offs = [i * ch for i in range(len(sizes))]
    mesh = plsc.VectorSubcoreMesh(core_axis_name="c", subcore_axis_name="s",
                                  num_cores=2)

    @functools.partial(
        pl.kernel,
        mesh=mesh,
        out_type=jax.ShapeDtypeStruct((_E, d), jnp.float32),
        scratch_types=[
            pltpu.VMEM((_BPW,), jnp.int32),
            pltpu.VMEM((ch, d), jnp.float32),
            pltpu.VMEM((ch, d), jnp.float32),
            pltpu.SemaphoreType.DMA,
            pltpu.SemaphoreType.DMA,
        ],
    )
    def k(tab_hbm, idx_hbm, out_hbm, idx_v, buf0, buf1, sem0, sem1):
        wid = lax.axis_index("s") * 2 + lax.axis_index("c")
        base = wid * _BPW
        pltpu.sync_copy(idx_hbm.at[pl.ds(base, _BPW)], idx_v)
        bufs = (buf0, buf1)
        sems = (sem0, sem1)
        cps = [None, None]
        cps[0] = pltpu.async_copy(
            tab_hbm.at[idx_v.at[pl.ds(0, sizes[0])]],
            bufs[0].at[pl.ds(0, sizes[0])], sems[0])
        for i in range(len(sizes)):
            if i + 1 < len(sizes):
                cps[(i + 1) % 2] = pltpu.async_copy(
                    tab_hbm.at[idx_v.at[pl.ds(offs[i + 1], sizes[i + 1])]],
                    bufs[(i + 1) % 2].at[pl.ds(0, sizes[i + 1])],
                    sems[(i + 1) % 2])
            cps[i % 2].wait()
            pltpu.sync_copy(bufs[i % 2].at[pl.ds(0, sizes[i])],
                            out_hbm.at[pl.ds(base + offs[i], sizes[i])])

    return k


_SC_GATHER_CACHE = {}
_GATHER_CHUNK = {256: 200, 512: 96}


def _gather_rows(tab, idx):
    d = tab.shape[1]
    if d not in _GATHER_CHUNK:
        return jnp.take(tab, idx, axis=0)
    if d not in _SC_GATHER_CACHE:
        _SC_GATHER_CACHE[d] = _sc_gather_fn(d, _GATHER_CHUNK[d])
    return _SC_GATHER_CACHE[d](tab, idx)


def _scatter_add_rows(rows, idx, n):
    return jax.ops.segment_sum(rows, idx, num_segments=n)


def kernel(x, edge_index, edge_attr, batch, params):
    src = edge_index[0]
    dst = edge_index[1]
    batch_row = batch.reshape(1, _N)

    h = _mm(x, params["node_W"], params["node_b"])
    ea = _mm(edge_attr, params["edge_W"], params["edge_b"], br=_EBR)

    for i in range(6):
        p = params["layer%d" % i]
        if i % 3 == 0:
            wq = jnp.concatenate([p["Wq"], p["Wk"], p["Wv"], p["Wskip"]], axis=1)
            bq = jnp.concatenate([p["bq"], p["bk"], p["bv"], p["bskip"]])
            q, kv, xs = _mm_qkvs(h, wq, bq)
            e = _mm(ea, p["We"], p["be"], br=_EBR)
            qd = _gather_rows(q, dst)
            kvg = _gather_rows(kv, src)
            a, bmax = _tconv_logits(qd, kvg, e)
            wmsg = _tconv_msg(a, bmax, kvg, e)
            seg = _scatter_add_rows(wmsg, dst, _N)
            wb = p["Wbeta"].reshape(3, _HID)
            wa_out = wb[0] + wb[2]
            wa_xs = wb[1] - wb[2]
            xa = _tconv_final(seg, xs, wa_out, wa_xs)
            h2 = _graph_norm(h, xa, p["n_w"], p["n_b"], p["n_ms"])
            f1 = _mm(h2, p["f_W1"], p["f_b1"], act="gelu")
            x_new = _mm(f1, p["f_W2"], p["f_b2"], add=h2)
        elif i % 3 == 1:
            hh = _mm(h, p["W"], jnp.zeros((_HID,), jnp.float32))
            adst = _gat_node(hh, p["att_dst"].reshape(_HID))
            e = _mm(ea, p["We"], jnp.zeros((_HID,), jnp.float32), br=_EBR)
            adg = _gather_rows(adst, dst)
            hs = _gather_rows(hh, src)
            a, bmax = _gat_logits(hs, adg, e, p["att_src"].reshape(_HID),
                                  p["att_edge"].reshape(_HID))
            wmsg = _gat_msg(a, bmax, hs)
            seg = _scatter_add_rows(wmsg, dst, _N)
            x_new = _gat_final(seg, p["bias"])
        else:
            agg = _scatter_add_rows(_gather_rows(h, src), dst, _N)
            t = _gin1(h, agg, p["eps"], p["W1"], p["b1"],
                      p["gamma"], p["beta"])
            x_new = _mm(t, p["W2"], p["b2"])
        q = params["norm%d" % i]
        h = _graph_norm(h, x_new, q["w"], q["b"], q["ms"])

    return _pool_head(h, batch_row, params)


# speedup vs baseline: 3.0620x; 1.2402x over previous
import functools
import math

import jax
import jax.numpy as jnp
from jax import lax
from jax.experimental import pallas as pl
from jax.experimental.pallas import tpu as pltpu
from jax.experimental.pallas import tpu_sc as plsc

_N = 10000
_E = 160000
_HID = 256
_H = 8
_C = 32
_G = 16

_INTERPRET = False

_EBR = 4000
_NBR = 2000


def _headsum(m):
    r = lax.broadcasted_iota(jnp.int32, (_HID, _H), 0) // _C
    c = lax.broadcasted_iota(jnp.int32, (_HID, _H), 1)
    s = (r == c).astype(jnp.float32)
    return jnp.dot(m, s, preferred_element_type=jnp.float32)


def _headexpand(w):
    r = lax.broadcasted_iota(jnp.int32, (_H, _HID), 0)
    c = lax.broadcasted_iota(jnp.int32, (_H, _HID), 1) // _C
    s = (r == c).astype(jnp.float32)
    return jnp.dot(w, s, preferred_element_type=jnp.float32)


def _mm_body(act, x_ref, w_ref, b_ref, o_ref):
    y = jnp.dot(x_ref[...], w_ref[...], preferred_element_type=jnp.float32)
    y = y + b_ref[...]
    if act == "relu":
        y = jnp.maximum(y, 0.0)
    elif act == "gelu":
        y = jax.nn.gelu(y)
    o_ref[...] = y


def _mm_add_body(act, x_ref, w_ref, b_ref, a_ref, o_ref):
    y = jnp.dot(x_ref[...], w_ref[...], preferred_element_type=jnp.float32)
    y = y + b_ref[...]
    if act == "relu":
        y = jnp.maximum(y, 0.0)
    elif act == "gelu":
        y = jax.nn.gelu(y)
    o_ref[...] = y + a_ref[...]


def _mm(x, w, b, act=None, br=_NBR, add=None):
    m, k = x.shape
    n = w.shape[1]
    grid = (m // br,)
    in_specs = [
        pl.BlockSpec((br, k), lambda i: (i, 0)),
        pl.BlockSpec((k, n), lambda i: (0, 0)),
        pl.BlockSpec((1, n), lambda i: (0, 0)),
    ]
    args = [x, w, b.reshape(1, n)]
    if add is None:
        body = functools.partial(_mm_body, act)
    else:
        body = functools.partial(_mm_add_body, act)
        in_specs.append(pl.BlockSpec((br, n), lambda i: (i, 0)))
        args.append(add)
    return pl.pallas_call(
        body,
        grid=grid,
        in_specs=in_specs,
        out_specs=pl.BlockSpec((br, n), lambda i: (i, 0)),
        out_shape=jax.ShapeDtypeStruct((m, n), jnp.float32),
        interpret=_INTERPRET,
    )(*args)


def _mm_qkvs_body(x_ref, w_ref, b_ref, q_ref, kv_ref, xs_ref):
    y = jnp.dot(x_ref[...], w_ref[...], preferred_element_type=jnp.float32)
    y = y + b_ref[...]
    q_ref[...] = y[:, :_HID]
    kv_ref[...] = y[:, _HID:3 * _HID]
    xs_ref[...] = y[:, 3 * _HID:]


def _mm_qkvs(x, w, b, br=_NBR):
    grid = (_N // br,)
    return pl.pallas_call(
        _mm_qkvs_body,
        grid=grid,
        in_specs=[
            pl.BlockSpec((br, _HID), lambda i: (i, 0)),
            pl.BlockSpec((_HID, 4 * _HID), lambda i: (0, 0)),
            pl.BlockSpec((1, 4 * _HID), lambda i: (0, 0)),
        ],
        out_specs=[
            pl.BlockSpec((br, _HID), lambda i: (i, 0)),
            pl.BlockSpec((br, 2 * _HID), lambda i: (i, 0)),
            pl.BlockSpec((br, _HID), lambda i: (i, 0)),
        ],
        out_shape=[
            jax.ShapeDtypeStruct((_N, _HID), jnp.float32),
            jax.ShapeDtypeStruct((_N, 2 * _HID), jnp.float32),
            jax.ShapeDtypeStruct((_N, _HID), jnp.float32),
        ],
        interpret=_INTERPRET,
    )(x, w, b.reshape(1, 4 * _HID))


def _norm_body(x_ref, a_ref, w_ref, b_ref, ms_ref, o_ref):
    x = x_ref[...] + a_ref[...]
    mean = jnp.mean(x, axis=0, keepdims=True)
    out = x - ms_ref[...] * mean
    var = jnp.mean(out * out, axis=0, keepdims=True)
    o_ref[...] = w_ref[...] * out / jnp.sqrt(var + 1e-5) + b_ref[...]


def _graph_norm(x, add, w, b, ms):
    return pl.pallas_call(
        _norm_body,
        out_shape=jax.ShapeDtypeStruct((_N, _HID), jnp.float32),
        interpret=_INTERPRET,
    )(x, add, w.reshape(1, _HID), b.reshape(1, _HID), ms.reshape(1, _HID))


def _tc_logits_body(qd_ref, ks_ref, e_ref, a_ref, m_ref):
    m = qd_ref[...] * (ks_ref[...] + e_ref[...])
    a = _headsum(m) * (1.0 / math.sqrt(float(_C)))
    a_ref[...] = a
    bm = jnp.max(a, axis=0, keepdims=True)

    @pl.when(pl.program_id(0) == 0)
    def _():
        m_ref[...] = bm

    @pl.when(pl.program_id(0) != 0)
    def _():
        m_ref[...] = jnp.maximum(m_ref[...], bm)


def _tconv_logits(qd, kvg, e):
    grid = (_E // _EBR,)
    return pl.pallas_call(
        _tc_logits_body,
        grid=grid,
        in_specs=[
            pl.BlockSpec((_EBR, _HID), lambda i: (i, 0)),
            pl.BlockSpec((_EBR, _HID), lambda i: (i, 0)),
            pl.BlockSpec((_EBR, _HID), lambda i: (i, 0)),
        ],
        out_specs=[
            pl.BlockSpec((_EBR, _H), lambda i: (i, 0)),
            pl.BlockSpec((1, _H), lambda i: (0, 0)),
        ],
        out_shape=[
            jax.ShapeDtypeStruct((_E, _H), jnp.float32),
            jax.ShapeDtypeStruct((1, _H), jnp.float32),
        ],
        interpret=_INTERPRET,
    )(qd, kvg, e)


def _tc_msg_body(a_ref, g_ref, vs_ref, e_ref, o_ref):
    w = jnp.exp(a_ref[...] - g_ref[...])
    o_ref[:, :_HID] = (vs_ref[...] + e_ref[...]) * _headexpand(w)
    o_ref[:, _HID:] = w


def _tconv_msg(a, gmax, kvg, e):
    grid = (_E // _EBR,)
    return pl.pallas_call(
        _tc_msg_body,
        grid=grid,
        in_specs=[
            pl.BlockSpec((_EBR, _H), lambda i: (i, 0)),
            pl.BlockSpec((1, _H), lambda i: (0, 0)),
            pl.BlockSpec((_EBR, _HID), lambda i: (i, 1)),
            pl.BlockSpec((_EBR, _HID), lambda i: (i, 0)),
        ],
        out_specs=pl.BlockSpec((_EBR, _HID + _H), lambda i: (i, 0)),
        out_shape=jax.ShapeDtypeStruct((_E, _HID + _H), jnp.float32),
        interpret=_INTERPRET,
    )(a, gmax, kvg, e)


def _tc_final_body(num_ref, den_ref, xs_ref, wa_ref, wb_ref, o_ref):
    denw = _headexpand(den_ref[...])
    out = num_ref[...] / (denw + 1e-16)
    xs = xs_ref[...]
    logit = jnp.sum(out * wa_ref[...] + xs * wb_ref[...], axis=1, keepdims=True)
    beta = jax.nn.sigmoid(logit)
    o_ref[...] = beta * xs + (1.0 - beta) * out


def _tconv_final(num, den, xs, wa, wb):
    return pl.pallas_call(
        _tc_final_body,
        out_shape=jax.ShapeDtypeStruct((_N, _HID), jnp.float32),
        interpret=_INTERPRET,
    )(num, den, xs, wa.reshape(1, _HID), wb.reshape(1, _HID))


def _gat_node_body(hh_ref, adf_ref, ad_ref):
    ad_ref[...] = _headsum(hh_ref[...] * adf_ref[...])


def _gat_node(hh, adf):
    return pl.pallas_call(
        _gat_node_body,
        out_shape=jax.ShapeDtypeStruct((_N, _H), jnp.float32),
        interpret=_INTERPRET,
    )(hh, adf.reshape(1, _HID))


def _gat_logits_body(hs_ref, adg_ref, e_ref, asf_ref, aef_ref, a_ref, m_ref):
    asg = _headsum(hs_ref[...] * asf_ref[...])
    ae = _headsum(e_ref[...] * aef_ref[...])
    a = asg + adg_ref[...] + ae
    a = jnp.where(a >= 0.0, a, 0.2 * a)
    a_ref[...] = a
    bm = jnp.max(a, axis=0, keepdims=True)

    @pl.when(pl.program_id(0) == 0)
    def _():
        m_ref[...] = bm

    @pl.when(pl.program_id(0) != 0)
    def _():
        m_ref[...] = jnp.maximum(m_ref[...], bm)


def _gat_logits(hs, adg, e, asf, aef):
    grid = (_E // _EBR,)
    return pl.pallas_call(
        _gat_logits_body,
        grid=grid,
        in_specs=[
            pl.BlockSpec((_EBR, _HID), lambda i: (i, 0)),
            pl.BlockSpec((_EBR, _H), lambda i: (i, 0)),
            pl.BlockSpec((_EBR, _HID), lambda i: (i, 0)),
            pl.BlockSpec((1, _HID), lambda i: (0, 0)),
            pl.BlockSpec((1, _HID), lambda i: (0, 0)),
        ],
        out_specs=[
            pl.BlockSpec((_EBR, _H), lambda i: (i, 0)),
            pl.BlockSpec((1, _H), lambda i: (0, 0)),
        ],
        out_shape=[
            jax.ShapeDtypeStruct((_E, _H), jnp.float32),
            jax.ShapeDtypeStruct((1, _H), jnp.float32),
        ],
        interpret=_INTERPRET,
    )(hs, adg, e, asf.reshape(1, _HID), aef.reshape(1, _HID))


def _gat_msg_body(a_ref, g_ref, hs_ref, o_ref):
    w = jnp.exp(a_ref[...] - g_ref[...])
    o_ref[:, :_HID] = hs_ref[...] * _headexpand(w)
    o_ref[:, _HID:] = w


def _gat_msg(a, gmax, hs):
    grid = (_E // _EBR,)
    return pl.pallas_call(
        _gat_msg_body,
        grid=grid,
        in_specs=[
            pl.BlockSpec((_EBR, _H), lambda i: (i, 0)),
            pl.BlockSpec((1, _H), lambda i: (0, 0)),
            pl.BlockSpec((_EBR, _HID), lambda i: (i, 0)),
        ],
        out_specs=pl.BlockSpec((_EBR, _HID + _H), lambda i: (i, 0)),
        out_shape=jax.ShapeDtypeStruct((_E, _HID + _H), jnp.float32),
        interpret=_INTERPRET,
    )(a, gmax, hs)


def _gat_final_body(num_ref, den_ref, b_ref, o_ref):
    denw = _headexpand(den_ref[...])
    o_ref[...] = num_ref[...] / (denw + 1e-16) + b_ref[...]


def _gat_final(num, den, bias):
    return pl.pallas_call(
        _gat_final_body,
        out_shape=jax.ShapeDtypeStruct((_N, _HID), jnp.float32),
        interpret=_INTERPRET,
    )(num, den, bias.reshape(1, _HID))


def _gin1_body(h_ref, agg_ref, eps_ref, w_ref, b_ref, g_ref, bt_ref, o_ref):
    h0 = (1.0 + eps_ref[0, 0]) * h_ref[...] + agg_ref[...]
    h1 = jnp.dot(h0, w_ref[...], preferred_element_type=jnp.float32) + b_ref[...]
    h1 = g_ref[...] * h1 / jnp.sqrt(1.0 + 1e-5) + bt_ref[...]
    o_ref[...] = jnp.maximum(h1, 0.0)


def _gin1(h, agg, eps, w1, b1, gamma, beta, br=_NBR):
    grid = (_N // br,)
    n2 = w1.shape[1]
    return pl.pallas_call(
        _gin1_body,
        grid=grid,
        in_specs=[
            pl.BlockSpec((br, _HID), lambda i: (i, 0)),
            pl.BlockSpec((br, _HID), lambda i: (i, 0)),
            pl.BlockSpec((1, 1), lambda i: (0, 0)),
            pl.BlockSpec((_HID, n2), lambda i: (0, 0)),
            pl.BlockSpec((1, n2), lambda i: (0, 0)),
            pl.BlockSpec((1, n2), lambda i: (0, 0)),
            pl.BlockSpec((1, n2), lambda i: (0, 0)),
        ],
        out_specs=pl.BlockSpec((br, n2), lambda i: (i, 0)),
        out_shape=jax.ShapeDtypeStruct((_N, n2), jnp.float32),
        interpret=_INTERPRET,
    )(h, agg, eps.reshape(1, 1), w1, b1.reshape(1, n2),
      gamma.reshape(1, n2), beta.reshape(1, n2))


_PBR = 1000


def _pool1_body(h_ref, w1_ref, b1_ref, w2_ref, bb2_ref, l_ref, m_ref):
    t = jnp.tanh(jnp.dot(h_ref[...], w1_ref[...],
                         preferred_element_type=jnp.float32) + b1_ref[...])
    logit = jnp.sum(t * w2_ref[...], axis=1, keepdims=True) + bb2_ref[0, 0]
    l_ref[...] = logit
    bm = jnp.max(logit, axis=0, keepdims=True)

    @pl.when(pl.program_id(0) == 0)
    def _():
        m_ref[...] = bm

    @pl.when(pl.program_id(0) != 0)
    def _():
        m_ref[...] = jnp.maximum(m_ref[...], bm)


def _pool2_body(h_ref, l_ref, g_ref, br_ref, bc_ref, s_ref, mx_ref, at_ref,
                cnt_ref, es_ref):
    h = h_ref[...]
    brow = br_ref[0]
    oh_t = (lax.broadcasted_iota(jnp.int32, (_G, _PBR), 0)
            == brow).astype(jnp.float32)
    ex = jnp.exp(l_ref[...] - g_ref[...])
    s_blk = jnp.dot(oh_t, h, preferred_element_type=jnp.float32)
    at_blk = jnp.dot(oh_t, h * ex, preferred_element_type=jnp.float32)
    cnt_blk = jnp.sum(oh_t, axis=1, keepdims=True)
    es_blk = jnp.sum(ex, axis=0, keepdims=True)

    first = pl.program_id(0) == 0

    @pl.when(first)
    def _():
        s_ref[...] = s_blk
        at_ref[...] = at_blk
        cnt_ref[...] = cnt_blk
        es_ref[...] = es_blk

    @pl.when(jnp.logical_not(first))
    def _():
        s_ref[...] = s_ref[...] + s_blk
        at_ref[...] = at_ref[...] + at_blk
        cnt_ref[...] = cnt_ref[...] + cnt_blk
        es_ref[...] = es_ref[...] + es_blk

    bcol = bc_ref[...]
    for g in range(_G):
        mg = jnp.max(jnp.where(bcol == g, h, -jnp.inf), axis=0, keepdims=True)

        @pl.when(first)
        def _():
            mx_ref[pl.ds(g, 1), :] = mg

        @pl.when(jnp.logical_not(first))
        def _():
            mx_ref[pl.ds(g, 1), :] = jnp.maximum(mx_ref[pl.ds(g, 1), :], mg)


def _pool3_body(s_ref, mx_ref, at_ref, cnt_ref, es_ref,
                ow1_ref, ob1_ref, ow2_ref, ob2_ref, o_ref):
    s = s_ref[...]
    cnt = cnt_ref[...]
    mean = s / jnp.maximum(cnt, 1.0)
    attn = at_ref[...] / es_ref[0, 0]
    pooled = jnp.concatenate([mean, mx_ref[...], s, attn], axis=1)
    o1 = jnp.dot(pooled, ow1_ref[...], preferred_element_type=jnp.float32)
    o1 = jnp.maximum(o1 + ob1_ref[...], 0.0)
    o_ref[...] = jnp.dot(o1, ow2_ref[...],
                         preferred_element_type=jnp.float32) + ob2_ref[...]


def _pool_head(h, batch_row, p):
    grid = (_N // _PBR,)
    nh = p["ap_W1"].shape[1]
    logits, gmax = pl.pallas_call(
        _pool1_body,
        grid=grid,
        in_specs=[
            pl.BlockSpec((_PBR, _HID), lambda i: (i, 0)),
            pl.BlockSpec((_HID, nh), lambda i: (0, 0)),
            pl.BlockSpec((1, nh), lambda i: (0, 0)),
            pl.BlockSpec((1, nh), lambda i: (0, 0)),
            pl.BlockSpec((1, 1), lambda i: (0, 0)),
        ],
        out_specs=[
            pl.BlockSpec((_PBR, 1), lambda i: (i, 0)),
            pl.BlockSpec((1, 1), lambda i: (0, 0)),
        ],
        out_shape=[
            jax.ShapeDtypeStruct((_N, 1), jnp.float32),
            jax.ShapeDtypeStruct((1, 1), jnp.float32),
        ],
        interpret=_INTERPRET,
    )(h, p["ap_W1"], p["ap_b1"].reshape(1, nh),
      p["ap_W2"].reshape(1, nh), p["ap_b2"].reshape(1, 1))

    s, mx, at, cnt, es = pl.pallas_call(
        _pool2_body,
        grid=grid,
        in_specs=[
            pl.BlockSpec((_PBR, _HID), lambda i: (i, 0)),
            pl.BlockSpec((_PBR, 1), lambda i: (i, 0)),
            pl.BlockSpec((1, 1), lambda i: (0, 0)),
            pl.BlockSpec((1, 1, _PBR), lambda i: (i, 0, 0)),
            pl.BlockSpec((_PBR, 1), lambda i: (i, 0)),
        ],
        out_specs=[
            pl.BlockSpec((_G, _HID), lambda i: (0, 0)),
            pl.BlockSpec((_G, _HID), lambda i: (0, 0)),
            pl.BlockSpec((_G, _HID), lambda i: (0, 0)),
            pl.BlockSpec((_G, 1), lambda i: (0, 0)),
            pl.BlockSpec((1, 1), lambda i: (0, 0)),
        ],
        out_shape=[
            jax.ShapeDtypeStruct((_G, _HID), jnp.float32),
            jax.ShapeDtypeStruct((_G, _HID), jnp.float32),
            jax.ShapeDtypeStruct((_G, _HID), jnp.float32),
            jax.ShapeDtypeStruct((_G, 1), jnp.float32),
            jax.ShapeDtypeStruct((1, 1), jnp.float32),
        ],
        interpret=_INTERPRET,
    )(h, logits, gmax, batch_row.reshape(_N // _PBR, 1, _PBR),
      batch_row.reshape(_N, 1))

    return pl.pallas_call(
        _pool3_body,
        out_shape=jax.ShapeDtypeStruct((_G, _HID), jnp.float32),
        interpret=_INTERPRET,
    )(s, mx, at, cnt, es,
      p["out_W1"], p["out_b1"].reshape(1, -1),
      p["out_W2"], p["out_b2"].reshape(1, -1))


_NW = 32
_BPW = _E // _NW


def _sc_gather_fn(d, ch):
    n_full, rem = divmod(_BPW, ch)
    sizes = [ch] * n_full + ([rem] if rem else [])
    offs = [i * ch for i in range(len(sizes))]
    mesh = plsc.VectorSubcoreMesh(core_axis_name="c", subcore_axis_name="s",
                                  num_cores=2)

    @functools.partial(
        pl.kernel,
        mesh=mesh,
        out_type=jax.ShapeDtypeStruct((_E, d), jnp.float32),
        scratch_types=[
            pltpu.VMEM((_BPW,), jnp.int32),
            pltpu.VMEM((ch, d), jnp.float32),
            pltpu.VMEM((ch, d), jnp.float32),
            pltpu.SemaphoreType.DMA,
            pltpu.SemaphoreType.DMA,
        ],
    )
    def k(tab_hbm, idx_hbm, out_hbm, idx_v, buf0, buf1, sem0, sem1):
        wid = lax.axis_index("s") * 2 + lax.axis_index("c")
        base = wid * _BPW
        pltpu.sync_copy(idx_hbm.at[pl.ds(base, _BPW)], idx_v)
        bufs = (buf0, buf1)
        sems = (sem0, sem1)
        cps = [None, None]
        cps[0] = pltpu.async_copy(
            tab_hbm.at[idx_v.at[pl.ds(0, sizes[0])]],
            bufs[0].at[pl.ds(0, sizes[0])], sems[0])
        for i in range(len(sizes)):
            if i + 1 < len(sizes):
                cps[(i + 1) % 2] = pltpu.async_copy(
                    tab_hbm.at[idx_v.at[pl.ds(offs[i + 1], sizes[i + 1])]],
                    bufs[(i + 1) % 2].at[pl.ds(0, sizes[i + 1])],
                    sems[(i + 1) % 2])
            cps[i % 2].wait()
            pltpu.sync_copy(bufs[i % 2].at[pl.ds(0, sizes[i])],
                            out_hbm.at[pl.ds(base + offs[i], sizes[i])])

    return k


_SC_GATHER_CACHE = {}
_GATHER_CHUNK = {256: 200, 512: 96}


def _gather_rows(tab, idx):
    d = tab.shape[1]
    if d not in _GATHER_CHUNK:
        return jnp.take(tab, idx, axis=0)
    if d not in _SC_GATHER_CACHE:
        _SC_GATHER_CACHE[d] = _sc_gather_fn(d, _GATHER_CHUNK[d])
    return _SC_GATHER_CACHE[d](tab, idx)


def _scatter_add_rows(rows, idx, n):
    return jax.ops.segment_sum(rows, idx, num_segments=n)


_EPT = _E // 16
_SCATTER_ECH = 80


def _sc_scatter_fn(dcols):
    ech = _SCATTER_ECH
    n_chunks = _EPT // ech
    rpt = 624
    mesh = plsc.VectorSubcoreMesh(core_axis_name="c", subcore_axis_name="s",
                                  num_cores=2)

    @functools.partial(
        pl.kernel,
        mesh=mesh,
        out_type=jax.ShapeDtypeStruct((_N, 256), jnp.float32),
        scratch_types=[
            pltpu.VMEM((n_chunks, ech), jnp.int32),
            pltpu.VMEM((ech, 128), jnp.float32),
            pltpu.VMEM((ech, 128), jnp.float32),
            pltpu.VMEM_SHARED((_N, 128), jnp.float32),
            pltpu.SemaphoreType.DMA,
            pltpu.SemaphoreType.DMA,
        ],
    )
    def k(rows_hbm, dst3_hbm, z_hbm, out_hbm,
          idx_v, buf0, buf1, accum, sem0, sem1):
        cid = lax.axis_index("c")
        sid = lax.axis_index("s")
        base = sid * _EPT
        pltpu.sync_copy(dst3_hbm.at[sid], idx_v)

        @pl.when(sid == 0)
        def _():
            pltpu.sync_copy(z_hbm, accum)

        plsc.subcore_barrier()
        bufs = (buf0, buf1)
        sems = (sem0, sem1)
        cps = [None, None]
        cps[0] = pltpu.async_copy(
            rows_hbm.at[pl.ds(base, ech), pl.ds(cid * 128, 128)],
            bufs[0], sems[0])
        for i in range(n_chunks):
            if i + 1 < n_chunks:
                cps[(i + 1) % 2] = pltpu.async_copy(
                    rows_hbm.at[pl.ds(base + (i + 1) * ech, ech),
                                pl.ds(cid * 128, 128)],
                    bufs[(i + 1) % 2], sems[(i + 1) % 2])
            cps[i % 2].wait()
            pltpu.sync_copy(bufs[i % 2], accum.at[idx_v.at[i]], add=True)
        plsc.subcore_barrier()

        @pl.when(sid < 15)
        def _():
            pltpu.sync_copy(
                accum.at[pl.ds(sid * rpt, rpt)],
                out_hbm.at[pl.ds(sid * rpt, rpt), pl.ds(cid * 128, 128)])

        @pl.when(sid == 15)
        def _():
            pltpu.sync_copy(
                accum.at[pl.ds(15 * rpt, _N - 15 * rpt)],
                out_hbm.at[pl.ds(15 * rpt, _N - 15 * rpt),
                           pl.ds(cid * 128, 128)])

    return k


_SC_SCATTER_CACHE = {}


def _sc_scatter256(rows, dst3, zeros_half):
    d = rows.shape[1]
    if d not in _SC_SCATTER_CACHE:
        _SC_SCATTER_CACHE[d] = _sc_scatter_fn(d)
    return _SC_SCATTER_CACHE[d](rows, dst3, zeros_half)


def kernel(x, edge_index, edge_attr, batch, params):
    src = edge_index[0]
    dst = edge_index[1]
    batch_row = batch.reshape(1, _N)
    dst3 = dst.reshape(16, _EPT // _SCATTER_ECH, _SCATTER_ECH)
    zhalf = jnp.zeros((_N, 128), jnp.float32)

    h = _mm(x, params["node_W"], params["node_b"])
    ea = _mm(edge_attr, params["edge_W"], params["edge_b"], br=_EBR)

    for i in range(6):
        p = params["layer%d" % i]
        if i % 3 == 0:
            wq = jnp.concatenate([p["Wq"], p["Wk"], p["Wv"], p["Wskip"]], axis=1)
            bq = jnp.concatenate([p["bq"], p["bk"], p["bv"], p["bskip"]])
            q, kv, xs = _mm_qkvs(h, wq, bq)
            e = _mm(ea, p["We"], p["be"], br=_EBR)
            qd = _gather_rows(q, dst)
            kvg = _gather_rows(kv, src)
            a, bmax = _tconv_logits(qd, kvg, e)
            wmsg = _tconv_msg(a, bmax, kvg, e)
            num = _sc_scatter256(wmsg, dst3, zhalf)
            den = _scatter_add_rows(wmsg[:, _HID:], dst, _N)
            wb = p["Wbeta"].reshape(3, _HID)
            wa_out = wb[0] + wb[2]
            wa_xs = wb[1] - wb[2]
            xa = _tconv_final(num, den, xs, wa_out, wa_xs)
            h2 = _graph_norm(h, xa, p["n_w"], p["n_b"], p["n_ms"])
            f1 = _mm(h2, p["f_W1"], p["f_b1"], act="gelu")
            x_new = _mm(f1, p["f_W2"], p["f_b2"], add=h2)
        elif i % 3 == 1:
            hh = _mm(h, p["W"], jnp.zeros((_HID,), jnp.float32))
            adst = _gat_node(hh, p["att_dst"].reshape(_HID))
            e = _mm(ea, p["We"], jnp.zeros((_HID,), jnp.float32), br=_EBR)
            adg = _gather_rows(adst, dst)
            hs = _gather_rows(hh, src)
            a, bmax = _gat_logits(hs, adg, e, p["att_src"].reshape(_HID),
                                  p["att_edge"].reshape(_HID))
            wmsg = _gat_msg(a, bmax, hs)
            num = _sc_scatter256(wmsg, dst3, zhalf)
            den = _scatter_add_rows(wmsg[:, _HID:], dst, _N)
            x_new = _gat_final(num, den, p["bias"])
        else:
            agg = _sc_scatter256(_gather_rows(h, src), dst3, zhalf)
            t = _gin1(h, agg, p["eps"], p["W1"], p["b1"],
                      p["gamma"], p["beta"])
            x_new = _mm(t, p["W2"], p["b2"])
        q = params["norm%d" % i]
        h = _graph_norm(h, x_new, q["w"], q["b"], q["ms"])

    return _pool_head(h, batch_row, params)


# speedup vs baseline: 4.6251x; 1.5105x over previous
import functools
import math

import jax
import jax.numpy as jnp
from jax import lax
from jax.experimental import pallas as pl
from jax.experimental.pallas import tpu as pltpu
from jax.experimental.pallas import tpu_sc as plsc

_N = 10000
_E = 160000
_HID = 256
_H = 8
_C = 32
_G = 16

_INTERPRET = False

_EBR = 4000
_NBR = 2000


def _headsum(m):
    r = lax.broadcasted_iota(jnp.int32, (_HID, _H), 0) // _C
    c = lax.broadcasted_iota(jnp.int32, (_HID, _H), 1)
    s = (r == c).astype(jnp.float32)
    return jnp.dot(m, s, preferred_element_type=jnp.float32)


def _headexpand(w):
    r = lax.broadcasted_iota(jnp.int32, (_H, _HID), 0)
    c = lax.broadcasted_iota(jnp.int32, (_H, _HID), 1) // _C
    s = (r == c).astype(jnp.float32)
    return jnp.dot(w, s, preferred_element_type=jnp.float32)


def _mm_body(act, x_ref, w_ref, b_ref, o_ref):
    y = jnp.dot(x_ref[...], w_ref[...], preferred_element_type=jnp.float32)
    y = y + b_ref[...]
    if act == "relu":
        y = jnp.maximum(y, 0.0)
    elif act == "gelu":
        y = jax.nn.gelu(y)
    o_ref[...] = y


def _mm_add_body(act, x_ref, w_ref, b_ref, a_ref, o_ref):
    y = jnp.dot(x_ref[...], w_ref[...], preferred_element_type=jnp.float32)
    y = y + b_ref[...]
    if act == "relu":
        y = jnp.maximum(y, 0.0)
    elif act == "gelu":
        y = jax.nn.gelu(y)
    o_ref[...] = y + a_ref[...]


def _mm(x, w, b, act=None, br=_NBR, add=None):
    m, k = x.shape
    n = w.shape[1]
    grid = (m // br,)
    in_specs = [
        pl.BlockSpec((br, k), lambda i: (i, 0)),
        pl.BlockSpec((k, n), lambda i: (0, 0)),
        pl.BlockSpec((1, n), lambda i: (0, 0)),
    ]
    args = [x, w, b.reshape(1, n)]
    if add is None:
        body = functools.partial(_mm_body, act)
    else:
        body = functools.partial(_mm_add_body, act)
        in_specs.append(pl.BlockSpec((br, n), lambda i: (i, 0)))
        args.append(add)
    return pl.pallas_call(
        body,
        grid=grid,
        in_specs=in_specs,
        out_specs=pl.BlockSpec((br, n), lambda i: (i, 0)),
        out_shape=jax.ShapeDtypeStruct((m, n), jnp.float32),
        interpret=_INTERPRET,
    )(*args)


def _mm_qkvs_body(x_ref, w_ref, b_ref, q_ref, kv_ref, xs_ref):
    y = jnp.dot(x_ref[...], w_ref[...], preferred_element_type=jnp.float32)
    y = y + b_ref[...]
    q_ref[...] = y[:, :_HID]
    kv_ref[...] = y[:, _HID:3 * _HID]
    xs_ref[...] = y[:, 3 * _HID:]


def _mm_qkvs(x, w, b, br=_NBR):
    grid = (_N // br,)
    return pl.pallas_call(
        _mm_qkvs_body,
        grid=grid,
        in_specs=[
            pl.BlockSpec((br, _HID), lambda i: (i, 0)),
            pl.BlockSpec((_HID, 4 * _HID), lambda i: (0, 0)),
            pl.BlockSpec((1, 4 * _HID), lambda i: (0, 0)),
        ],
        out_specs=[
            pl.BlockSpec((br, _HID), lambda i: (i, 0)),
            pl.BlockSpec((br, 2 * _HID), lambda i: (i, 0)),
            pl.BlockSpec((br, _HID), lambda i: (i, 0)),
        ],
        out_shape=[
            jax.ShapeDtypeStruct((_N, _HID), jnp.float32),
            jax.ShapeDtypeStruct((_N, 2 * _HID), jnp.float32),
            jax.ShapeDtypeStruct((_N, _HID), jnp.float32),
        ],
        interpret=_INTERPRET,
    )(x, w, b.reshape(1, 4 * _HID))


def _norm_body(x_ref, a_ref, w_ref, b_ref, ms_ref, o_ref):
    x = x_ref[...] + a_ref[...]
    mean = jnp.mean(x, axis=0, keepdims=True)
    out = x - ms_ref[...] * mean
    var = jnp.mean(out * out, axis=0, keepdims=True)
    o_ref[...] = w_ref[...] * out / jnp.sqrt(var + 1e-5) + b_ref[...]


def _graph_norm(x, add, w, b, ms):
    return pl.pallas_call(
        _norm_body,
        out_shape=jax.ShapeDtypeStruct((_N, _HID), jnp.float32),
        interpret=_INTERPRET,
    )(x, add, w.reshape(1, _HID), b.reshape(1, _HID), ms.reshape(1, _HID))


def _tc_edge_body(qd_ref, ks_ref, vs_ref, e_ref, msg_ref, wd_ref):
    e = e_ref[...]
    m = qd_ref[...] * (ks_ref[...] + e)
    a = _headsum(m) * (1.0 / math.sqrt(float(_C)))
    w = jnp.exp(a)
    msg_ref[...] = (vs_ref[...] + e) * _headexpand(w)
    wd_ref[:, :_H] = w
    wd_ref[:, _H:] = jnp.zeros((_EBR, 128 - _H), jnp.float32)


def _tconv_edge(qd, kvg, e):
    grid = (_E // _EBR,)
    return pl.pallas_call(
        _tc_edge_body,
        grid=grid,
        in_specs=[
            pl.BlockSpec((_EBR, _HID), lambda i: (i, 0)),
            pl.BlockSpec((_EBR, _HID), lambda i: (i, 0)),
            pl.BlockSpec((_EBR, _HID), lambda i: (i, 1)),
            pl.BlockSpec((_EBR, _HID), lambda i: (i, 0)),
        ],
        out_specs=[
            pl.BlockSpec((_EBR, _HID), lambda i: (i, 0)),
            pl.BlockSpec((_EBR, 128), lambda i: (i, 0)),
        ],
        out_shape=[
            jax.ShapeDtypeStruct((_E, _HID), jnp.float32),
            jax.ShapeDtypeStruct((_E, 128), jnp.float32),
        ],
        interpret=_INTERPRET,
    )(qd, kvg, kvg, e)


def _tc_final_body(num_ref, den_ref, xs_ref, wa_ref, wb_ref, o_ref):
    den = den_ref[0, :, :_H] + den_ref[1, :, :_H]
    denw = _headexpand(den)
    out = num_ref[...] / (denw + 1e-16)
    xs = xs_ref[...]
    logit = jnp.sum(out * wa_ref[...] + xs * wb_ref[...], axis=1, keepdims=True)
    beta = jax.nn.sigmoid(logit)
    o_ref[...] = beta * xs + (1.0 - beta) * out


def _tconv_final(num, den, xs, wa, wb):
    return pl.pallas_call(
        _tc_final_body,
        out_shape=jax.ShapeDtypeStruct((_N, _HID), jnp.float32),
        interpret=_INTERPRET,
    )(num, den, xs, wa.reshape(1, _HID), wb.reshape(1, _HID))


def _gat_node_body(hh_ref, adf_ref, ad_ref):
    ad = _headsum(hh_ref[...] * adf_ref[...])
    ad_ref[:, :_H] = ad
    ad_ref[:, _H:] = jnp.zeros((_N, 128 - _H), jnp.float32)


def _gat_node(hh, adf):
    return pl.pallas_call(
        _gat_node_body,
        out_shape=jax.ShapeDtypeStruct((_N, 128), jnp.float32),
        interpret=_INTERPRET,
    )(hh, adf.reshape(1, _HID))


def _gat_edge_body(hs_ref, adg_ref, e_ref, asf_ref, aef_ref, msg_ref, wd_ref):
    hs = hs_ref[...]
    asg = _headsum(hs * asf_ref[...])
    ae = _headsum(e_ref[...] * aef_ref[...])
    a = asg + adg_ref[:, :_H] + ae
    a = jnp.where(a >= 0.0, a, 0.2 * a)
    w = jnp.exp(a)
    msg_ref[...] = hs * _headexpand(w)
    wd_ref[:, :_H] = w
    wd_ref[:, _H:] = jnp.zeros((_EBR, 128 - _H), jnp.float32)


def _gat_edge(hs, adg, e, asf, aef):
    grid = (_E // _EBR,)
    return pl.pallas_call(
        _gat_edge_body,
        grid=grid,
        in_specs=[
            pl.BlockSpec((_EBR, _HID), lambda i: (i, 0)),
            pl.BlockSpec((_EBR, 128), lambda i: (i, 0)),
            pl.BlockSpec((_EBR, _HID), lambda i: (i, 0)),
            pl.BlockSpec((1, _HID), lambda i: (0, 0)),
            pl.BlockSpec((1, _HID), lambda i: (0, 0)),
        ],
        out_specs=[
            pl.BlockSpec((_EBR, _HID), lambda i: (i, 0)),
            pl.BlockSpec((_EBR, 128), lambda i: (i, 0)),
        ],
        out_shape=[
            jax.ShapeDtypeStruct((_E, _HID), jnp.float32),
            jax.ShapeDtypeStruct((_E, 128), jnp.float32),
        ],
        interpret=_INTERPRET,
    )(hs, adg, e, asf.reshape(1, _HID), aef.reshape(1, _HID))


def _gat_final_body(num_ref, den_ref, b_ref, o_ref):
    den = den_ref[0, :, :_H] + den_ref[1, :, :_H]
    denw = _headexpand(den)
    o_ref[...] = num_ref[...] / (denw + 1e-16) + b_ref[...]


def _gat_final(num, den, bias):
    return pl.pallas_call(
        _gat_final_body,
        out_shape=jax.ShapeDtypeStruct((_N, _HID), jnp.float32),
        interpret=_INTERPRET,
    )(num, den, bias.reshape(1, _HID))


def _gin1_body(h_ref, agg_ref, eps_ref, w_ref, b_ref, g_ref, bt_ref, o_ref):
    h0 = (1.0 + eps_ref[0, 0]) * h_ref[...] + agg_ref[...]
    h1 = jnp.dot(h0, w_ref[...], preferred_element_type=jnp.float32) + b_ref[...]
    h1 = g_ref[...] * h1 / jnp.sqrt(1.0 + 1e-5) + bt_ref[...]
    o_ref[...] = jnp.maximum(h1, 0.0)


def _gin1(h, agg, eps, w1, b1, gamma, beta, br=_NBR):
    grid = (_N // br,)
    n2 = w1.shape[1]
    return pl.pallas_call(
        _gin1_body,
        grid=grid,
        in_specs=[
            pl.BlockSpec((br, _HID), lambda i: (i, 0)),
            pl.BlockSpec((br, _HID), lambda i: (i, 0)),
            pl.BlockSpec((1, 1), lambda i: (0, 0)),
            pl.BlockSpec((_HID, n2), lambda i: (0, 0)),
            pl.BlockSpec((1, n2), lambda i: (0, 0)),
            pl.BlockSpec((1, n2), lambda i: (0, 0)),
            pl.BlockSpec((1, n2), lambda i: (0, 0)),
        ],
        out_specs=pl.BlockSpec((br, n2), lambda i: (i, 0)),
        out_shape=jax.ShapeDtypeStruct((_N, n2), jnp.float32),
        interpret=_INTERPRET,
    )(h, agg, eps.reshape(1, 1), w1, b1.reshape(1, n2),
      gamma.reshape(1, n2), beta.reshape(1, n2))


_PBR = 1000


def _pool1_body(h_ref, w1_ref, b1_ref, w2_ref, bb2_ref, l_ref, m_ref):
    t = jnp.tanh(jnp.dot(h_ref[...], w1_ref[...],
                         preferred_element_type=jnp.float32) + b1_ref[...])
    logit = jnp.sum(t * w2_ref[...], axis=1, keepdims=True) + bb2_ref[0, 0]
    l_ref[...] = logit
    bm = jnp.max(logit, axis=0, keepdims=True)

    @pl.when(pl.program_id(0) == 0)
    def _():
        m_ref[...] = bm

    @pl.when(pl.program_id(0) != 0)
    def _():
        m_ref[...] = jnp.maximum(m_ref[...], bm)


def _pool2_body(h_ref, l_ref, g_ref, br_ref, bc_ref, s_ref, mx_ref, at_ref,
                cnt_ref, es_ref):
    h = h_ref[...]
    brow = br_ref[0]
    oh_t = (lax.broadcasted_iota(jnp.int32, (_G, _PBR), 0)
            == brow).astype(jnp.float32)
    ex = jnp.exp(l_ref[...] - g_ref[...])
    s_blk = jnp.dot(oh_t, h, preferred_element_type=jnp.float32)
    at_blk = jnp.dot(oh_t, h * ex, preferred_element_type=jnp.float32)
    cnt_blk = jnp.sum(oh_t, axis=1, keepdims=True)
    es_blk = jnp.sum(ex, axis=0, keepdims=True)

    first = pl.program_id(0) == 0

    @pl.when(first)
    def _():
        s_ref[...] = s_blk
        at_ref[...] = at_blk
        cnt_ref[...] = cnt_blk
        es_ref[...] = es_blk

    @pl.when(jnp.logical_not(first))
    def _():
        s_ref[...] = s_ref[...] + s_blk
        at_ref[...] = at_ref[...] + at_blk
        cnt_ref[...] = cnt_ref[...] + cnt_blk
        es_ref[...] = es_ref[...] + es_blk

    bcol = bc_ref[...]
    for g in range(_G):
        mg = jnp.max(jnp.where(bcol == g, h, -jnp.inf), axis=0, keepdims=True)

        @pl.when(first)
        def _():
            mx_ref[pl.ds(g, 1), :] = mg

        @pl.when(jnp.logical_not(first))
        def _():
            mx_ref[pl.ds(g, 1), :] = jnp.maximum(mx_ref[pl.ds(g, 1), :], mg)


def _pool3_body(s_ref, mx_ref, at_ref, cnt_ref, es_ref,
                ow1_ref, ob1_ref, ow2_ref, ob2_ref, o_ref):
    s = s_ref[...]
    cnt = cnt_ref[...]
    mean = s / jnp.maximum(cnt, 1.0)
    attn = at_ref[...] / es_ref[0, 0]
    pooled = jnp.concatenate([mean, mx_ref[...], s, attn], axis=1)
    o1 = jnp.dot(pooled, ow1_ref[...], preferred_element_type=jnp.float32)
    o1 = jnp.maximum(o1 + ob1_ref[...], 0.0)
    o_ref[...] = jnp.dot(o1, ow2_ref[...],
                         preferred_element_type=jnp.float32) + ob2_ref[...]


def _pool_head(h, batch_row, p):
    grid = (_N // _PBR,)
    nh = p["ap_W1"].shape[1]
    logits, gmax = pl.pallas_call(
        _pool1_body,
        grid=grid,
        in_specs=[
            pl.BlockSpec((_PBR, _HID), lambda i: (i, 0)),
            pl.BlockSpec((_HID, nh), lambda i: (0, 0)),
            pl.BlockSpec((1, nh), lambda i: (0, 0)),
            pl.BlockSpec((1, nh), lambda i: (0, 0)),
            pl.BlockSpec((1, 1), lambda i: (0, 0)),
        ],
        out_specs=[
            pl.BlockSpec((_PBR, 1), lambda i: (i, 0)),
            pl.BlockSpec((1, 1), lambda i: (0, 0)),
        ],
        out_shape=[
            jax.ShapeDtypeStruct((_N, 1), jnp.float32),
            jax.ShapeDtypeStruct((1, 1), jnp.float32),
        ],
        interpret=_INTERPRET,
    )(h, p["ap_W1"], p["ap_b1"].reshape(1, nh),
      p["ap_W2"].reshape(1, nh), p["ap_b2"].reshape(1, 1))

    s, mx, at, cnt, es = pl.pallas_call(
        _pool2_body,
        grid=grid,
        in_specs=[
            pl.BlockSpec((_PBR, _HID), lambda i: (i, 0)),
            pl.BlockSpec((_PBR, 1), lambda i: (i, 0)),
            pl.BlockSpec((1, 1), lambda i: (0, 0)),
            pl.BlockSpec((1, 1, _PBR), lambda i: (i, 0, 0)),
            pl.BlockSpec((_PBR, 1), lambda i: (i, 0)),
        ],
        out_specs=[
            pl.BlockSpec((_G, _HID), lambda i: (0, 0)),
            pl.BlockSpec((_G, _HID), lambda i: (0, 0)),
            pl.BlockSpec((_G, _HID), lambda i: (0, 0)),
            pl.BlockSpec((_G, 1), lambda i: (0, 0)),
            pl.BlockSpec((1, 1), lambda i: (0, 0)),
        ],
        out_shape=[
            jax.ShapeDtypeStruct((_G, _HID), jnp.float32),
            jax.ShapeDtypeStruct((_G, _HID), jnp.float32),
            jax.ShapeDtypeStruct((_G, _HID), jnp.float32),
            jax.ShapeDtypeStruct((_G, 1), jnp.float32),
            jax.ShapeDtypeStruct((1, 1), jnp.float32),
        ],
        interpret=_INTERPRET,
    )(h, logits, gmax, batch_row.reshape(_N // _PBR, 1, _PBR),
      batch_row.reshape(_N, 1))

    return pl.pallas_call(
        _pool3_body,
        out_shape=jax.ShapeDtypeStruct((_G, _HID), jnp.float32),
        interpret=_INTERPRET,
    )(s, mx, at, cnt, es,
      p["out_W1"], p["out_b1"].reshape(1, -1),
      p["out_W2"], p["out_b2"].reshape(1, -1))


_NW = 32
_BPW = _E // _NW


def _sc_gather_fn(d, ch):
    n_full, rem = divmod(_BPW, ch)
    sizes = [ch] * n_full + ([rem] if rem else [])
    offs = [i * ch for i in range(len(sizes))]
    mesh = plsc.VectorSubcoreMesh(core_axis_name="c", subcore_axis_name="s",
                                  num_cores=2)

    @functools.partial(
        pl.kernel,
        mesh=mesh,
        out_type=jax.ShapeDtypeStruct((_E, d), jnp.float32),
        scratch_types=[
            pltpu.VMEM((_BPW,), jnp.int32),
            pltpu.VMEM((ch, d), jnp.float32),
            pltpu.VMEM((ch, d), jnp.float32),
            pltpu.SemaphoreType.DMA,
            pltpu.SemaphoreType.DMA,
        ],
    )
    def k(tab_hbm, idx_hbm, out_hbm, idx_v, buf0, buf1, sem0, sem1):
        wid = lax.axis_index("s") * 2 + lax.axis_index("c")
        base = wid * _BPW
        pltpu.sync_copy(idx_hbm.at[pl.ds(base, _BPW)], idx_v)
        bufs = (buf0, buf1)
        sems = (sem0, sem1)
        cps = [None, None]
        cps[0] = pltpu.async_copy(
            tab_hbm.at[idx_v.at[pl.ds(0, sizes[0])]],
            bufs[0].at[pl.ds(0, sizes[0])], sems[0])
        for i in range(len(sizes)):
            if i + 1 < len(sizes):
                cps[(i + 1) % 2] = pltpu.async_copy(
                    tab_hbm.at[idx_v.at[pl.ds(offs[i + 1], sizes[i + 1])]],
                    bufs[(i + 1) % 2].at[pl.ds(0, sizes[i + 1])],
                    sems[(i + 1) % 2])
            cps[i % 2].wait()
            pltpu.sync_copy(bufs[i % 2].at[pl.ds(0, sizes[i])],
                            out_hbm.at[pl.ds(base + offs[i], sizes[i])])

    return k


_SC_GATHER_CACHE = {}
_GATHER_CHUNK = {256: 200, 512: 96, 128: 200}


def _gather_rows(tab, idx):
    d = tab.shape[1]
    if d not in _GATHER_CHUNK:
        return jnp.take(tab, idx, axis=0)
    if d not in _SC_GATHER_CACHE:
        _SC_GATHER_CACHE[d] = _sc_gather_fn(d, _GATHER_CHUNK[d])
    return _SC_GATHER_CACHE[d](tab, idx)


def _scatter_add_rows(rows, idx, n):
    return jax.ops.segment_sum(rows, idx, num_segments=n)


_EPT = _E // 16
_SCATTER_ECH = 80


def _sc_scatter_fn(dcols):
    ech = _SCATTER_ECH
    n_chunks = _EPT // ech
    rpt = 624
    mesh = plsc.VectorSubcoreMesh(core_axis_name="c", subcore_axis_name="s",
                                  num_cores=2)

    @functools.partial(
        pl.kernel,
        mesh=mesh,
        out_type=jax.ShapeDtypeStruct((_N, 256), jnp.float32),
        scratch_types=[
            pltpu.VMEM((n_chunks, ech), jnp.int32),
            pltpu.VMEM((ech, 128), jnp.float32),
            pltpu.VMEM((ech, 128), jnp.float32),
            pltpu.VMEM_SHARED((_N, 128), jnp.float32),
            pltpu.SemaphoreType.DMA,
            pltpu.SemaphoreType.DMA,
        ],
    )
    def k(rows_hbm, dst3_hbm, z_hbm, out_hbm,
          idx_v, buf0, buf1, accum, sem0, sem1):
        cid = lax.axis_index("c")
        sid = lax.axis_index("s")
        base = sid * _EPT
        pltpu.sync_copy(dst3_hbm.at[sid], idx_v)

        @pl.when(sid == 0)
        def _():
            pltpu.sync_copy(z_hbm, accum)

        plsc.subcore_barrier()
        bufs = (buf0, buf1)
        sems = (sem0, sem1)
        cps = [None, None]
        cps[0] = pltpu.async_copy(
            rows_hbm.at[pl.ds(base, ech), pl.ds(cid * 128, 128)],
            bufs[0], sems[0])
        for i in range(n_chunks):
            if i + 1 < n_chunks:
                cps[(i + 1) % 2] = pltpu.async_copy(
                    rows_hbm.at[pl.ds(base + (i + 1) * ech, ech),
                                pl.ds(cid * 128, 128)],
                    bufs[(i + 1) % 2], sems[(i + 1) % 2])
            cps[i % 2].wait()
            pltpu.sync_copy(bufs[i % 2], accum.at[idx_v.at[i]], add=True)
        plsc.subcore_barrier()

        @pl.when(sid < 15)
        def _():
            pltpu.sync_copy(
                accum.at[pl.ds(sid * rpt, rpt)],
                out_hbm.at[pl.ds(sid * rpt, rpt), pl.ds(cid * 128, 128)])

        @pl.when(sid == 15)
        def _():
            pltpu.sync_copy(
                accum.at[pl.ds(15 * rpt, _N - 15 * rpt)],
                out_hbm.at[pl.ds(15 * rpt, _N - 15 * rpt),
                           pl.ds(cid * 128, 128)])

    return k


_DEN_ECH = 40
_DEN_BPW = _E // 32


def _sc_scatter_den_fn():
    ech = _DEN_ECH
    n_chunks = _DEN_BPW // ech
    rpt = 624
    mesh = plsc.VectorSubcoreMesh(core_axis_name="c", subcore_axis_name="s",
                                  num_cores=2)

    @functools.partial(
        pl.kernel,
        mesh=mesh,
        out_type=jax.ShapeDtypeStruct((2, _N, 128), jnp.float32),
        scratch_types=[
            pltpu.VMEM((n_chunks, ech), jnp.int32),
            pltpu.VMEM((ech, 128), jnp.float32),
            pltpu.VMEM((ech, 128), jnp.float32),
            pltpu.VMEM_SHARED((_N, 128), jnp.float32),
            pltpu.SemaphoreType.DMA,
            pltpu.SemaphoreType.DMA,
        ],
    )
    def k(rows_hbm, dst4_hbm, z_hbm, out_hbm,
          idx_v, buf0, buf1, accum, sem0, sem1):
        cid = lax.axis_index("c")
        sid = lax.axis_index("s")
        wid = cid * 16 + sid
        base = wid * _DEN_BPW
        pltpu.sync_copy(dst4_hbm.at[wid], idx_v)

        @pl.when(sid == 0)
        def _():
            pltpu.sync_copy(z_hbm, accum)

        plsc.subcore_barrier()
        bufs = (buf0, buf1)
        sems = (sem0, sem1)
        cps = [None, None]
        cps[0] = pltpu.async_copy(rows_hbm.at[pl.ds(base, ech)],
                                  bufs[0], sems[0])
        for i in range(n_chunks):
            if i + 1 < n_chunks:
                cps[(i + 1) % 2] = pltpu.async_copy(
                    rows_hbm.at[pl.ds(base + (i + 1) * ech, ech)],
                    bufs[(i + 1) % 2], sems[(i + 1) % 2])
            cps[i % 2].wait()
            pltpu.sync_copy(bufs[i % 2], accum.at[idx_v.at[i]], add=True)
        plsc.subcore_barrier()

        @pl.when(sid < 15)
        def _():
            pltpu.sync_copy(accum.at[pl.ds(sid * rpt, rpt)],
                            out_hbm.at[cid, pl.ds(sid * rpt, rpt)])

        @pl.when(sid == 15)
        def _():
            pltpu.sync_copy(accum.at[pl.ds(15 * rpt, _N - 15 * rpt)],
                            out_hbm.at[cid, pl.ds(15 * rpt, _N - 15 * rpt)])

    return k


_SC_SCATTER_CACHE = {}


def _sc_scatter_den(rows, dst4, zeros_half):
    if "den" not in _SC_SCATTER_CACHE:
        _SC_SCATTER_CACHE["den"] = _sc_scatter_den_fn()
    return _SC_SCATTER_CACHE["den"](rows, dst4, zeros_half)


def _sc_scatter256(rows, dst3, zeros_half):
    d = rows.shape[1]
    if d not in _SC_SCATTER_CACHE:
        _SC_SCATTER_CACHE[d] = _sc_scatter_fn(d)
    return _SC_SCATTER_CACHE[d](rows, dst3, zeros_half)


def kernel(x, edge_index, edge_attr, batch, params):
    src = edge_index[0]
    dst = edge_index[1]
    batch_row = batch.reshape(1, _N)
    dst3 = dst.reshape(16, _EPT // _SCATTER_ECH, _SCATTER_ECH)
    dst4 = dst.reshape(32, _DEN_BPW // _DEN_ECH, _DEN_ECH)
    zhalf = jnp.zeros((_N, 128), jnp.float32)

    h = _mm(x, params["node_W"], params["node_b"])
    ea = _mm(edge_attr, params["edge_W"], params["edge_b"], br=_EBR)

    for i in range(6):
        p = params["layer%d" % i]
        if i % 3 == 0:
            wq = jnp.concatenate([p["Wq"], p["Wk"], p["Wv"], p["Wskip"]], axis=1)
            bq = jnp.concatenate([p["bq"], p["bk"], p["bv"], p["bskip"]])
            q, kv, xs = _mm_qkvs(h, wq, bq)
            e = _mm(ea, p["We"], p["be"], br=_EBR)
            qd = _gather_rows(q, dst)
            kvg = _gather_rows(kv, src)
            msg, wden = _tconv_edge(qd, kvg, e)
            num = _sc_scatter256(msg, dst3, zhalf)
            den = _sc_scatter_den(wden, dst4, zhalf)
            wb = p["Wbeta"].reshape(3, _HID)
            wa_out = wb[0] + wb[2]
            wa_xs = wb[1] - wb[2]
            xa = _tconv_final(num, den, xs, wa_out, wa_xs)
            h2 = _graph_norm(h, xa, p["n_w"], p["n_b"], p["n_ms"])
            f1 = _mm(h2, p["f_W1"], p["f_b1"], act="gelu")
            x_new = _mm(f1, p["f_W2"], p["f_b2"], add=h2)
        elif i % 3 == 1:
            hh = _mm(h, p["W"], jnp.zeros((_HID,), jnp.float32))
            adst = _gat_node(hh, p["att_dst"].reshape(_HID))
            e = _mm(ea, p["We"], jnp.zeros((_HID,), jnp.float32), br=_EBR)
            adg = _gather_rows(adst, dst)
            hs = _gather_rows(hh, src)
            msg, wden = _gat_edge(hs, adg, e, p["att_src"].reshape(_HID),
                                  p["att_edge"].reshape(_HID))
            num = _sc_scatter256(msg, dst3, zhalf)
            den = _sc_scatter_den(wden, dst4, zhalf)
            x_new = _gat_final(num, den, p["bias"])
        else:
            agg = _sc_scatter256(_gather_rows(h, src), dst3, zhalf)
            t = _gin1(h, agg, p["eps"], p["W1"], p["b1"],
                      p["gamma"], p["beta"])
            x_new = _mm(t, p["W2"], p["b2"])
        q = params["norm%d" % i]
        h = _graph_norm(h, x_new, q["w"], q["b"], q["ms"])

    return _pool_head(h, batch_row, params)


# speedup vs baseline: 5.2488x; 1.1348x over previous
import functools
import math

import jax
import jax.numpy as jnp
from jax import lax
from jax.experimental import pallas as pl
from jax.experimental.pallas import tpu as pltpu
from jax.experimental.pallas import tpu_sc as plsc

_N = 10000
_E = 160000
_HID = 256
_H = 8
_C = 32
_G = 16
_EDGE_DIM = 10

_INTERPRET = False

_EBR = 4000
_NBR = 2000


def _headsum(m):
    r = lax.broadcasted_iota(jnp.int32, (_HID, _H), 0) // _C
    c = lax.broadcasted_iota(jnp.int32, (_HID, _H), 1)
    s = (r == c).astype(jnp.float32)
    return jnp.dot(m, s, preferred_element_type=jnp.float32)


def _headexpand(w):
    r = lax.broadcasted_iota(jnp.int32, (_H, _HID), 0)
    c = lax.broadcasted_iota(jnp.int32, (_H, _HID), 1) // _C
    s = (r == c).astype(jnp.float32)
    return jnp.dot(w, s, preferred_element_type=jnp.float32)


def _mm_body(act, x_ref, w_ref, b_ref, o_ref):
    y = jnp.dot(x_ref[...], w_ref[...], preferred_element_type=jnp.float32)
    y = y + b_ref[...]
    if act == "relu":
        y = jnp.maximum(y, 0.0)
    elif act == "gelu":
        y = jax.nn.gelu(y)
    o_ref[...] = y


def _mm_add_body(act, x_ref, w_ref, b_ref, a_ref, o_ref):
    y = jnp.dot(x_ref[...], w_ref[...], preferred_element_type=jnp.float32)
    y = y + b_ref[...]
    if act == "relu":
        y = jnp.maximum(y, 0.0)
    elif act == "gelu":
        y = jax.nn.gelu(y)
    o_ref[...] = y + a_ref[...]


def _mm(x, w, b, act=None, br=_NBR, add=None):
    m, k = x.shape
    n = w.shape[1]
    grid = (m // br,)
    in_specs = [
        pl.BlockSpec((br, k), lambda i: (i, 0)),
        pl.BlockSpec((k, n), lambda i: (0, 0)),
        pl.BlockSpec((1, n), lambda i: (0, 0)),
    ]
    args = [x, w, b.reshape(1, n)]
    if add is None:
        body = functools.partial(_mm_body, act)
    else:
        body = functools.partial(_mm_add_body, act)
        in_specs.append(pl.BlockSpec((br, n), lambda i: (i, 0)))
        args.append(add)
    return pl.pallas_call(
        body,
        grid=grid,
        in_specs=in_specs,
        out_specs=pl.BlockSpec((br, n), lambda i: (i, 0)),
        out_shape=jax.ShapeDtypeStruct((m, n), jnp.float32),
        interpret=_INTERPRET,
    )(*args)


def _mm_qkvs_body(x_ref, w_ref, b_ref, q_ref, kv_ref, xs_ref):
    y = jnp.dot(x_ref[...], w_ref[...], preferred_element_type=jnp.float32)
    y = y + b_ref[...]
    q_ref[...] = y[:, :_HID]
    kv_ref[...] = y[:, _HID:3 * _HID]
    xs_ref[...] = y[:, 3 * _HID:]


def _mm_qkvs(x, w, b, br=_NBR):
    grid = (_N // br,)
    return pl.pallas_call(
        _mm_qkvs_body,
        grid=grid,
        in_specs=[
            pl.BlockSpec((br, _HID), lambda i: (i, 0)),
            pl.BlockSpec((_HID, 4 * _HID), lambda i: (0, 0)),
            pl.BlockSpec((1, 4 * _HID), lambda i: (0, 0)),
        ],
        out_specs=[
            pl.BlockSpec((br, _HID), lambda i: (i, 0)),
            pl.BlockSpec((br, 2 * _HID), lambda i: (i, 0)),
            pl.BlockSpec((br, _HID), lambda i: (i, 0)),
        ],
        out_shape=[
            jax.ShapeDtypeStruct((_N, _HID), jnp.float32),
            jax.ShapeDtypeStruct((_N, 2 * _HID), jnp.float32),
            jax.ShapeDtypeStruct((_N, _HID), jnp.float32),
        ],
        interpret=_INTERPRET,
    )(x, w, b.reshape(1, 4 * _HID))


def _norm_body(x_ref, a_ref, w_ref, b_ref, ms_ref, o_ref):
    x = x_ref[...] + a_ref[...]
    mean = jnp.mean(x, axis=0, keepdims=True)
    out = x - ms_ref[...] * mean
    var = jnp.mean(out * out, axis=0, keepdims=True)
    o_ref[...] = w_ref[...] * out / jnp.sqrt(var + 1e-5) + b_ref[...]


def _graph_norm(x, add, w, b, ms):
    return pl.pallas_call(
        _norm_body,
        out_shape=jax.ShapeDtypeStruct((_N, _HID), jnp.float32),
        interpret=_INTERPRET,
    )(x, add, w.reshape(1, _HID), b.reshape(1, _HID), ms.reshape(1, _HID))


def _tc_edge_body(qd_ref, ks_ref, vs_ref, eat_ref, wp_ref, bp_ref,
                  msg_ref, wd_ref):
    e = jnp.dot(eat_ref[...], wp_ref[...],
                preferred_element_type=jnp.float32) + bp_ref[...]
    m = qd_ref[...] * (ks_ref[...] + e)
    a = _headsum(m) * (1.0 / math.sqrt(float(_C)))
    w = jnp.exp(a)
    msg_ref[...] = (vs_ref[...] + e) * _headexpand(w)
    wd_ref[:, :_H] = w
    wd_ref[:, _H:] = jnp.zeros((_EBR, 128 - _H), jnp.float32)


def _tconv_edge(qd, kvg, eattr, wp, bp):
    grid = (_E // _EBR,)
    return pl.pallas_call(
        _tc_edge_body,
        grid=grid,
        in_specs=[
            pl.BlockSpec((_EBR, _HID), lambda i: (i, 0)),
            pl.BlockSpec((_EBR, _HID), lambda i: (i, 0)),
            pl.BlockSpec((_EBR, _HID), lambda i: (i, 1)),
            pl.BlockSpec((_EBR, _EDGE_DIM), lambda i: (i, 0)),
            pl.BlockSpec((_EDGE_DIM, _HID), lambda i: (0, 0)),
            pl.BlockSpec((1, _HID), lambda i: (0, 0)),
        ],
        out_specs=[
            pl.BlockSpec((_EBR, _HID), lambda i: (i, 0)),
            pl.BlockSpec((_EBR, 128), lambda i: (i, 0)),
        ],
        out_shape=[
            jax.ShapeDtypeStruct((_E, _HID), jnp.float32),
            jax.ShapeDtypeStruct((_E, 128), jnp.float32),
        ],
        interpret=_INTERPRET,
    )(qd, kvg, kvg, eattr, wp, bp.reshape(1, _HID))


def _tc_final_body(num_ref, den_ref, xs_ref, wa_ref, wb_ref, o_ref):
    den = den_ref[0, :, :_H] + den_ref[1, :, :_H]
    denw = _headexpand(den)
    out = num_ref[...] / (denw + 1e-16)
    xs = xs_ref[...]
    logit = jnp.sum(out * wa_ref[...] + xs * wb_ref[...], axis=1, keepdims=True)
    beta = jax.nn.sigmoid(logit)
    o_ref[...] = beta * xs + (1.0 - beta) * out


def _tconv_final(num, den, xs, wa, wb):
    return pl.pallas_call(
        _tc_final_body,
        out_shape=jax.ShapeDtypeStruct((_N, _HID), jnp.float32),
        interpret=_INTERPRET,
    )(num, den, xs, wa.reshape(1, _HID), wb.reshape(1, _HID))


def _gat_node_body(hh_ref, adf_ref, ad_ref):
    ad = _headsum(hh_ref[...] * adf_ref[...])
    ad_ref[:, :_H] = ad
    ad_ref[:, _H:] = jnp.zeros((_N, 128 - _H), jnp.float32)


def _gat_node(hh, adf):
    return pl.pallas_call(
        _gat_node_body,
        out_shape=jax.ShapeDtypeStruct((_N, 128), jnp.float32),
        interpret=_INTERPRET,
    )(hh, adf.reshape(1, _HID))


def _gat_edge_body(hs_ref, adg_ref, eat_ref, wp_ref, bp_ref,
                   asf_ref, aef_ref, msg_ref, wd_ref):
    hs = hs_ref[...]
    e = jnp.dot(eat_ref[...], wp_ref[...],
                preferred_element_type=jnp.float32) + bp_ref[...]
    asg = _headsum(hs * asf_ref[...])
    ae = _headsum(e * aef_ref[...])
    a = asg + adg_ref[:, :_H] + ae
    a = jnp.where(a >= 0.0, a, 0.2 * a)
    w = jnp.exp(a)
    msg_ref[...] = hs * _headexpand(w)
    wd_ref[:, :_H] = w
    wd_ref[:, _H:] = jnp.zeros((_EBR, 128 - _H), jnp.float32)


def _gat_edge(hs, adg, eattr, wp, bp, asf, aef):
    grid = (_E // _EBR,)
    return pl.pallas_call(
        _gat_edge_body,
        grid=grid,
        in_specs=[
            pl.BlockSpec((_EBR, _HID), lambda i: (i, 0)),
            pl.BlockSpec((_EBR, 128), lambda i: (i, 0)),
            pl.BlockSpec((_EBR, _EDGE_DIM), lambda i: (i, 0)),
            pl.BlockSpec((_EDGE_DIM, _HID), lambda i: (0, 0)),
            pl.BlockSpec((1, _HID), lambda i: (0, 0)),
            pl.BlockSpec((1, _HID), lambda i: (0, 0)),
            pl.BlockSpec((1, _HID), lambda i: (0, 0)),
        ],
        out_specs=[
            pl.BlockSpec((_EBR, _HID), lambda i: (i, 0)),
            pl.BlockSpec((_EBR, 128), lambda i: (i, 0)),
        ],
        out_shape=[
            jax.ShapeDtypeStruct((_E, _HID), jnp.float32),
            jax.ShapeDtypeStruct((_E, 128), jnp.float32),
        ],
        interpret=_INTERPRET,
    )(hs, adg, eattr, wp, bp.reshape(1, _HID),
      asf.reshape(1, _HID), aef.reshape(1, _HID))


def _gat_final_body(num_ref, den_ref, b_ref, o_ref):
    den = den_ref[0, :, :_H] + den_ref[1, :, :_H]
    denw = _headexpand(den)
    o_ref[...] = num_ref[...] / (denw + 1e-16) + b_ref[...]


def _gat_final(num, den, bias):
    return pl.pallas_call(
        _gat_final_body,
        out_shape=jax.ShapeDtypeStruct((_N, _HID), jnp.float32),
        interpret=_INTERPRET,
    )(num, den, bias.reshape(1, _HID))


def _gin1_body(h_ref, agg_ref, eps_ref, w_ref, b_ref, g_ref, bt_ref, o_ref):
    h0 = (1.0 + eps_ref[0, 0]) * h_ref[...] + agg_ref[...]
    h1 = jnp.dot(h0, w_ref[...], preferred_element_type=jnp.float32) + b_ref[...]
    h1 = g_ref[...] * h1 / jnp.sqrt(1.0 + 1e-5) + bt_ref[...]
    o_ref[...] = jnp.maximum(h1, 0.0)


def _gin1(h, agg, eps, w1, b1, gamma, beta, br=_NBR):
    grid = (_N // br,)
    n2 = w1.shape[1]
    return pl.pallas_call(
        _gin1_body,
        grid=grid,
        in_specs=[
            pl.BlockSpec((br, _HID), lambda i: (i, 0)),
            pl.BlockSpec((br, _HID), lambda i: (i, 0)),
            pl.BlockSpec((1, 1), lambda i: (0, 0)),
            pl.BlockSpec((_HID, n2), lambda i: (0, 0)),
            pl.BlockSpec((1, n2), lambda i: (0, 0)),
            pl.BlockSpec((1, n2), lambda i: (0, 0)),
            pl.BlockSpec((1, n2), lambda i: (0, 0)),
        ],
        out_specs=pl.BlockSpec((br, n2), lambda i: (i, 0)),
        out_shape=jax.ShapeDtypeStruct((_N, n2), jnp.float32),
        interpret=_INTERPRET,
    )(h, agg, eps.reshape(1, 1), w1, b1.reshape(1, n2),
      gamma.reshape(1, n2), beta.reshape(1, n2))


_PBR = 1000


def _pool1_body(h_ref, w1_ref, b1_ref, w2_ref, bb2_ref, l_ref, m_ref):
    t = jnp.tanh(jnp.dot(h_ref[...], w1_ref[...],
                         preferred_element_type=jnp.float32) + b1_ref[...])
    logit = jnp.sum(t * w2_ref[...], axis=1, keepdims=True) + bb2_ref[0, 0]
    l_ref[...] = logit
    bm = jnp.max(logit, axis=0, keepdims=True)

    @pl.when(pl.program_id(0) == 0)
    def _():
        m_ref[...] = bm

    @pl.when(pl.program_id(0) != 0)
    def _():
        m_ref[...] = jnp.maximum(m_ref[...], bm)


def _pool2_body(h_ref, l_ref, g_ref, br_ref, bc_ref, s_ref, mx_ref, at_ref,
                cnt_ref, es_ref):
    h = h_ref[...]
    brow = br_ref[0]
    oh_t = (lax.broadcasted_iota(jnp.int32, (_G, _PBR), 0)
            == brow).astype(jnp.float32)
    ex = jnp.exp(l_ref[...] - g_ref[...])
    s_blk = jnp.dot(oh_t, h, preferred_element_type=jnp.float32)
    at_blk = jnp.dot(oh_t, h * ex, preferred_element_type=jnp.float32)
    cnt_blk = jnp.sum(oh_t, axis=1, keepdims=True)
    es_blk = jnp.sum(ex, axis=0, keepdims=True)

    first = pl.program_id(0) == 0

    @pl.when(first)
    def _():
        s_ref[...] = s_blk
        at_ref[...] = at_blk
        cnt_ref[...] = cnt_blk
        es_ref[...] = es_blk

    @pl.when(jnp.logical_not(first))
    def _():
        s_ref[...] = s_ref[...] + s_blk
        at_ref[...] = at_ref[...] + at_blk
        cnt_ref[...] = cnt_ref[...] + cnt_blk
        es_ref[...] = es_ref[...] + es_blk

    bcol = bc_ref[...]
    for g in range(_G):
        mg = jnp.max(jnp.where(bcol == g, h, -jnp.inf), axis=0, keepdims=True)

        @pl.when(first)
        def _():
            mx_ref[pl.ds(g, 1), :] = mg

        @pl.when(jnp.logical_not(first))
        def _():
            mx_ref[pl.ds(g, 1), :] = jnp.maximum(mx_ref[pl.ds(g, 1), :], mg)


def _pool3_body(s_ref, mx_ref, at_ref, cnt_ref, es_ref,
                ow1_ref, ob1_ref, ow2_ref, ob2_ref, o_ref):
    s = s_ref[...]
    cnt = cnt_ref[...]
    mean = s / jnp.maximum(cnt, 1.0)
    attn = at_ref[...] / es_ref[0, 0]
    pooled = jnp.concatenate([mean, mx_ref[...], s, attn], axis=1)
    o1 = jnp.dot(pooled, ow1_ref[...], preferred_element_type=jnp.float32)
    o1 = jnp.maximum(o1 + ob1_ref[...], 0.0)
    o_ref[...] = jnp.dot(o1, ow2_ref[...],
                         preferred_element_type=jnp.float32) + ob2_ref[...]


def _pool_head(h, batch_row, p):
    grid = (_N // _PBR,)
    nh = p["ap_W1"].shape[1]
    logits, gmax = pl.pallas_call(
        _pool1_body,
        grid=grid,
        in_specs=[
            pl.BlockSpec((_PBR, _HID), lambda i: (i, 0)),
            pl.BlockSpec((_HID, nh), lambda i: (0, 0)),
            pl.BlockSpec((1, nh), lambda i: (0, 0)),
            pl.BlockSpec((1, nh), lambda i: (0, 0)),
            pl.BlockSpec((1, 1), lambda i: (0, 0)),
        ],
        out_specs=[
            pl.BlockSpec((_PBR, 1), lambda i: (i, 0)),
            pl.BlockSpec((1, 1), lambda i: (0, 0)),
        ],
        out_shape=[
            jax.ShapeDtypeStruct((_N, 1), jnp.float32),
            jax.ShapeDtypeStruct((1, 1), jnp.float32),
        ],
        interpret=_INTERPRET,
    )(h, p["ap_W1"], p["ap_b1"].reshape(1, nh),
      p["ap_W2"].reshape(1, nh), p["ap_b2"].reshape(1, 1))

    s, mx, at, cnt, es = pl.pallas_call(
        _pool2_body,
        grid=grid,
        in_specs=[
            pl.BlockSpec((_PBR, _HID), lambda i: (i, 0)),
            pl.BlockSpec((_PBR, 1), lambda i: (i, 0)),
            pl.BlockSpec((1, 1), lambda i: (0, 0)),
            pl.BlockSpec((1, 1, _PBR), lambda i: (i, 0, 0)),
            pl.BlockSpec((_PBR, 1), lambda i: (i, 0)),
        ],
        out_specs=[
            pl.BlockSpec((_G, _HID), lambda i: (0, 0)),
            pl.BlockSpec((_G, _HID), lambda i: (0, 0)),
            pl.BlockSpec((_G, _HID), lambda i: (0, 0)),
            pl.BlockSpec((_G, 1), lambda i: (0, 0)),
            pl.BlockSpec((1, 1), lambda i: (0, 0)),
        ],
        out_shape=[
            jax.ShapeDtypeStruct((_G, _HID), jnp.float32),
            jax.ShapeDtypeStruct((_G, _HID), jnp.float32),
            jax.ShapeDtypeStruct((_G, _HID), jnp.float32),
            jax.ShapeDtypeStruct((_G, 1), jnp.float32),
            jax.ShapeDtypeStruct((1, 1), jnp.float32),
        ],
        interpret=_INTERPRET,
    )(h, logits, gmax, batch_row.reshape(_N // _PBR, 1, _PBR),
      batch_row.reshape(_N, 1))

    return pl.pallas_call(
        _pool3_body,
        out_shape=jax.ShapeDtypeStruct((_G, _HID), jnp.float32),
        interpret=_INTERPRET,
    )(s, mx, at, cnt, es,
      p["out_W1"], p["out_b1"].reshape(1, -1),
      p["out_W2"], p["out_b2"].reshape(1, -1))


_NW = 32
_BPW = _E // _NW


def _sc_gather_fn(d, ch):
    n_full, rem = divmod(_BPW, ch)
    sizes = [ch] * n_full + ([rem] if rem else [])
    offs = [i * ch for i in range(len(sizes))]
    mesh = plsc.VectorSubcoreMesh(core_axis_name="c", subcore_axis_name="s",
                                  num_cores=2)

    @functools.partial(
        pl.kernel,
        mesh=mesh,
        out_type=jax.ShapeDtypeStruct((_E, d), jnp.float32),
        scratch_types=[
            pltpu.VMEM((_BPW,), jnp.int32),
            pltpu.VMEM((ch, d), jnp.float32),
            pltpu.VMEM((ch, d), jnp.float32),
            pltpu.SemaphoreType.DMA,
            pltpu.SemaphoreType.DMA,
        ],
    )
    def k(tab_hbm, idx_hbm, out_hbm, idx_v, buf0, buf1, sem0, sem1):
        wid = lax.axis_index("s") * 2 + lax.axis_index("c")
        base = wid * _BPW
        pltpu.sync_copy(idx_hbm.at[pl.ds(base, _BPW)], idx_v)
        bufs = (buf0, buf1)
        sems = (sem0, sem1)
        cps = [None, None]
        cps[0] = pltpu.async_copy(
            tab_hbm.at[idx_v.at[pl.ds(0, sizes[0])]],
            bufs[0].at[pl.ds(0, sizes[0])], sems[0])
        for i in range(len(sizes)):
            if i + 1 < len(sizes):
                cps[(i + 1) % 2] = pltpu.async_copy(
                    tab_hbm.at[idx_v.at[pl.ds(offs[i + 1], sizes[i + 1])]],
                    bufs[(i + 1) % 2].at[pl.ds(0, sizes[i + 1])],
                    sems[(i + 1) % 2])
            cps[i % 2].wait()
            pltpu.sync_copy(bufs[i % 2].at[pl.ds(0, sizes[i])],
                            out_hbm.at[pl.ds(base + offs[i], sizes[i])])

    return k


_SC_GATHER_CACHE = {}
_GATHER_CHUNK = {256: 200, 512: 96, 128: 200}


def _gather_rows(tab, idx):
    d = tab.shape[1]
    if d not in _GATHER_CHUNK:
        return jnp.take(tab, idx, axis=0)
    if d not in _SC_GATHER_CACHE:
        _SC_GATHER_CACHE[d] = _sc_gather_fn(d, _GATHER_CHUNK[d])
    return _SC_GATHER_CACHE[d](tab, idx)


def _scatter_add_rows(rows, idx, n):
    return jax.ops.segment_sum(rows, idx, num_segments=n)


_EPT = _E // 16
_SCATTER_ECH = 80


def _sc_scatter_fn(dcols):
    ech = _SCATTER_ECH
    n_chunks = _EPT // ech
    rpt = 624
    mesh = plsc.VectorSubcoreMesh(core_axis_name="c", subcore_axis_name="s",
                                  num_cores=2)

    @functools.partial(
        pl.kernel,
        mesh=mesh,
        out_type=jax.ShapeDtypeStruct((_N, 256), jnp.float32),
        scratch_types=[
            pltpu.VMEM((n_chunks, ech), jnp.int32),
            pltpu.VMEM((ech, 128), jnp.float32),
            pltpu.VMEM((ech, 128), jnp.float32),
            pltpu.VMEM_SHARED((_N, 128), jnp.float32),
            pltpu.SemaphoreType.DMA,
            pltpu.SemaphoreType.DMA,
        ],
    )
    def k(rows_hbm, dst3_hbm, z_hbm, out_hbm,
          idx_v, buf0, buf1, accum, sem0, sem1):
        cid = lax.axis_index("c")
        sid = lax.axis_index("s")
        base = sid * _EPT
        pltpu.sync_copy(dst3_hbm.at[sid], idx_v)

        @pl.when(sid == 0)
        def _():
            pltpu.sync_copy(z_hbm, accum)

        plsc.subcore_barrier()
        bufs = (buf0, buf1)
        sems = (sem0, sem1)
        cps = [None, None]
        cps[0] = pltpu.async_copy(
            rows_hbm.at[pl.ds(base, ech), pl.ds(cid * 128, 128)],
            bufs[0], sems[0])
        for i in range(n_chunks):
            if i + 1 < n_chunks:
                cps[(i + 1) % 2] = pltpu.async_copy(
                    rows_hbm.at[pl.ds(base + (i + 1) * ech, ech),
                                pl.ds(cid * 128, 128)],
                    bufs[(i + 1) % 2], sems[(i + 1) % 2])
            cps[i % 2].wait()
            pltpu.sync_copy(bufs[i % 2], accum.at[idx_v.at[i]], add=True)
        plsc.subcore_barrier()

        @pl.when(sid < 15)
        def _():
            pltpu.sync_copy(
                accum.at[pl.ds(sid * rpt, rpt)],
                out_hbm.at[pl.ds(sid * rpt, rpt), pl.ds(cid * 128, 128)])

        @pl.when(sid == 15)
        def _():
            pltpu.sync_copy(
                accum.at[pl.ds(15 * rpt, _N - 15 * rpt)],
                out_hbm.at[pl.ds(15 * rpt, _N - 15 * rpt),
                           pl.ds(cid * 128, 128)])

    return k


_DEN_ECH = 40
_DEN_BPW = _E // 32


def _sc_scatter_den_fn():
    ech = _DEN_ECH
    n_chunks = _DEN_BPW // ech
    rpt = 624
    mesh = plsc.VectorSubcoreMesh(core_axis_name="c", subcore_axis_name="s",
                                  num_cores=2)

    @functools.partial(
        pl.kernel,
        mesh=mesh,
        out_type=jax.ShapeDtypeStruct((2, _N, 128), jnp.float32),
        scratch_types=[
            pltpu.VMEM((n_chunks, ech), jnp.int32),
            pltpu.VMEM((ech, 128), jnp.float32),
            pltpu.VMEM((ech, 128), jnp.float32),
            pltpu.VMEM_SHARED((_N, 128), jnp.float32),
            pltpu.SemaphoreType.DMA,
            pltpu.SemaphoreType.DMA,
        ],
    )
    def k(rows_hbm, dst4_hbm, z_hbm, out_hbm,
          idx_v, buf0, buf1, accum, sem0, sem1):
        cid = lax.axis_index("c")
        sid = lax.axis_index("s")
        wid = cid * 16 + sid
        base = wid * _DEN_BPW
        pltpu.sync_copy(dst4_hbm.at[wid], idx_v)

        @pl.when(sid == 0)
        def _():
            pltpu.sync_copy(z_hbm, accum)

        plsc.subcore_barrier()
        bufs = (buf0, buf1)
        sems = (sem0, sem1)
        cps = [None, None]
        cps[0] = pltpu.async_copy(rows_hbm.at[pl.ds(base, ech)],
                                  bufs[0], sems[0])
        for i in range(n_chunks):
            if i + 1 < n_chunks:
                cps[(i + 1) % 2] = pltpu.async_copy(
                    rows_hbm.at[pl.ds(base + (i + 1) * ech, ech)],
                    bufs[(i + 1) % 2], sems[(i + 1) % 2])
            cps[i % 2].wait()
            pltpu.sync_copy(bufs[i % 2], accum.at[idx_v.at[i]], add=True)
        plsc.subcore_barrier()

        @pl.when(sid < 15)
        def _():
            pltpu.sync_copy(accum.at[pl.ds(sid * rpt, rpt)],
                            out_hbm.at[cid, pl.ds(sid * rpt, rpt)])

        @pl.when(sid == 15)
        def _():
            pltpu.sync_copy(accum.at[pl.ds(15 * rpt, _N - 15 * rpt)],
                            out_hbm.at[cid, pl.ds(15 * rpt, _N - 15 * rpt)])

    return k


_SC_SCATTER_CACHE = {}


def _sc_scatter_den(rows, dst4, zeros_half):
    if "den" not in _SC_SCATTER_CACHE:
        _SC_SCATTER_CACHE["den"] = _sc_scatter_den_fn()
    return _SC_SCATTER_CACHE["den"](rows, dst4, zeros_half)


def _sc_scatter256(rows, dst3, zeros_half):
    d = rows.shape[1]
    if d not in _SC_SCATTER_CACHE:
        _SC_SCATTER_CACHE[d] = _sc_scatter_fn(d)
    return _SC_SCATTER_CACHE[d](rows, dst3, zeros_half)


def kernel(x, edge_index, edge_attr, batch, params):
    src = edge_index[0]
    dst = edge_index[1]
    batch_row = batch.reshape(1, _N)
    dst3 = dst.reshape(16, _EPT // _SCATTER_ECH, _SCATTER_ECH)
    dst4 = dst.reshape(32, _DEN_BPW // _DEN_ECH, _DEN_ECH)
    zhalf = jnp.zeros((_N, 128), jnp.float32)

    h = _mm(x, params["node_W"], params["node_b"])
    ew = params["edge_W"]
    eb = params["edge_b"]

    for i in range(6):
        p = params["layer%d" % i]
        if i % 3 == 0:
            wq = jnp.concatenate([p["Wq"], p["Wk"], p["Wv"], p["Wskip"]], axis=1)
            bq = jnp.concatenate([p["bq"], p["bk"], p["bv"], p["bskip"]])
            q, kv, xs = _mm_qkvs(h, wq, bq)
            wp = ew @ p["We"]
            bp = eb @ p["We"] + p["be"]
            qd = _gather_rows(q, dst)
            kvg = _gather_rows(kv, src)
            msg, wden = _tconv_edge(qd, kvg, edge_attr, wp, bp)
            num = _sc_scatter256(msg, dst3, zhalf)
            den = _sc_scatter_den(wden, dst4, zhalf)
            wb = p["Wbeta"].reshape(3, _HID)
            wa_out = wb[0] + wb[2]
            wa_xs = wb[1] - wb[2]
            xa = _tconv_final(num, den, xs, wa_out, wa_xs)
            h2 = _graph_norm(h, xa, p["n_w"], p["n_b"], p["n_ms"])
            f1 = _mm(h2, p["f_W1"], p["f_b1"], act="gelu")
            x_new = _mm(f1, p["f_W2"], p["f_b2"], add=h2)
        elif i % 3 == 1:
            hh = _mm(h, p["W"], jnp.zeros((_HID,), jnp.float32))
            adst = _gat_node(hh, p["att_dst"].reshape(_HID))
            wp = ew @ p["We"]
            bp = eb @ p["We"]
            adg = _gather_rows(adst, dst)
            hs = _gather_rows(hh, src)
            msg, wden = _gat_edge(hs, adg, edge_attr, wp, bp,
                                  p["att_src"].reshape(_HID),
                                  p["att_edge"].reshape(_HID))
            num = _sc_scatter256(msg, dst3, zhalf)
            den = _sc_scatter_den(wden, dst4, zhalf)
            x_new = _gat_final(num, den, p["bias"])
        else:
            agg = _sc_scatter256(_gather_rows(h, src), dst3, zhalf)
            t = _gin1(h, agg, p["eps"], p["W1"], p["b1"],
                      p["gamma"], p["beta"])
            x_new = _mm(t, p["W2"], p["b2"])
        q = params["norm%d" % i]
        h = _graph_norm(h, x_new, q["w"], q["b"], q["ms"])

    return _pool_head(h, batch_row, params)


# speedup vs baseline: 5.3306x; 1.0156x over previous
import functools
import math

import jax
import jax.numpy as jnp
from jax import lax
from jax.experimental import pallas as pl
from jax.experimental.pallas import tpu as pltpu
from jax.experimental.pallas import tpu_sc as plsc

_N = 10000
_E = 160000
_HID = 256
_H = 8
_C = 32
_G = 16
_EDGE_DIM = 10

_INTERPRET = False

_EBR = 4000
_NBR = 2000


def _headsum(m):
    r = lax.broadcasted_iota(jnp.int32, (_HID, _H), 0) // _C
    c = lax.broadcasted_iota(jnp.int32, (_HID, _H), 1)
    s = (r == c).astype(jnp.float32)
    return jnp.dot(m, s, preferred_element_type=jnp.float32)


def _headexpand(w):
    r = lax.broadcasted_iota(jnp.int32, (_H, _HID), 0)
    c = lax.broadcasted_iota(jnp.int32, (_H, _HID), 1) // _C
    s = (r == c).astype(jnp.float32)
    return jnp.dot(w, s, preferred_element_type=jnp.float32)


def _mm_body(act, x_ref, w_ref, b_ref, o_ref):
    y = jnp.dot(x_ref[...], w_ref[...], preferred_element_type=jnp.float32)
    y = y + b_ref[...]
    if act == "relu":
        y = jnp.maximum(y, 0.0)
    elif act == "gelu":
        y = jax.nn.gelu(y)
    o_ref[...] = y


def _mm_add_body(act, x_ref, w_ref, b_ref, a_ref, o_ref):
    y = jnp.dot(x_ref[...], w_ref[...], preferred_element_type=jnp.float32)
    y = y + b_ref[...]
    if act == "relu":
        y = jnp.maximum(y, 0.0)
    elif act == "gelu":
        y = jax.nn.gelu(y)
    o_ref[...] = y + a_ref[...]


def _mm(x, w, b, act=None, br=_NBR, add=None):
    m, k = x.shape
    n = w.shape[1]
    grid = (m // br,)
    in_specs = [
        pl.BlockSpec((br, k), lambda i: (i, 0)),
        pl.BlockSpec((k, n), lambda i: (0, 0)),
        pl.BlockSpec((1, n), lambda i: (0, 0)),
    ]
    args = [x, w, b.reshape(1, n)]
    if add is None:
        body = functools.partial(_mm_body, act)
    else:
        body = functools.partial(_mm_add_body, act)
        in_specs.append(pl.BlockSpec((br, n), lambda i: (i, 0)))
        args.append(add)
    return pl.pallas_call(
        body,
        grid=grid,
        in_specs=in_specs,
        out_specs=pl.BlockSpec((br, n), lambda i: (i, 0)),
        out_shape=jax.ShapeDtypeStruct((m, n), jnp.float32),
        interpret=_INTERPRET,
    )(*args)


def _mm_qkvs_body(x_ref, w_ref, b_ref, q_ref, kv_ref, xs_ref):
    y = jnp.dot(x_ref[...], w_ref[...], preferred_element_type=jnp.float32)
    y = y + b_ref[...]
    q_ref[...] = y[:, :_HID]
    kv_ref[...] = y[:, _HID:3 * _HID]
    xs_ref[...] = y[:, 3 * _HID:]


def _mm_qkvs(x, w, b, br=_NBR):
    grid = (_N // br,)
    return pl.pallas_call(
        _mm_qkvs_body,
        grid=grid,
        in_specs=[
            pl.BlockSpec((br, _HID), lambda i: (i, 0)),
            pl.BlockSpec((_HID, 4 * _HID), lambda i: (0, 0)),
            pl.BlockSpec((1, 4 * _HID), lambda i: (0, 0)),
        ],
        out_specs=[
            pl.BlockSpec((br, _HID), lambda i: (i, 0)),
            pl.BlockSpec((br, 2 * _HID), lambda i: (i, 0)),
            pl.BlockSpec((br, _HID), lambda i: (i, 0)),
        ],
        out_shape=[
            jax.ShapeDtypeStruct((_N, _HID), jnp.float32),
            jax.ShapeDtypeStruct((_N, 2 * _HID), jnp.float32),
            jax.ShapeDtypeStruct((_N, _HID), jnp.float32),
        ],
        interpret=_INTERPRET,
    )(x, w, b.reshape(1, 4 * _HID))


def _norm_body(x_ref, a_ref, w_ref, b_ref, ms_ref, o_ref):
    x = x_ref[...] + a_ref[...]
    mean = jnp.mean(x, axis=0, keepdims=True)
    out = x - ms_ref[...] * mean
    var = jnp.mean(out * out, axis=0, keepdims=True)
    o_ref[...] = w_ref[...] * out / jnp.sqrt(var + 1e-5) + b_ref[...]


def _norm2_body(x_ref, a_ref, w_ref, b_ref, ms_ref, o_ref, o2_ref):
    x = x_ref[...] + a_ref[...]
    mean = jnp.mean(x, axis=0, keepdims=True)
    out = x - ms_ref[...] * mean
    var = jnp.mean(out * out, axis=0, keepdims=True)
    y = w_ref[...] * out / jnp.sqrt(var + 1e-5) + b_ref[...]
    o_ref[...] = y
    o2_ref[0] = y[:, :128]
    o2_ref[1] = y[:, 128:]


def _graph_norm(x, add, w, b, ms, halves=False):
    args = (x, add, w.reshape(1, _HID), b.reshape(1, _HID),
            ms.reshape(1, _HID))
    if not halves:
        return pl.pallas_call(
            _norm_body,
            out_shape=jax.ShapeDtypeStruct((_N, _HID), jnp.float32),
            interpret=_INTERPRET,
        )(*args)
    return pl.pallas_call(
        _norm2_body,
        out_shape=[
            jax.ShapeDtypeStruct((_N, _HID), jnp.float32),
            jax.ShapeDtypeStruct((2, _N, 128), jnp.float32),
        ],
        interpret=_INTERPRET,
    )(*args)


def _tc_edge_body(qd_ref, ks_ref, vs_ref, eat_ref, wp_ref, bp_ref,
                  msg_ref, wd_ref):
    e = jnp.dot(eat_ref[...], wp_ref[...],
                preferred_element_type=jnp.float32) + bp_ref[...]
    m = qd_ref[...] * (ks_ref[...] + e)
    a = _headsum(m) * (1.0 / math.sqrt(float(_C)))
    w = jnp.exp(a)
    msg_ref[...] = (vs_ref[...] + e) * _headexpand(w)
    wd_ref[:, :_H] = w
    wd_ref[:, _H:] = jnp.zeros((_EBR, 128 - _H), jnp.float32)


def _tconv_edge(qd, kvg, eattr, wp, bp):
    grid = (_E // _EBR,)
    return pl.pallas_call(
        _tc_edge_body,
        grid=grid,
        in_specs=[
            pl.BlockSpec((_EBR, _HID), lambda i: (i, 0)),
            pl.BlockSpec((_EBR, _HID), lambda i: (i, 0)),
            pl.BlockSpec((_EBR, _HID), lambda i: (i, 1)),
            pl.BlockSpec((_EBR, _EDGE_DIM), lambda i: (i, 0)),
            pl.BlockSpec((_EDGE_DIM, _HID), lambda i: (0, 0)),
            pl.BlockSpec((1, _HID), lambda i: (0, 0)),
        ],
        out_specs=[
            pl.BlockSpec((_EBR, _HID), lambda i: (i, 0)),
            pl.BlockSpec((_EBR, 128), lambda i: (i, 0)),
        ],
        out_shape=[
            jax.ShapeDtypeStruct((_E, _HID), jnp.float32),
            jax.ShapeDtypeStruct((_E, 128), jnp.float32),
        ],
        interpret=_INTERPRET,
    )(qd, kvg, kvg, eattr, wp, bp.reshape(1, _HID))


def _tc_final_body(num_ref, den_ref, xs_ref, wa_ref, wb_ref, o_ref):
    den = den_ref[0, :, :_H] + den_ref[1, :, :_H]
    denw = _headexpand(den)
    out = num_ref[...] / (denw + 1e-16)
    xs = xs_ref[...]
    logit = jnp.sum(out * wa_ref[...] + xs * wb_ref[...], axis=1, keepdims=True)
    beta = jax.nn.sigmoid(logit)
    o_ref[...] = beta * xs + (1.0 - beta) * out


def _tconv_final(num, den, xs, wa, wb):
    return pl.pallas_call(
        _tc_final_body,
        out_shape=jax.ShapeDtypeStruct((_N, _HID), jnp.float32),
        interpret=_INTERPRET,
    )(num, den, xs, wa.reshape(1, _HID), wb.reshape(1, _HID))


def _gat_node_body(hh_ref, adf_ref, ad_ref):
    ad = _headsum(hh_ref[...] * adf_ref[...])
    ad_ref[:, :_H] = ad
    ad_ref[:, _H:] = jnp.zeros((_N, 128 - _H), jnp.float32)


def _gat_node(hh, adf):
    return pl.pallas_call(
        _gat_node_body,
        out_shape=jax.ShapeDtypeStruct((_N, 128), jnp.float32),
        interpret=_INTERPRET,
    )(hh, adf.reshape(1, _HID))


def _gat_edge_body(hs_ref, adg_ref, eat_ref, wp_ref, bp_ref,
                   asf_ref, aef_ref, msg_ref, wd_ref):
    hs = hs_ref[...]
    e = jnp.dot(eat_ref[...], wp_ref[...],
                preferred_element_type=jnp.float32) + bp_ref[...]
    asg = _headsum(hs * asf_ref[...])
    ae = _headsum(e * aef_ref[...])
    a = asg + adg_ref[:, :_H] + ae
    a = jnp.where(a >= 0.0, a, 0.2 * a)
    w = jnp.exp(a)
    msg_ref[...] = hs * _headexpand(w)
    wd_ref[:, :_H] = w
    wd_ref[:, _H:] = jnp.zeros((_EBR, 128 - _H), jnp.float32)


def _gat_edge(hs, adg, eattr, wp, bp, asf, aef):
    grid = (_E // _EBR,)
    return pl.pallas_call(
        _gat_edge_body,
        grid=grid,
        in_specs=[
            pl.BlockSpec((_EBR, _HID), lambda i: (i, 0)),
            pl.BlockSpec((_EBR, 128), lambda i: (i, 0)),
            pl.BlockSpec((_EBR, _EDGE_DIM), lambda i: (i, 0)),
            pl.BlockSpec((_EDGE_DIM, _HID), lambda i: (0, 0)),
            pl.BlockSpec((1, _HID), lambda i: (0, 0)),
            pl.BlockSpec((1, _HID), lambda i: (0, 0)),
            pl.BlockSpec((1, _HID), lambda i: (0, 0)),
        ],
        out_specs=[
            pl.BlockSpec((_EBR, _HID), lambda i: (i, 0)),
            pl.BlockSpec((_EBR, 128), lambda i: (i, 0)),
        ],
        out_shape=[
            jax.ShapeDtypeStruct((_E, _HID), jnp.float32),
            jax.ShapeDtypeStruct((_E, 128), jnp.float32),
        ],
        interpret=_INTERPRET,
    )(hs, adg, eattr, wp, bp.reshape(1, _HID),
      asf.reshape(1, _HID), aef.reshape(1, _HID))


def _gat_final_body(num_ref, den_ref, b_ref, o_ref):
    den = den_ref[0, :, :_H] + den_ref[1, :, :_H]
    denw = _headexpand(den)
    o_ref[...] = num_ref[...] / (denw + 1e-16) + b_ref[...]


def _gat_final(num, den, bias):
    return pl.pallas_call(
        _gat_final_body,
        out_shape=jax.ShapeDtypeStruct((_N, _HID), jnp.float32),
        interpret=_INTERPRET,
    )(num, den, bias.reshape(1, _HID))


def _gin1_body(h_ref, agg_ref, eps_ref, w_ref, b_ref, g_ref, bt_ref, o_ref):
    h0 = (1.0 + eps_ref[0, 0]) * h_ref[...] + agg_ref[...]
    h1 = jnp.dot(h0, w_ref[...], preferred_element_type=jnp.float32) + b_ref[...]
    h1 = g_ref[...] * h1 / jnp.sqrt(1.0 + 1e-5) + bt_ref[...]
    o_ref[...] = jnp.maximum(h1, 0.0)


def _gin1(h, agg, eps, w1, b1, gamma, beta, br=_NBR):
    grid = (_N // br,)
    n2 = w1.shape[1]
    return pl.pallas_call(
        _gin1_body,
        grid=grid,
        in_specs=[
            pl.BlockSpec((br, _HID), lambda i: (i, 0)),
            pl.BlockSpec((br, _HID), lambda i: (i, 0)),
            pl.BlockSpec((1, 1), lambda i: (0, 0)),
            pl.BlockSpec((_HID, n2), lambda i: (0, 0)),
            pl.BlockSpec((1, n2), lambda i: (0, 0)),
            pl.BlockSpec((1, n2), lambda i: (0, 0)),
            pl.BlockSpec((1, n2), lambda i: (0, 0)),
        ],
        out_specs=pl.BlockSpec((br, n2), lambda i: (i, 0)),
        out_shape=jax.ShapeDtypeStruct((_N, n2), jnp.float32),
        interpret=_INTERPRET,
    )(h, agg, eps.reshape(1, 1), w1, b1.reshape(1, n2),
      gamma.reshape(1, n2), beta.reshape(1, n2))


_PBR = 1000


def _pool1_body(h_ref, w1_ref, b1_ref, w2_ref, bb2_ref, l_ref, m_ref):
    t = jnp.tanh(jnp.dot(h_ref[...], w1_ref[...],
                         preferred_element_type=jnp.float32) + b1_ref[...])
    logit = jnp.sum(t * w2_ref[...], axis=1, keepdims=True) + bb2_ref[0, 0]
    l_ref[...] = logit
    bm = jnp.max(logit, axis=0, keepdims=True)

    @pl.when(pl.program_id(0) == 0)
    def _():
        m_ref[...] = bm

    @pl.when(pl.program_id(0) != 0)
    def _():
        m_ref[...] = jnp.maximum(m_ref[...], bm)


def _pool2_body(h_ref, l_ref, g_ref, br_ref, bc_ref, s_ref, mx_ref, at_ref,
                cnt_ref, es_ref):
    h = h_ref[...]
    brow = br_ref[0]
    oh_t = (lax.broadcasted_iota(jnp.int32, (_G, _PBR), 0)
            == brow).astype(jnp.float32)
    ex = jnp.exp(l_ref[...] - g_ref[...])
    s_blk = jnp.dot(oh_t, h, preferred_element_type=jnp.float32)
    at_blk = jnp.dot(oh_t, h * ex, preferred_element_type=jnp.float32)
    cnt_blk = jnp.sum(oh_t, axis=1, keepdims=True)
    es_blk = jnp.sum(ex, axis=0, keepdims=True)

    first = pl.program_id(0) == 0

    @pl.when(first)
    def _():
        s_ref[...] = s_blk
        at_ref[...] = at_blk
        cnt_ref[...] = cnt_blk
        es_ref[...] = es_blk

    @pl.when(jnp.logical_not(first))
    def _():
        s_ref[...] = s_ref[...] + s_blk
        at_ref[...] = at_ref[...] + at_blk
        cnt_ref[...] = cnt_ref[...] + cnt_blk
        es_ref[...] = es_ref[...] + es_blk

    bcol = bc_ref[...]
    for g in range(_G):
        mg = jnp.max(jnp.where(bcol == g, h, -jnp.inf), axis=0, keepdims=True)

        @pl.when(first)
        def _():
            mx_ref[pl.ds(g, 1), :] = mg

        @pl.when(jnp.logical_not(first))
        def _():
            mx_ref[pl.ds(g, 1), :] = jnp.maximum(mx_ref[pl.ds(g, 1), :], mg)


def _pool3_body(s_ref, mx_ref, at_ref, cnt_ref, es_ref,
                ow1_ref, ob1_ref, ow2_ref, ob2_ref, o_ref):
    s = s_ref[...]
    cnt = cnt_ref[...]
    mean = s / jnp.maximum(cnt, 1.0)
    attn = at_ref[...] / es_ref[0, 0]
    pooled = jnp.concatenate([mean, mx_ref[...], s, attn], axis=1)
    o1 = jnp.dot(pooled, ow1_ref[...], preferred_element_type=jnp.float32)
    o1 = jnp.maximum(o1 + ob1_ref[...], 0.0)
    o_ref[...] = jnp.dot(o1, ow2_ref[...],
                         preferred_element_type=jnp.float32) + ob2_ref[...]


def _pool_head(h, batch_row, p):
    grid = (_N // _PBR,)
    nh = p["ap_W1"].shape[1]
    logits, gmax = pl.pallas_call(
        _pool1_body,
        grid=grid,
        in_specs=[
            pl.BlockSpec((_PBR, _HID), lambda i: (i, 0)),
            pl.BlockSpec((_HID, nh), lambda i: (0, 0)),
            pl.BlockSpec((1, nh), lambda i: (0, 0)),
            pl.BlockSpec((1, nh), lambda i: (0, 0)),
            pl.BlockSpec((1, 1), lambda i: (0, 0)),
        ],
        out_specs=[
            pl.BlockSpec((_PBR, 1), lambda i: (i, 0)),
            pl.BlockSpec((1, 1), lambda i: (0, 0)),
        ],
        out_shape=[
            jax.ShapeDtypeStruct((_N, 1), jnp.float32),
            jax.ShapeDtypeStruct((1, 1), jnp.float32),
        ],
        interpret=_INTERPRET,
    )(h, p["ap_W1"], p["ap_b1"].reshape(1, nh),
      p["ap_W2"].reshape(1, nh), p["ap_b2"].reshape(1, 1))

    s, mx, at, cnt, es = pl.pallas_call(
        _pool2_body,
        grid=grid,
        in_specs=[
            pl.BlockSpec((_PBR, _HID), lambda i: (i, 0)),
            pl.BlockSpec((_PBR, 1), lambda i: (i, 0)),
            pl.BlockSpec((1, 1), lambda i: (0, 0)),
            pl.BlockSpec((1, 1, _PBR), lambda i: (i, 0, 0)),
            pl.BlockSpec((_PBR, 1), lambda i: (i, 0)),
        ],
        out_specs=[
            pl.BlockSpec((_G, _HID), lambda i: (0, 0)),
            pl.BlockSpec((_G, _HID), lambda i: (0, 0)),
            pl.BlockSpec((_G, _HID), lambda i: (0, 0)),
            pl.BlockSpec((_G, 1), lambda i: (0, 0)),
            pl.BlockSpec((1, 1), lambda i: (0, 0)),
        ],
        out_shape=[
            jax.ShapeDtypeStruct((_G, _HID), jnp.float32),
            jax.ShapeDtypeStruct((_G, _HID), jnp.float32),
            jax.ShapeDtypeStruct((_G, _HID), jnp.float32),
            jax.ShapeDtypeStruct((_G, 1), jnp.float32),
            jax.ShapeDtypeStruct((1, 1), jnp.float32),
        ],
        interpret=_INTERPRET,
    )(h, logits, gmax, batch_row.reshape(_N // _PBR, 1, _PBR),
      batch_row.reshape(_N, 1))

    return pl.pallas_call(
        _pool3_body,
        out_shape=jax.ShapeDtypeStruct((_G, _HID), jnp.float32),
        interpret=_INTERPRET,
    )(s, mx, at, cnt, es,
      p["out_W1"], p["out_b1"].reshape(1, -1),
      p["out_W2"], p["out_b2"].reshape(1, -1))


_NW = 32
_BPW = _E // _NW


def _sc_gather_fn(d, ch):
    n_full, rem = divmod(_BPW, ch)
    sizes = [ch] * n_full + ([rem] if rem else [])
    offs = [i * ch for i in range(len(sizes))]
    mesh = plsc.VectorSubcoreMesh(core_axis_name="c", subcore_axis_name="s",
                                  num_cores=2)

    @functools.partial(
        pl.kernel,
        mesh=mesh,
        out_type=jax.ShapeDtypeStruct((_E, d), jnp.float32),
        scratch_types=[
            pltpu.VMEM((_BPW,), jnp.int32),
            pltpu.VMEM((ch, d), jnp.float32),
            pltpu.VMEM((ch, d), jnp.float32),
            pltpu.SemaphoreType.DMA,
            pltpu.SemaphoreType.DMA,
        ],
    )
    def k(tab_hbm, idx_hbm, out_hbm, idx_v, buf0, buf1, sem0, sem1):
        wid = lax.axis_index("s") * 2 + lax.axis_index("c")
        base = wid * _BPW
        pltpu.sync_copy(idx_hbm.at[pl.ds(base, _BPW)], idx_v)
        bufs = (buf0, buf1)
        sems = (sem0, sem1)
        cps = [None, None]
        cps[0] = pltpu.async_copy(
            tab_hbm.at[idx_v.at[pl.ds(0, sizes[0])]],
            bufs[0].at[pl.ds(0, sizes[0])], sems[0])
        for i in range(len(sizes)):
            if i + 1 < len(sizes):
                cps[(i + 1) % 2] = pltpu.async_copy(
                    tab_hbm.at[idx_v.at[pl.ds(offs[i + 1], sizes[i + 1])]],
                    bufs[(i + 1) % 2].at[pl.ds(0, sizes[i + 1])],
                    sems[(i + 1) % 2])
            cps[i % 2].wait()
            pltpu.sync_copy(bufs[i % 2].at[pl.ds(0, sizes[i])],
                            out_hbm.at[pl.ds(base + offs[i], sizes[i])])

    return k


_SC_GATHER_CACHE = {}
_GATHER_CHUNK = {256: 200, 512: 96, 128: 200}


def _gather_rows(tab, idx):
    d = tab.shape[1]
    if d not in _GATHER_CHUNK:
        return jnp.take(tab, idx, axis=0)
    if d not in _SC_GATHER_CACHE:
        _SC_GATHER_CACHE[d] = _sc_gather_fn(d, _GATHER_CHUNK[d])
    return _SC_GATHER_CACHE[d](tab, idx)


def _scatter_add_rows(rows, idx, n):
    return jax.ops.segment_sum(rows, idx, num_segments=n)


_EPT = _E // 16
_SCATTER_ECH = 80


def _sc_scatter_fn(dcols):
    ech = _SCATTER_ECH
    n_chunks = _EPT // ech
    rpt = 624
    mesh = plsc.VectorSubcoreMesh(core_axis_name="c", subcore_axis_name="s",
                                  num_cores=2)

    @functools.partial(
        pl.kernel,
        mesh=mesh,
        out_type=jax.ShapeDtypeStruct((_N, 256), jnp.float32),
        scratch_types=[
            pltpu.VMEM((n_chunks, ech), jnp.int32),
            pltpu.VMEM((ech, 128), jnp.float32),
            pltpu.VMEM((ech, 128), jnp.float32),
            pltpu.VMEM_SHARED((_N, 128), jnp.float32),
            pltpu.SemaphoreType.DMA,
            pltpu.SemaphoreType.DMA,
        ],
    )
    def k(rows_hbm, dst3_hbm, z_hbm, out_hbm,
          idx_v, buf0, buf1, accum, sem0, sem1):
        cid = lax.axis_index("c")
        sid = lax.axis_index("s")
        base = sid * _EPT
        pltpu.sync_copy(dst3_hbm.at[sid], idx_v)

        @pl.when(sid == 0)
        def _():
            pltpu.sync_copy(z_hbm, accum)

        plsc.subcore_barrier()
        bufs = (buf0, buf1)
        sems = (sem0, sem1)
        cps = [None, None]
        cps[0] = pltpu.async_copy(
            rows_hbm.at[pl.ds(base, ech), pl.ds(cid * 128, 128)],
            bufs[0], sems[0])
        for i in range(n_chunks):
            if i + 1 < n_chunks:
                cps[(i + 1) % 2] = pltpu.async_copy(
                    rows_hbm.at[pl.ds(base + (i + 1) * ech, ech),
                                pl.ds(cid * 128, 128)],
                    bufs[(i + 1) % 2], sems[(i + 1) % 2])
            cps[i % 2].wait()
            pltpu.sync_copy(bufs[i % 2], accum.at[idx_v.at[i]], add=True)
        plsc.subcore_barrier()

        @pl.when(sid < 15)
        def _():
            pltpu.sync_copy(
                accum.at[pl.ds(sid * rpt, rpt)],
                out_hbm.at[pl.ds(sid * rpt, rpt), pl.ds(cid * 128, 128)])

        @pl.when(sid == 15)
        def _():
            pltpu.sync_copy(
                accum.at[pl.ds(15 * rpt, _N - 15 * rpt)],
                out_hbm.at[pl.ds(15 * rpt, _N - 15 * rpt),
                           pl.ds(cid * 128, 128)])

    return k


_DEN_ECH = 40
_DEN_BPW = _E // 32


def _sc_scatter_den_fn():
    ech = _DEN_ECH
    n_chunks = _DEN_BPW // ech
    rpt = 624
    mesh = plsc.VectorSubcoreMesh(core_axis_name="c", subcore_axis_name="s",
                                  num_cores=2)

    @functools.partial(
        pl.kernel,
        mesh=mesh,
        out_type=jax.ShapeDtypeStruct((2, _N, 128), jnp.float32),
        scratch_types=[
            pltpu.VMEM((n_chunks, ech), jnp.int32),
            pltpu.VMEM((ech, 128), jnp.float32),
            pltpu.VMEM((ech, 128), jnp.float32),
            pltpu.VMEM_SHARED((_N, 128), jnp.float32),
            pltpu.SemaphoreType.DMA,
            pltpu.SemaphoreType.DMA,
        ],
    )
    def k(rows_hbm, dst4_hbm, z_hbm, out_hbm,
          idx_v, buf0, buf1, accum, sem0, sem1):
        cid = lax.axis_index("c")
        sid = lax.axis_index("s")
        wid = cid * 16 + sid
        base = wid * _DEN_BPW
        pltpu.sync_copy(dst4_hbm.at[wid], idx_v)

        @pl.when(sid == 0)
        def _():
            pltpu.sync_copy(z_hbm, accum)

        plsc.subcore_barrier()
        bufs = (buf0, buf1)
        sems = (sem0, sem1)
        cps = [None, None]
        cps[0] = pltpu.async_copy(rows_hbm.at[pl.ds(base, ech)],
                                  bufs[0], sems[0])
        for i in range(n_chunks):
            if i + 1 < n_chunks:
                cps[(i + 1) % 2] = pltpu.async_copy(
                    rows_hbm.at[pl.ds(base + (i + 1) * ech, ech)],
                    bufs[(i + 1) % 2], sems[(i + 1) % 2])
            cps[i % 2].wait()
            pltpu.sync_copy(bufs[i % 2], accum.at[idx_v.at[i]], add=True)
        plsc.subcore_barrier()

        @pl.when(sid < 15)
        def _():
            pltpu.sync_copy(accum.at[pl.ds(sid * rpt, rpt)],
                            out_hbm.at[cid, pl.ds(sid * rpt, rpt)])

        @pl.when(sid == 15)
        def _():
            pltpu.sync_copy(accum.at[pl.ds(15 * rpt, _N - 15 * rpt)],
                            out_hbm.at[cid, pl.ds(15 * rpt, _N - 15 * rpt)])

    return k


def _sc_gin_fn():
    ech = _SCATTER_ECH
    n_chunks = _EPT // ech
    rpt = 624
    mesh = plsc.VectorSubcoreMesh(core_axis_name="c", subcore_axis_name="s",
                                  num_cores=2)

    @functools.partial(
        pl.kernel,
        mesh=mesh,
        out_type=jax.ShapeDtypeStruct((_N, 256), jnp.float32),
        scratch_types=[
            pltpu.VMEM((n_chunks, ech), jnp.int32),
            pltpu.VMEM((n_chunks, ech), jnp.int32),
            pltpu.VMEM((ech, 128), jnp.float32),
            pltpu.VMEM_SHARED((_N, 128), jnp.float32),
            pltpu.SemaphoreType.DMA,
        ],
    )
    def k(h2c_hbm, src3_hbm, dst3_hbm, z_hbm,
          out_hbm, sidx_v, didx_v, buf0, accum, sem0):
        cid = lax.axis_index("c")
        sid = lax.axis_index("s")
        pltpu.sync_copy(src3_hbm.at[sid], sidx_v)
        pltpu.sync_copy(dst3_hbm.at[sid], didx_v)

        @pl.when(sid == 0)
        def _():
            pltpu.sync_copy(z_hbm, accum)

        plsc.subcore_barrier()
        for i in range(n_chunks):
            pltpu.async_copy(h2c_hbm.at[cid].at[sidx_v.at[i]],
                             buf0, sem0).wait()
            pltpu.sync_copy(buf0, accum.at[didx_v.at[i]], add=True)
        plsc.subcore_barrier()

        @pl.when(sid < 15)
        def _():
            pltpu.sync_copy(
                accum.at[pl.ds(sid * rpt, rpt)],
                out_hbm.at[pl.ds(sid * rpt, rpt), pl.ds(cid * 128, 128)])

        @pl.when(sid == 15)
        def _():
            pltpu.sync_copy(
                accum.at[pl.ds(15 * rpt, _N - 15 * rpt)],
                out_hbm.at[pl.ds(15 * rpt, _N - 15 * rpt),
                           pl.ds(cid * 128, 128)])

    return k


_SC_SCATTER_CACHE = {}


def _sc_gin(h2c, src3, dst3, zeros_half):
    if "gin" not in _SC_SCATTER_CACHE:
        _SC_SCATTER_CACHE["gin"] = _sc_gin_fn()
    return _SC_SCATTER_CACHE["gin"](h2c, src3, dst3, zeros_half)


def _sc_scatter_den(rows, dst4, zeros_half):
    if "den" not in _SC_SCATTER_CACHE:
        _SC_SCATTER_CACHE["den"] = _sc_scatter_den_fn()
    return _SC_SCATTER_CACHE["den"](rows, dst4, zeros_half)


def _sc_scatter256(rows, dst3, zeros_half):
    d = rows.shape[1]
    if d not in _SC_SCATTER_CACHE:
        _SC_SCATTER_CACHE[d] = _sc_scatter_fn(d)
    return _SC_SCATTER_CACHE[d](rows, dst3, zeros_half)


def kernel(x, edge_index, edge_attr, batch, params):
    src = edge_index[0]
    dst = edge_index[1]
    batch_row = batch.reshape(1, _N)
    dst3 = dst.reshape(16, _EPT // _SCATTER_ECH, _SCATTER_ECH)
    dst4 = dst.reshape(32, _DEN_BPW // _DEN_ECH, _DEN_ECH)
    src3 = src.reshape(16, _EPT // _SCATTER_ECH, _SCATTER_ECH)
    zhalf = jnp.zeros((_N, 128), jnp.float32)

    h = _mm(x, params["node_W"], params["node_b"])
    ew = params["edge_W"]
    eb = params["edge_b"]

    for i in range(6):
        p = params["layer%d" % i]
        if i % 3 == 0:
            wq = jnp.concatenate([p["Wq"], p["Wk"], p["Wv"], p["Wskip"]], axis=1)
            bq = jnp.concatenate([p["bq"], p["bk"], p["bv"], p["bskip"]])
            q, kv, xs = _mm_qkvs(h, wq, bq)
            wp = ew @ p["We"]
            bp = eb @ p["We"] + p["be"]
            qd = _gather_rows(q, dst)
            kvg = _gather_rows(kv, src)
            msg, wden = _tconv_edge(qd, kvg, edge_attr, wp, bp)
            num = _sc_scatter256(msg, dst3, zhalf)
            den = _sc_scatter_den(wden, dst4, zhalf)
            wb = p["Wbeta"].reshape(3, _HID)
            wa_out = wb[0] + wb[2]
            wa_xs = wb[1] - wb[2]
            xa = _tconv_final(num, den, xs, wa_out, wa_xs)
            h2 = _graph_norm(h, xa, p["n_w"], p["n_b"], p["n_ms"])
            f1 = _mm(h2, p["f_W1"], p["f_b1"], act="gelu")
            x_new = _mm(f1, p["f_W2"], p["f_b2"], add=h2)
        elif i % 3 == 1:
            hh = _mm(h, p["W"], jnp.zeros((_HID,), jnp.float32))
            adst = _gat_node(hh, p["att_dst"].reshape(_HID))
            wp = ew @ p["We"]
            bp = eb @ p["We"]
            adg = _gather_rows(adst, dst)
            hs = _gather_rows(hh, src)
            msg, wden = _gat_edge(hs, adg, edge_attr, wp, bp,
                                  p["att_src"].reshape(_HID),
                                  p["att_edge"].reshape(_HID))
            num = _sc_scatter256(msg, dst3, zhalf)
            den = _sc_scatter_den(wden, dst4, zhalf)
            x_new = _gat_final(num, den, p["bias"])
        else:
            agg = _sc_gin(h2c, src3, dst3, zhalf)
            t = _gin1(h, agg, p["eps"], p["W1"], p["b1"],
                      p["gamma"], p["beta"])
            x_new = _mm(t, p["W2"], p["b2"])
        q = params["norm%d" % i]
        if i % 3 == 1:
            h, h2c = _graph_norm(h, x_new, q["w"], q["b"], q["ms"],
                                 halves=True)
        else:
            h = _graph_norm(h, x_new, q["w"], q["b"], q["ms"])

    return _pool_head(h, batch_row, params)


# speedup vs baseline: 5.3546x; 1.0045x over previous
import functools
import math

import jax
import jax.numpy as jnp
from jax import lax
from jax.experimental import pallas as pl
from jax.experimental.pallas import tpu as pltpu
from jax.experimental.pallas import tpu_sc as plsc

_N = 10000
_E = 160000
_HID = 256
_H = 8
_C = 32
_G = 16
_EDGE_DIM = 10

_INTERPRET = False

_EBR = 4000
_NBR = 2000


def _headsum(m):
    r = lax.broadcasted_iota(jnp.int32, (_HID, _H), 0) // _C
    c = lax.broadcasted_iota(jnp.int32, (_HID, _H), 1)
    s = (r == c).astype(jnp.float32)
    return jnp.dot(m, s, preferred_element_type=jnp.float32)


def _headexpand(w):
    r = lax.broadcasted_iota(jnp.int32, (_H, _HID), 0)
    c = lax.broadcasted_iota(jnp.int32, (_H, _HID), 1) // _C
    s = (r == c).astype(jnp.float32)
    return jnp.dot(w, s, preferred_element_type=jnp.float32)


def _mm_body(act, x_ref, w_ref, b_ref, o_ref):
    y = jnp.dot(x_ref[...], w_ref[...], preferred_element_type=jnp.float32)
    y = y + b_ref[...]
    if act == "relu":
        y = jnp.maximum(y, 0.0)
    elif act == "gelu":
        y = jax.nn.gelu(y)
    o_ref[...] = y


def _mm_add_body(act, x_ref, w_ref, b_ref, a_ref, o_ref):
    y = jnp.dot(x_ref[...], w_ref[...], preferred_element_type=jnp.float32)
    y = y + b_ref[...]
    if act == "relu":
        y = jnp.maximum(y, 0.0)
    elif act == "gelu":
        y = jax.nn.gelu(y)
    o_ref[...] = y + a_ref[...]


def _mm(x, w, b, act=None, br=_NBR, add=None):
    m, k = x.shape
    n = w.shape[1]
    grid = (m // br,)
    in_specs = [
        pl.BlockSpec((br, k), lambda i: (i, 0)),
        pl.BlockSpec((k, n), lambda i: (0, 0)),
        pl.BlockSpec((1, n), lambda i: (0, 0)),
    ]
    args = [x, w, b.reshape(1, n)]
    if add is None:
        body = functools.partial(_mm_body, act)
    else:
        body = functools.partial(_mm_add_body, act)
        in_specs.append(pl.BlockSpec((br, n), lambda i: (i, 0)))
        args.append(add)
    return pl.pallas_call(
        body,
        grid=grid,
        in_specs=in_specs,
        out_specs=pl.BlockSpec((br, n), lambda i: (i, 0)),
        out_shape=jax.ShapeDtypeStruct((m, n), jnp.float32),
        interpret=_INTERPRET,
    )(*args)


def _mm_qkvs_body(x_ref, w_ref, b_ref, q_ref, kv_ref, xs_ref):
    y = jnp.dot(x_ref[...], w_ref[...], preferred_element_type=jnp.float32)
    y = y + b_ref[...]
    q_ref[...] = y[:, :_HID]
    kv_ref[...] = y[:, _HID:3 * _HID]
    xs_ref[...] = y[:, 3 * _HID:]


def _mm_qkvs(x, w, b, br=_NBR):
    grid = (_N // br,)
    return pl.pallas_call(
        _mm_qkvs_body,
        grid=grid,
        in_specs=[
            pl.BlockSpec((br, _HID), lambda i: (i, 0)),
            pl.BlockSpec((_HID, 4 * _HID), lambda i: (0, 0)),
            pl.BlockSpec((1, 4 * _HID), lambda i: (0, 0)),
        ],
        out_specs=[
            pl.BlockSpec((br, _HID), lambda i: (i, 0)),
            pl.BlockSpec((br, 2 * _HID), lambda i: (i, 0)),
            pl.BlockSpec((br, _HID), lambda i: (i, 0)),
        ],
        out_shape=[
            jax.ShapeDtypeStruct((_N, _HID), jnp.float32),
            jax.ShapeDtypeStruct((_N, 2 * _HID), jnp.float32),
            jax.ShapeDtypeStruct((_N, _HID), jnp.float32),
        ],
        interpret=_INTERPRET,
    )(x, w, b.reshape(1, 4 * _HID))


def _norm_body(x_ref, a_ref, w_ref, b_ref, ms_ref, o_ref):
    x = x_ref[...] + a_ref[...]
    mean = jnp.mean(x, axis=0, keepdims=True)
    out = x - ms_ref[...] * mean
    var = jnp.mean(out * out, axis=0, keepdims=True)
    o_ref[...] = w_ref[...] * out / jnp.sqrt(var + 1e-5) + b_ref[...]


def _norm2_body(x_ref, a_ref, w_ref, b_ref, ms_ref, o_ref, o2_ref):
    x = x_ref[...] + a_ref[...]
    mean = jnp.mean(x, axis=0, keepdims=True)
    out = x - ms_ref[...] * mean
    var = jnp.mean(out * out, axis=0, keepdims=True)
    y = w_ref[...] * out / jnp.sqrt(var + 1e-5) + b_ref[...]
    o_ref[...] = y
    o2_ref[0] = y[:, :128]
    o2_ref[1] = y[:, 128:]


def _graph_norm(x, add, w, b, ms, halves=False):
    args = (x, add, w.reshape(1, _HID), b.reshape(1, _HID),
            ms.reshape(1, _HID))
    if not halves:
        return pl.pallas_call(
            _norm_body,
            out_shape=jax.ShapeDtypeStruct((_N, _HID), jnp.float32),
            interpret=_INTERPRET,
        )(*args)
    return pl.pallas_call(
        _norm2_body,
        out_shape=[
            jax.ShapeDtypeStruct((_N, _HID), jnp.float32),
            jax.ShapeDtypeStruct((2, _N, 128), jnp.float32),
        ],
        interpret=_INTERPRET,
    )(*args)


def _tc_edge_body(ebr, qd_ref, ks_ref, vs_ref, eat_ref, wp_ref, bp_ref,
                  msg_ref, wd_ref):
    e = jnp.dot(eat_ref[...], wp_ref[...],
                preferred_element_type=jnp.float32) + bp_ref[...]
    m = qd_ref[...] * (ks_ref[...] + e)
    a = _headsum(m) * (1.0 / math.sqrt(float(_C)))
    w = jnp.exp(a)
    msg_ref[...] = (vs_ref[...] + e) * _headexpand(w)
    wd_ref[:, :_H] = w
    wd_ref[:, _H:] = jnp.zeros((ebr, 128 - _H), jnp.float32)


def _tconv_edge(qd, kvg, eattr, wp, bp, ebr=_EBR):
    elen = qd.shape[0]
    grid = (elen // ebr,)
    return pl.pallas_call(
        functools.partial(_tc_edge_body, ebr),
        grid=grid,
        in_specs=[
            pl.BlockSpec((ebr, _HID), lambda i: (i, 0)),
            pl.BlockSpec((ebr, _HID), lambda i: (i, 0)),
            pl.BlockSpec((ebr, _HID), lambda i: (i, 1)),
            pl.BlockSpec((ebr, _EDGE_DIM), lambda i: (i, 0)),
            pl.BlockSpec((_EDGE_DIM, _HID), lambda i: (0, 0)),
            pl.BlockSpec((1, _HID), lambda i: (0, 0)),
        ],
        out_specs=[
            pl.BlockSpec((ebr, _HID), lambda i: (i, 0)),
            pl.BlockSpec((ebr, 128), lambda i: (i, 0)),
        ],
        out_shape=[
            jax.ShapeDtypeStruct((elen, _HID), jnp.float32),
            jax.ShapeDtypeStruct((elen, 128), jnp.float32),
        ],
        interpret=_INTERPRET,
    )(qd, kvg, kvg, eattr, wp, bp.reshape(1, _HID))


def _tc_final_body(na_ref, nb_ref, da_ref, db_ref, xs_ref,
                   wa_ref, wb_ref, o_ref):
    den = (da_ref[0, :, :_H] + da_ref[1, :, :_H]
           + db_ref[0, :, :_H] + db_ref[1, :, :_H])
    denw = _headexpand(den)
    out = (na_ref[...] + nb_ref[...]) / (denw + 1e-16)
    xs = xs_ref[...]
    logit = jnp.sum(out * wa_ref[...] + xs * wb_ref[...], axis=1, keepdims=True)
    beta = jax.nn.sigmoid(logit)
    o_ref[...] = beta * xs + (1.0 - beta) * out


def _tconv_final(na, nb, da, db, xs, wa, wb, br=_NBR):
    grid = (_N // br,)
    return pl.pallas_call(
        _tc_final_body,
        grid=grid,
        in_specs=[
            pl.BlockSpec((br, _HID), lambda i: (i, 0)),
            pl.BlockSpec((br, _HID), lambda i: (i, 0)),
            pl.BlockSpec((2, br, 128), lambda i: (0, i, 0)),
            pl.BlockSpec((2, br, 128), lambda i: (0, i, 0)),
            pl.BlockSpec((br, _HID), lambda i: (i, 0)),
            pl.BlockSpec((1, _HID), lambda i: (0, 0)),
            pl.BlockSpec((1, _HID), lambda i: (0, 0)),
        ],
        out_specs=pl.BlockSpec((br, _HID), lambda i: (i, 0)),
        out_shape=jax.ShapeDtypeStruct((_N, _HID), jnp.float32),
        interpret=_INTERPRET,
    )(na, nb, da, db, xs, wa.reshape(1, _HID), wb.reshape(1, _HID))


def _gat_node_body(hh_ref, adf_ref, ad_ref):
    ad = _headsum(hh_ref[...] * adf_ref[...])
    ad_ref[:, :_H] = ad
    ad_ref[:, _H:] = jnp.zeros((_N, 128 - _H), jnp.float32)


def _gat_node(hh, adf):
    return pl.pallas_call(
        _gat_node_body,
        out_shape=jax.ShapeDtypeStruct((_N, 128), jnp.float32),
        interpret=_INTERPRET,
    )(hh, adf.reshape(1, _HID))


def _gat_edge_body(ebr, hs_ref, adg_ref, eat_ref, wp_ref, bp_ref,
                   asf_ref, aef_ref, msg_ref, wd_ref):
    hs = hs_ref[...]
    e = jnp.dot(eat_ref[...], wp_ref[...],
                preferred_element_type=jnp.float32) + bp_ref[...]
    asg = _headsum(hs * asf_ref[...])
    ae = _headsum(e * aef_ref[...])
    a = asg + adg_ref[:, :_H] + ae
    a = jnp.where(a >= 0.0, a, 0.2 * a)
    w = jnp.exp(a)
    msg_ref[...] = hs * _headexpand(w)
    wd_ref[:, :_H] = w
    wd_ref[:, _H:] = jnp.zeros((ebr, 128 - _H), jnp.float32)


def _gat_edge(hs, adg, eattr, wp, bp, asf, aef, ebr=_EBR):
    elen = hs.shape[0]
    grid = (elen // ebr,)
    return pl.pallas_call(
        functools.partial(_gat_edge_body, ebr),
        grid=grid,
        in_specs=[
            pl.BlockSpec((ebr, _HID), lambda i: (i, 0)),
            pl.BlockSpec((ebr, 128), lambda i: (i, 0)),
            pl.BlockSpec((ebr, _EDGE_DIM), lambda i: (i, 0)),
            pl.BlockSpec((_EDGE_DIM, _HID), lambda i: (0, 0)),
            pl.BlockSpec((1, _HID), lambda i: (0, 0)),
            pl.BlockSpec((1, _HID), lambda i: (0, 0)),
            pl.BlockSpec((1, _HID), lambda i: (0, 0)),
        ],
        out_specs=[
            pl.BlockSpec((ebr, _HID), lambda i: (i, 0)),
            pl.BlockSpec((ebr, 128), lambda i: (i, 0)),
        ],
        out_shape=[
            jax.ShapeDtypeStruct((elen, _HID), jnp.float32),
            jax.ShapeDtypeStruct((elen, 128), jnp.float32),
        ],
        interpret=_INTERPRET,
    )(hs, adg, eattr, wp, bp.reshape(1, _HID),
      asf.reshape(1, _HID), aef.reshape(1, _HID))


def _gat_final_body(na_ref, nb_ref, da_ref, db_ref, b_ref, o_ref):
    den = (da_ref[0, :, :_H] + da_ref[1, :, :_H]
           + db_ref[0, :, :_H] + db_ref[1, :, :_H])
    denw = _headexpand(den)
    o_ref[...] = (na_ref[...] + nb_ref[...]) / (denw + 1e-16) + b_ref[...]


def _gat_final(na, nb, da, db, bias, br=_NBR):
    grid = (_N // br,)
    return pl.pallas_call(
        _gat_final_body,
        grid=grid,
        in_specs=[
            pl.BlockSpec((br, _HID), lambda i: (i, 0)),
            pl.BlockSpec((br, _HID), lambda i: (i, 0)),
            pl.BlockSpec((2, br, 128), lambda i: (0, i, 0)),
            pl.BlockSpec((2, br, 128), lambda i: (0, i, 0)),
            pl.BlockSpec((1, _HID), lambda i: (0, 0)),
        ],
        out_specs=pl.BlockSpec((br, _HID), lambda i: (i, 0)),
        out_shape=jax.ShapeDtypeStruct((_N, _HID), jnp.float32),
        interpret=_INTERPRET,
    )(na, nb, da, db, bias.reshape(1, _HID))


def _gin1_body(h_ref, agg_ref, eps_ref, w_ref, b_ref, g_ref, bt_ref, o_ref):
    h0 = (1.0 + eps_ref[0, 0]) * h_ref[...] + agg_ref[...]
    h1 = jnp.dot(h0, w_ref[...], preferred_element_type=jnp.float32) + b_ref[...]
    h1 = g_ref[...] * h1 / jnp.sqrt(1.0 + 1e-5) + bt_ref[...]
    o_ref[...] = jnp.maximum(h1, 0.0)


def _gin1(h, agg, eps, w1, b1, gamma, beta, br=_NBR):
    grid = (_N // br,)
    n2 = w1.shape[1]
    return pl.pallas_call(
        _gin1_body,
        grid=grid,
        in_specs=[
            pl.BlockSpec((br, _HID), lambda i: (i, 0)),
            pl.BlockSpec((br, _HID), lambda i: (i, 0)),
            pl.BlockSpec((1, 1), lambda i: (0, 0)),
            pl.BlockSpec((_HID, n2), lambda i: (0, 0)),
            pl.BlockSpec((1, n2), lambda i: (0, 0)),
            pl.BlockSpec((1, n2), lambda i: (0, 0)),
            pl.BlockSpec((1, n2), lambda i: (0, 0)),
        ],
        out_specs=pl.BlockSpec((br, n2), lambda i: (i, 0)),
        out_shape=jax.ShapeDtypeStruct((_N, n2), jnp.float32),
        interpret=_INTERPRET,
    )(h, agg, eps.reshape(1, 1), w1, b1.reshape(1, n2),
      gamma.reshape(1, n2), beta.reshape(1, n2))


_PBR = 1000


def _pool1_body(h_ref, w1_ref, b1_ref, w2_ref, bb2_ref, l_ref, m_ref):
    t = jnp.tanh(jnp.dot(h_ref[...], w1_ref[...],
                         preferred_element_type=jnp.float32) + b1_ref[...])
    logit = jnp.sum(t * w2_ref[...], axis=1, keepdims=True) + bb2_ref[0, 0]
    l_ref[...] = logit
    bm = jnp.max(logit, axis=0, keepdims=True)

    @pl.when(pl.program_id(0) == 0)
    def _():
        m_ref[...] = bm

    @pl.when(pl.program_id(0) != 0)
    def _():
        m_ref[...] = jnp.maximum(m_ref[...], bm)


def _pool2_body(h_ref, l_ref, g_ref, br_ref, bc_ref, s_ref, mx_ref, at_ref,
                cnt_ref, es_ref):
    h = h_ref[...]
    brow = br_ref[0]
    oh_t = (lax.broadcasted_iota(jnp.int32, (_G, _PBR), 0)
            == brow).astype(jnp.float32)
    ex = jnp.exp(l_ref[...] - g_ref[...])
    s_blk = jnp.dot(oh_t, h, preferred_element_type=jnp.float32)
    at_blk = jnp.dot(oh_t, h * ex, preferred_element_type=jnp.float32)
    cnt_blk = jnp.sum(oh_t, axis=1, keepdims=True)
    es_blk = jnp.sum(ex, axis=0, keepdims=True)

    first = pl.program_id(0) == 0

    @pl.when(first)
    def _():
        s_ref[...] = s_blk
        at_ref[...] = at_blk
        cnt_ref[...] = cnt_blk
        es_ref[...] = es_blk

    @pl.when(jnp.logical_not(first))
    def _():
        s_ref[...] = s_ref[...] + s_blk
        at_ref[...] = at_ref[...] + at_blk
        cnt_ref[...] = cnt_ref[...] + cnt_blk
        es_ref[...] = es_ref[...] + es_blk

    bcol = bc_ref[...]
    for g in range(_G):
        mg = jnp.max(jnp.where(bcol == g, h, -jnp.inf), axis=0, keepdims=True)

        @pl.when(first)
        def _():
            mx_ref[pl.ds(g, 1), :] = mg

        @pl.when(jnp.logical_not(first))
        def _():
            mx_ref[pl.ds(g, 1), :] = jnp.maximum(mx_ref[pl.ds(g, 1), :], mg)


def _pool3_body(s_ref, mx_ref, at_ref, cnt_ref, es_ref,
                ow1_ref, ob1_ref, ow2_ref, ob2_ref, o_ref):
    s = s_ref[...]
    cnt = cnt_ref[...]
    mean = s / jnp.maximum(cnt, 1.0)
    attn = at_ref[...] / es_ref[0, 0]
    pooled = jnp.concatenate([mean, mx_ref[...], s, attn], axis=1)
    o1 = jnp.dot(pooled, ow1_ref[...], preferred_element_type=jnp.float32)
    o1 = jnp.maximum(o1 + ob1_ref[...], 0.0)
    o_ref[...] = jnp.dot(o1, ow2_ref[...],
                         preferred_element_type=jnp.float32) + ob2_ref[...]


def _pool_head(h, batch_row, p):
    grid = (_N // _PBR,)
    nh = p["ap_W1"].shape[1]
    logits, gmax = pl.pallas_call(
        _pool1_body,
        grid=grid,
        in_specs=[
            pl.BlockSpec((_PBR, _HID), lambda i: (i, 0)),
            pl.BlockSpec((_HID, nh), lambda i: (0, 0)),
            pl.BlockSpec((1, nh), lambda i: (0, 0)),
            pl.BlockSpec((1, nh), lambda i: (0, 0)),
            pl.BlockSpec((1, 1), lambda i: (0, 0)),
        ],
        out_specs=[
            pl.BlockSpec((_PBR, 1), lambda i: (i, 0)),
            pl.BlockSpec((1, 1), lambda i: (0, 0)),
        ],
        out_shape=[
            jax.ShapeDtypeStruct((_N, 1), jnp.float32),
            jax.ShapeDtypeStruct((1, 1), jnp.float32),
        ],
        interpret=_INTERPRET,
    )(h, p["ap_W1"], p["ap_b1"].reshape(1, nh),
      p["ap_W2"].reshape(1, nh), p["ap_b2"].reshape(1, 1))

    s, mx, at, cnt, es = pl.pallas_call(
        _pool2_body,
        grid=grid,
        in_specs=[
            pl.BlockSpec((_PBR, _HID), lambda i: (i, 0)),
            pl.BlockSpec((_PBR, 1), lambda i: (i, 0)),
            pl.BlockSpec((1, 1), lambda i: (0, 0)),
            pl.BlockSpec((1, 1, _PBR), lambda i: (i, 0, 0)),
            pl.BlockSpec((_PBR, 1), lambda i: (i, 0)),
        ],
        out_specs=[
            pl.BlockSpec((_G, _HID), lambda i: (0, 0)),
            pl.BlockSpec((_G, _HID), lambda i: (0, 0)),
            pl.BlockSpec((_G, _HID), lambda i: (0, 0)),
            pl.BlockSpec((_G, 1), lambda i: (0, 0)),
            pl.BlockSpec((1, 1), lambda i: (0, 0)),
        ],
        out_shape=[
            jax.ShapeDtypeStruct((_G, _HID), jnp.float32),
            jax.ShapeDtypeStruct((_G, _HID), jnp.float32),
            jax.ShapeDtypeStruct((_G, _HID), jnp.float32),
            jax.ShapeDtypeStruct((_G, 1), jnp.float32),
            jax.ShapeDtypeStruct((1, 1), jnp.float32),
        ],
        interpret=_INTERPRET,
    )(h, logits, gmax, batch_row.reshape(_N // _PBR, 1, _PBR),
      batch_row.reshape(_N, 1))

    return pl.pallas_call(
        _pool3_body,
        out_shape=jax.ShapeDtypeStruct((_G, _HID), jnp.float32),
        interpret=_INTERPRET,
    )(s, mx, at, cnt, es,
      p["out_W1"], p["out_b1"].reshape(1, -1),
      p["out_W2"], p["out_b2"].reshape(1, -1))


_NW = 32
_BPW = _E // _NW


def _sc_gather_fn(d, ch, bpw=_BPW):
    n_full, rem = divmod(bpw, ch)
    sizes = [ch] * n_full + ([rem] if rem else [])
    offs = [i * ch for i in range(len(sizes))]
    mesh = plsc.VectorSubcoreMesh(core_axis_name="c", subcore_axis_name="s",
                                  num_cores=2)

    @functools.partial(
        pl.kernel,
        mesh=mesh,
        out_type=jax.ShapeDtypeStruct((bpw * _NW, d), jnp.float32),
        scratch_types=[
            pltpu.VMEM((bpw,), jnp.int32),
            pltpu.VMEM((ch, d), jnp.float32),
            pltpu.VMEM((ch, d), jnp.float32),
            pltpu.SemaphoreType.DMA,
            pltpu.SemaphoreType.DMA,
        ],
    )
    def k(tab_hbm, idx_hbm, out_hbm, idx_v, buf0, buf1, sem0, sem1):
        wid = lax.axis_index("s") * 2 + lax.axis_index("c")
        base = wid * bpw
        pltpu.sync_copy(idx_hbm.at[pl.ds(base, bpw)], idx_v)
        bufs = (buf0, buf1)
        sems = (sem0, sem1)
        cps = [None, None]
        cps[0] = pltpu.async_copy(
            tab_hbm.at[idx_v.at[pl.ds(0, sizes[0])]],
            bufs[0].at[pl.ds(0, sizes[0])], sems[0])
        for i in range(len(sizes)):
            if i + 1 < len(sizes):
                cps[(i + 1) % 2] = pltpu.async_copy(
                    tab_hbm.at[idx_v.at[pl.ds(offs[i + 1], sizes[i + 1])]],
                    bufs[(i + 1) % 2].at[pl.ds(0, sizes[i + 1])],
                    sems[(i + 1) % 2])
            cps[i % 2].wait()
            pltpu.sync_copy(bufs[i % 2].at[pl.ds(0, sizes[i])],
                            out_hbm.at[pl.ds(base + offs[i], sizes[i])])

    return k


_SC_GATHER_CACHE = {}
_GATHER_CHUNK = {256: 200, 512: 96, 128: 200}


def _gather_rows(tab, idx):
    d = tab.shape[1]
    if d not in _GATHER_CHUNK:
        return jnp.take(tab, idx, axis=0)
    key = (d, idx.shape[0])
    if key not in _SC_GATHER_CACHE:
        _SC_GATHER_CACHE[key] = _sc_gather_fn(d, _GATHER_CHUNK[d],
                                              idx.shape[0] // _NW)
    return _SC_GATHER_CACHE[key](tab, idx)


def _scatter_add_rows(rows, idx, n):
    return jax.ops.segment_sum(rows, idx, num_segments=n)


_EPT = _E // 16
_SCATTER_ECH = 80


def _sc_scatter_fn(dcols, ept=_EPT):
    ech = _SCATTER_ECH
    n_chunks = ept // ech
    rpt = 624
    mesh = plsc.VectorSubcoreMesh(core_axis_name="c", subcore_axis_name="s",
                                  num_cores=2)

    @functools.partial(
        pl.kernel,
        mesh=mesh,
        out_type=jax.ShapeDtypeStruct((_N, 256), jnp.float32),
        scratch_types=[
            pltpu.VMEM((n_chunks, ech), jnp.int32),
            pltpu.VMEM((ech, 128), jnp.float32),
            pltpu.VMEM((ech, 128), jnp.float32),
            pltpu.VMEM_SHARED((_N, 128), jnp.float32),
            pltpu.SemaphoreType.DMA,
            pltpu.SemaphoreType.DMA,
        ],
    )
    def k(rows_hbm, dst3_hbm, z_hbm, out_hbm,
          idx_v, buf0, buf1, accum, sem0, sem1):
        cid = lax.axis_index("c")
        sid = lax.axis_index("s")
        base = sid * ept
        pltpu.sync_copy(dst3_hbm.at[sid], idx_v)

        @pl.when(sid == 0)
        def _():
            pltpu.sync_copy(z_hbm, accum)

        plsc.subcore_barrier()
        bufs = (buf0, buf1)
        sems = (sem0, sem1)
        cps = [None, None]
        cps[0] = pltpu.async_copy(
            rows_hbm.at[pl.ds(base, ech), pl.ds(cid * 128, 128)],
            bufs[0], sems[0])
        for i in range(n_chunks):
            if i + 1 < n_chunks:
                cps[(i + 1) % 2] = pltpu.async_copy(
                    rows_hbm.at[pl.ds(base + (i + 1) * ech, ech),
                                pl.ds(cid * 128, 128)],
                    bufs[(i + 1) % 2], sems[(i + 1) % 2])
            cps[i % 2].wait()
            pltpu.sync_copy(bufs[i % 2], accum.at[idx_v.at[i]], add=True)
        plsc.subcore_barrier()

        @pl.when(sid < 15)
        def _():
            pltpu.sync_copy(
                accum.at[pl.ds(sid * rpt, rpt)],
                out_hbm.at[pl.ds(sid * rpt, rpt), pl.ds(cid * 128, 128)])

        @pl.when(sid == 15)
        def _():
            pltpu.sync_copy(
                accum.at[pl.ds(15 * rpt, _N - 15 * rpt)],
                out_hbm.at[pl.ds(15 * rpt, _N - 15 * rpt),
                           pl.ds(cid * 128, 128)])

    return k


_DEN_ECH = 40
_DEN_BPW = _E // 32


def _sc_scatter_den_fn(bpw=_DEN_BPW):
    ech = _DEN_ECH
    n_chunks = bpw // ech
    rpt = 624
    mesh = plsc.VectorSubcoreMesh(core_axis_name="c", subcore_axis_name="s",
                                  num_cores=2)

    @functools.partial(
        pl.kernel,
        mesh=mesh,
        out_type=jax.ShapeDtypeStruct((2, _N, 128), jnp.float32),
        scratch_types=[
            pltpu.VMEM((n_chunks, ech), jnp.int32),
            pltpu.VMEM((ech, 128), jnp.float32),
            pltpu.VMEM((ech, 128), jnp.float32),
            pltpu.VMEM_SHARED((_N, 128), jnp.float32),
            pltpu.SemaphoreType.DMA,
            pltpu.SemaphoreType.DMA,
        ],
    )
    def k(rows_hbm, dst4_hbm, z_hbm, out_hbm,
          idx_v, buf0, buf1, accum, sem0, sem1):
        cid = lax.axis_index("c")
        sid = lax.axis_index("s")
        wid = cid * 16 + sid
        base = wid * bpw
        pltpu.sync_copy(dst4_hbm.at[wid], idx_v)

        @pl.when(sid == 0)
        def _():
            pltpu.sync_copy(z_hbm, accum)

        plsc.subcore_barrier()
        bufs = (buf0, buf1)
        sems = (sem0, sem1)
        cps = [None, None]
        cps[0] = pltpu.async_copy(rows_hbm.at[pl.ds(base, ech)],
                                  bufs[0], sems[0])
        for i in range(n_chunks):
            if i + 1 < n_chunks:
                cps[(i + 1) % 2] = pltpu.async_copy(
                    rows_hbm.at[pl.ds(base + (i + 1) * ech, ech)],
                    bufs[(i + 1) % 2], sems[(i + 1) % 2])
            cps[i % 2].wait()
            pltpu.sync_copy(bufs[i % 2], accum.at[idx_v.at[i]], add=True)
        plsc.subcore_barrier()

        @pl.when(sid < 15)
        def _():
            pltpu.sync_copy(accum.at[pl.ds(sid * rpt, rpt)],
                            out_hbm.at[cid, pl.ds(sid * rpt, rpt)])

        @pl.when(sid == 15)
        def _():
            pltpu.sync_copy(accum.at[pl.ds(15 * rpt, _N - 15 * rpt)],
                            out_hbm.at[cid, pl.ds(15 * rpt, _N - 15 * rpt)])

    return k


def _sc_gin_fn():
    ech = _SCATTER_ECH
    n_chunks = _EPT // ech
    rpt = 624
    mesh = plsc.VectorSubcoreMesh(core_axis_name="c", subcore_axis_name="s",
                                  num_cores=2)

    @functools.partial(
        pl.kernel,
        mesh=mesh,
        out_type=jax.ShapeDtypeStruct((_N, 256), jnp.float32),
        scratch_types=[
            pltpu.VMEM((n_chunks, ech), jnp.int32),
            pltpu.VMEM((n_chunks, ech), jnp.int32),
            pltpu.VMEM((ech, 128), jnp.float32),
            pltpu.VMEM_SHARED((_N, 128), jnp.float32),
            pltpu.SemaphoreType.DMA,
        ],
    )
    def k(h2c_hbm, src3_hbm, dst3_hbm, z_hbm,
          out_hbm, sidx_v, didx_v, buf0, accum, sem0):
        cid = lax.axis_index("c")
        sid = lax.axis_index("s")
        pltpu.sync_copy(src3_hbm.at[sid], sidx_v)
        pltpu.sync_copy(dst3_hbm.at[sid], didx_v)

        @pl.when(sid == 0)
        def _():
            pltpu.sync_copy(z_hbm, accum)

        plsc.subcore_barrier()
        for i in range(n_chunks):
            pltpu.async_copy(h2c_hbm.at[cid].at[sidx_v.at[i]],
                             buf0, sem0).wait()
            pltpu.sync_copy(buf0, accum.at[didx_v.at[i]], add=True)
        plsc.subcore_barrier()

        @pl.when(sid < 15)
        def _():
            pltpu.sync_copy(
                accum.at[pl.ds(sid * rpt, rpt)],
                out_hbm.at[pl.ds(sid * rpt, rpt), pl.ds(cid * 128, 128)])

        @pl.when(sid == 15)
        def _():
            pltpu.sync_copy(
                accum.at[pl.ds(15 * rpt, _N - 15 * rpt)],
                out_hbm.at[pl.ds(15 * rpt, _N - 15 * rpt),
                           pl.ds(cid * 128, 128)])

    return k


_SC_SCATTER_CACHE = {}


def _sc_gin(h2c, src3, dst3, zeros_half):
    if "gin" not in _SC_SCATTER_CACHE:
        _SC_SCATTER_CACHE["gin"] = _sc_gin_fn()
    return _SC_SCATTER_CACHE["gin"](h2c, src3, dst3, zeros_half)


def _sc_scatter_den(rows, dst4, zeros_half):
    key = ("den", rows.shape[0])
    if key not in _SC_SCATTER_CACHE:
        _SC_SCATTER_CACHE[key] = _sc_scatter_den_fn(rows.shape[0] // 32)
    return _SC_SCATTER_CACHE[key](rows, dst4, zeros_half)


def _sc_scatter256(rows, dst3, zeros_half):
    key = (rows.shape[1], rows.shape[0])
    if key not in _SC_SCATTER_CACHE:
        _SC_SCATTER_CACHE[key] = _sc_scatter_fn(rows.shape[1],
                                                rows.shape[0] // 16)
    return _SC_SCATTER_CACHE[key](rows, dst3, zeros_half)


def kernel(x, edge_index, edge_attr, batch, params):
    src = edge_index[0]
    dst = edge_index[1]
    batch_row = batch.reshape(1, _N)
    dst3 = dst.reshape(16, _EPT // _SCATTER_ECH, _SCATTER_ECH)
    src3 = src.reshape(16, _EPT // _SCATTER_ECH, _SCATTER_ECH)
    zhalf = jnp.zeros((_N, 128), jnp.float32)
    ecut = 81920
    halves = []
    for lo, hi in ((0, ecut), (ecut, _E)):
        ln = hi - lo
        halves.append(dict(
            sl=slice(lo, hi),
            src=src[lo:hi], dst=dst[lo:hi],
            dst3=dst[lo:hi].reshape(16, ln // 16 // _SCATTER_ECH,
                                    _SCATTER_ECH),
            dst4=dst[lo:hi].reshape(32, ln // 32 // _DEN_ECH, _DEN_ECH),
            eattr=edge_attr[lo:hi],
            ebr=(4096 if ln == 81920 else 3904),
        ))

    h = _mm(x, params["node_W"], params["node_b"])
    ew = params["edge_W"]
    eb = params["edge_b"]

    for i in range(6):
        p = params["layer%d" % i]
        if i % 3 == 0:
            wq = jnp.concatenate([p["Wq"], p["Wk"], p["Wv"], p["Wskip"]], axis=1)
            bq = jnp.concatenate([p["bq"], p["bk"], p["bv"], p["bskip"]])
            q, kv, xs = _mm_qkvs(h, wq, bq)
            wp = ew @ p["We"]
            bp = eb @ p["We"] + p["be"]
            nums, dens = [], []
            for hv in halves:
                qd = _gather_rows(q, hv["dst"])
                kvg = _gather_rows(kv, hv["src"])
                msg, wden = _tconv_edge(qd, kvg, hv["eattr"], wp, bp,
                                        ebr=hv["ebr"])
                nums.append(_sc_scatter256(msg, hv["dst3"], zhalf))
                dens.append(_sc_scatter_den(wden, hv["dst4"], zhalf))
            wb = p["Wbeta"].reshape(3, _HID)
            wa_out = wb[0] + wb[2]
            wa_xs = wb[1] - wb[2]
            xa = _tconv_final(nums[0], nums[1], dens[0], dens[1],
                              xs, wa_out, wa_xs)
            h2 = _graph_norm(h, xa, p["n_w"], p["n_b"], p["n_ms"])
            f1 = _mm(h2, p["f_W1"], p["f_b1"], act="gelu")
            x_new = _mm(f1, p["f_W2"], p["f_b2"], add=h2)
        elif i % 3 == 1:
            hh = _mm(h, p["W"], jnp.zeros((_HID,), jnp.float32))
            adst = _gat_node(hh, p["att_dst"].reshape(_HID))
            wp = ew @ p["We"]
            bp = eb @ p["We"]
            nums, dens = [], []
            for hv in halves:
                adg = _gather_rows(adst, hv["dst"])
                hs = _gather_rows(hh, hv["src"])
                msg, wden = _gat_edge(hs, adg, hv["eattr"], wp, bp,
                                      p["att_src"].reshape(_HID),
                                      p["att_edge"].reshape(_HID),
                                      ebr=hv["ebr"])
                nums.append(_sc_scatter256(msg, hv["dst3"], zhalf))
                dens.append(_sc_scatter_den(wden, hv["dst4"], zhalf))
            x_new = _gat_final(nums[0], nums[1], dens[0], dens[1],
                               p["bias"])
        else:
            agg = _sc_gin(h2c, src3, dst3, zhalf)
            t = _gin1(h, agg, p["eps"], p["W1"], p["b1"],
                      p["gamma"], p["beta"])
            x_new = _mm(t, p["W2"], p["b2"])
        q = params["norm%d" % i]
        if i % 3 == 1:
            h, h2c = _graph_norm(h, x_new, q["w"], q["b"], q["ms"],
                                 halves=True)
        else:
            h = _graph_norm(h, x_new, q["w"], q["b"], q["ms"])

    return _pool_head(h, batch_row, params)
